# baseline (speedup 1.0000x reference)
import functools

import jax
import jax.numpy as jnp
from jax import lax
from jax.experimental import pallas as pl
from jax.experimental.pallas import tpu as pltpu
from jax.experimental.pallas import tpu_sc as plsc

_N = 10000
_E = 320000
_F = 128
_HID = 64
_OUT = 128
_K = 32
_LAT = 64
_B = 8

_NW = 32
_EPW = _E // _NW
_C = 80
_NSUB = 5
_STEP = _C * _NSUB
_NITER = _EPW // _STEP
_ROWS = _N // 16
_MW = 128

_f32 = jnp.float32

_MESH = plsc.VectorSubcoreMesh(core_axis_name="c", subcore_axis_name="s")


def _silu(x):
    return x / (1.0 + jnp.exp(-x))


def _gather(pd, ps, src, dst, geo_tabs=None):
    with_geo = geo_tabs is not None
    scratch = (
        [pltpu.VMEM((_C,), jnp.int32)] * (2 * _NSUB)
        + [pltpu.VMEM((_C, 128), _f32)] * (2 * _NSUB)
        + ([pltpu.VMEM((_C,), _f32)] * (12 * _NSUB) if with_geo else [])
        + [pltpu.SemaphoreType.DMA]
    )
    out_type = [
        jax.ShapeDtypeStruct((_E, 128), _f32),
        jax.ShapeDtypeStruct((_E, 128), _f32),
    ] + ([jax.ShapeDtypeStruct((_E,), _f32)] * 4 if with_geo else [])

    @functools.partial(
        pl.kernel, mesh=_MESH, out_type=out_type, scratch_types=scratch,
    )
    def k(*args):
        n_in = 8 if with_geo else 4
        n_out = 6 if with_geo else 2
        ins = args[:n_in]
        outs = args[n_in:n_in + n_out]
        scr = args[n_in + n_out:]
        if with_geo:
            pd_h, ps_h, src_h, dst_h, px_h, py_h, vx_h, vy_h = ins
            gd_o, gs_o, rx_o, ry_o, dd_o, dt_o = outs
            tabs = (px_h, py_h, vx_h, vy_h)
            gouts = (rx_o, ry_o, dd_o, dt_o)
        else:
            pd_h, ps_h, src_h, dst_h = ins
            gd_o, gs_o = outs
        idxd = scr[0:_NSUB]
        idxs = scr[_NSUB:2 * _NSUB]
        bufd = scr[2 * _NSUB:3 * _NSUB]
        bufs = scr[3 * _NSUB:4 * _NSUB]
        if with_geo:
            gb = scr[4 * _NSUB:12 * _NSUB]
            ob = scr[12 * _NSUB:16 * _NSUB]
            sem = scr[16 * _NSUB]
        else:
            sem = scr[4 * _NSUB]
        wid = lax.axis_index("s") * 2 + lax.axis_index("c")
        base = wid * _EPW

        def step(i, _):
            offs = [pl.multiple_of(base + i * _STEP + j * _C, 8)
                    for j in range(_NSUB)]
            cps = []
            for j in range(_NSUB):
                cps.append(pltpu.async_copy(dst_h.at[pl.ds(offs[j], _C)], idxd[j], sem))
                cps.append(pltpu.async_copy(src_h.at[pl.ds(offs[j], _C)], idxs[j], sem))
            for cp in cps:
                cp.wait()
            cps = []
            for j in range(_NSUB):
                cps.append(pltpu.async_copy(pd_h.at[idxd[j]], bufd[j], sem))
                cps.append(pltpu.async_copy(ps_h.at[idxs[j]], bufs[j], sem))
                if with_geo:
                    for t in range(4):
                        cps.append(pltpu.async_copy(
                            tabs[t].at[idxd[j]], gb[8 * j + t], sem))
                        cps.append(pltpu.async_copy(
                            tabs[t].at[idxs[j]], gb[8 * j + 4 + t], sem))
            for cp in cps:
                cp.wait()
            if with_geo:
                for j in range(_NSUB):
                    for g in range(_C // 16):
                        sl = pl.ds(pl.multiple_of(g * 16, 8), 16)
                        rx = gb[8 * j + 4][sl] - gb[8 * j + 0][sl]
                        ry = gb[8 * j + 5][sl] - gb[8 * j + 1][sl]
                        wx = gb[8 * j + 6][sl] - gb[8 * j + 2][sl]
                        wy = gb[8 * j + 7][sl] - gb[8 * j + 3][sl]
                        ob[4 * j + 0][sl] = rx
                        ob[4 * j + 1][sl] = ry
                        ob[4 * j + 2][sl] = rx * rx + ry * ry
                        ob[4 * j + 3][sl] = wx * rx + wy * ry
            cps = []
            for j in range(_NSUB):
                cps.append(pltpu.async_copy(bufd[j], gd_o.at[pl.ds(offs[j], _C)], sem))
                cps.append(pltpu.async_copy(bufs[j], gs_o.at[pl.ds(offs[j], _C)], sem))
                if with_geo:
                    for t in range(4):
                        cps.append(pltpu.async_copy(
                            ob[4 * j + t], gouts[t].at[pl.ds(offs[j], _C)], sem))
            for cp in cps:
                cp.wait()
            return 0

        lax.fori_loop(0, _NITER, step, 0)

    if with_geo:
        return k(pd, ps, src, dst, *geo_tabs)
    return k(pd, ps, src, dst)


_CS = 40
_SSTEP = _CS * _NSUB


def _scatter(m, dst, zeros):
    scratch = (
        [pltpu.VMEM((_CS,), jnp.int32)] * _NSUB
        + [pltpu.VMEM((_CS, _MW), _f32)] * _NSUB
        + [pltpu.VMEM_SHARED((_N, _MW), _f32), pltpu.SemaphoreType.DMA]
    )

    @functools.partial(
        pl.kernel,
        mesh=_MESH,
        out_type=jax.ShapeDtypeStruct((2, _N, _MW), _f32),
        scratch_types=scratch,
    )
    def k(m_h, dst_h, z_h, out_h, *scr):
        idx = scr[0:_NSUB]
        buf = scr[_NSUB:2 * _NSUB]
        table = scr[2 * _NSUB]
        sem = scr[2 * _NSUB + 1]
        cid = lax.axis_index("c")
        sid = lax.axis_index("s")
        row0 = pl.multiple_of(sid * 624, 8)

        @pl.when(sid < 15)
        def _():
            pltpu.sync_copy(z_h.at[pl.ds(row0, 624)],
                            table.at[pl.ds(row0, 624)])

        @pl.when(sid == 15)
        def _():
            pltpu.sync_copy(z_h.at[pl.ds(9360, 640)],
                            table.at[pl.ds(9360, 640)])

        plsc.subcore_barrier()
        base = cid * (_E // 2) + sid * _EPW

        def step(i, _):
            offs = [pl.multiple_of(base + i * _SSTEP + j * _CS, 8)
                    for j in range(_NSUB)]
            cps = []
            for j in range(_NSUB):
                cps.append(pltpu.async_copy(dst_h.at[pl.ds(offs[j], _CS)], idx[j], sem))
                cps.append(pltpu.async_copy(m_h.at[pl.ds(offs[j], _CS)], buf[j], sem))
            for cp in cps:
                cp.wait()
            cps = []
            for j in range(_NSUB):
                cps.append(pltpu.async_copy(buf[j], table.at[idx[j]], sem, add=True))
            for cp in cps:
                cp.wait()
            return 0

        lax.fori_loop(0, _EPW // _SSTEP, step, 0)
        plsc.subcore_barrier()

        @pl.when(sid < 15)
        def _():
            pltpu.sync_copy(table.at[pl.ds(row0, 624)],
                            out_h.at[cid, pl.ds(row0, 624)])

        @pl.when(sid == 15)
        def _():
            pltpu.sync_copy(table.at[pl.ds(9360, 640)],
                            out_h.at[cid, pl.ds(9360, 640)])

    return k(m, dst, zeros)


def _proj(feat, wcat, bcat):
    nb = 2000

    def body(f_ref, w_ref, b_ref, pd_ref, ps_ref):
        p = jnp.dot(f_ref[...], w_ref[...], preferred_element_type=_f32)
        p = p + b_ref[...]
        pd_ref[...] = p[:, :128]
        ps_ref[...] = p[:, 128:]

    return pl.pallas_call(
        body,
        grid=(_N // nb,),
        in_specs=[
            pl.BlockSpec((nb, 128), lambda i: (i, 0)),
            pl.BlockSpec((128, 256), lambda i: (0, 0)),
            pl.BlockSpec((1, 256), lambda i: (0, 0)),
        ],
        out_specs=[pl.BlockSpec((nb, 128), lambda i: (i, 0))] * 2,
        out_shape=[jax.ShapeDtypeStruct((_N, 128), _f32)] * 2,
    )(feat, wcat, bcat)


def _edge_call(gd, gs, rx, ry, dd, dt, wg, e2w, e2b, e3w, e3b, v2row, v2b):
    eb = 5000

    def body(gd_ref, gs_ref, rx_ref, ry_ref, dd_ref, dt_ref, wg_ref, e2w_ref,
             e2b_ref, e3w_ref, e3b_ref, v2_ref, v2b_ref, m_ref):
        gdv = gd_ref[...]
        gsv = gs_ref[...]
        dist = dd_ref[...]
        dot = dt_ref[...]
        wgv = wg_ref[...]
        th = gdv[:, :64] + gsv[:, :64] + dist * wgv[0:1] + dot * wgv[1:2]
        th = _silu(th)
        th = _silu(jnp.dot(th, e2w_ref[...], preferred_element_type=_f32)
                   + e2b_ref[...])
        mh = jnp.dot(th, e3w_ref[...], preferred_element_type=_f32) + e3b_ref[...]
        tv = gdv[:, 64:] + gsv[:, 64:] + dist * wgv[2:3] + dot * wgv[3:4]
        tv = _silu(tv)
        vw = jnp.sum(tv * v2_ref[...], axis=1, keepdims=True) + v2b_ref[...]
        mv = jnp.concatenate([vw * rx_ref[...], vw * ry_ref[...]], axis=1)
        m_ref[...] = jnp.concatenate(
            [mh, mv, jnp.zeros((eb, _MW - 66), _f32)], axis=1)

    return pl.pallas_call(
        body,
        grid=(_E // eb,),
        in_specs=[
            pl.BlockSpec((eb, 128), lambda i: (i, 0)),
            pl.BlockSpec((eb, 128), lambda i: (i, 0)),
            pl.BlockSpec((eb, 1), lambda i: (i, 0)),
            pl.BlockSpec((eb, 1), lambda i: (i, 0)),
            pl.BlockSpec((eb, 1), lambda i: (i, 0)),
            pl.BlockSpec((eb, 1), lambda i: (i, 0)),
            pl.BlockSpec((4, 64), lambda i: (0, 0)),
            pl.BlockSpec((64, 64), lambda i: (0, 0)),
            pl.BlockSpec((1, 64), lambda i: (0, 0)),
            pl.BlockSpec((64, 64), lambda i: (0, 0)),
            pl.BlockSpec((1, 64), lambda i: (0, 0)),
            pl.BlockSpec((1, 64), lambda i: (0, 0)),
            pl.BlockSpec((1, 1), lambda i: (0, 0)),
        ],
        out_specs=pl.BlockSpec((eb, _MW), lambda i: (i, 0)),
        out_shape=jax.ShapeDtypeStruct((_E, _MW), _f32),
    )(gd, gs, rx, ry, dd, dt, wg, e2w, e2b, e3w, e3b, v2row, v2b)


def _node_proj(feat, msum, w, wcat2, bcat2):
    nb = 2000

    def body(f_ref, ms_ref, wx_ref, wm_ref, wn_ref, h1b_ref, h2w_ref,
             h2b_ref, g_ref, b_ref, wc_ref, bc_ref, o_ref, pd_ref, ps_ref):
        f = f_ref[...]
        m = ms_ref[0] + ms_ref[1]
        mvx = m[:, 64:65]
        mvy = m[:, 65:66]
        mvn = jnp.sqrt(mvx * mvx + mvy * mvy + 1e-12)
        hh = (jnp.dot(f, wx_ref[...], preferred_element_type=_f32)
              + jnp.dot(m, wm_ref[...], preferred_element_type=_f32)
              + mvn * wn_ref[...] + h1b_ref[...])
        hh = _silu(hh)
        up = jnp.dot(hh, h2w_ref[...], preferred_element_type=_f32) + h2b_ref[...]
        y = jnp.maximum(f + up, 0.0)
        mu = jnp.mean(y, axis=1, keepdims=True)
        yc = y - mu
        var = jnp.mean(yc * yc, axis=1, keepdims=True)
        h = yc * jax.lax.rsqrt(var + 1e-5) * g_ref[...] + b_ref[...]
        o_ref[...] = h
        p = jnp.dot(h, wc_ref[...], preferred_element_type=_f32) + bc_ref[...]
        pd_ref[...] = p[:, :128]
        ps_ref[...] = p[:, 128:]

    return pl.pallas_call(
        body,
        grid=(_N // nb,),
        in_specs=[
            pl.BlockSpec((nb, 128), lambda i: (i, 0)),
            pl.BlockSpec((2, nb, _MW), lambda i: (0, i, 0)),
            pl.BlockSpec((128, 64), lambda i: (0, 0)),
            pl.BlockSpec((_MW, 64), lambda i: (0, 0)),
            pl.BlockSpec((1, 64), lambda i: (0, 0)),
            pl.BlockSpec((1, 64), lambda i: (0, 0)),
            pl.BlockSpec((64, 128), lambda i: (0, 0)),
            pl.BlockSpec((1, 128), lambda i: (0, 0)),
            pl.BlockSpec((1, 128), lambda i: (0, 0)),
            pl.BlockSpec((1, 128), lambda i: (0, 0)),
            pl.BlockSpec((128, 256), lambda i: (0, 0)),
            pl.BlockSpec((1, 256), lambda i: (0, 0)),
        ],
        out_specs=[
            pl.BlockSpec((nb, 128), lambda i: (i, 0)),
            pl.BlockSpec((nb, 128), lambda i: (i, 0)),
            pl.BlockSpec((nb, 128), lambda i: (i, 0)),
        ],
        out_shape=[jax.ShapeDtypeStruct((_N, 128), _f32)] * 3,
    )(feat, msum, w['wx'], w['wm'], w['wn'], w['h1b'], w['h2w'], w['h2b'],
      w['g'], w['b'], wcat2, bcat2)


def _node_pool(feat, msum, w, bcol, pos, poolw, poolb, o1w, o1b, o2w, o2b,
               gain):
    nb = 2000
    nsteps = _N // nb

    def body(f_ref, ms_ref, wx_ref, wm_ref, wn_ref, h1b_ref, h2w_ref,
             h2b_ref, g_ref, b_ref, bcol_ref, p_ref, pw_ref, pb_ref,
             o1w_ref, o1b_ref, o2w_ref, o2b_ref, gn_ref,
             s_ref, lat_ref, mu_ref, loss_ref, acc_ref):
        f = f_ref[...]
        m = ms_ref[0] + ms_ref[1]
        mvx = m[:, 64:65]
        mvy = m[:, 65:66]
        mvn = jnp.sqrt(mvx * mvx + mvy * mvy + 1e-12)
        hh = (jnp.dot(f, wx_ref[...], preferred_element_type=_f32)
              + jnp.dot(m, wm_ref[...], preferred_element_type=_f32)
              + mvn * wn_ref[...] + h1b_ref[...])
        hh = _silu(hh)
        up = jnp.dot(hh, h2w_ref[...], preferred_element_type=_f32) + h2b_ref[...]
        y = jnp.maximum(f + up, 0.0)
        mu_ = jnp.mean(y, axis=1, keepdims=True)
        yc = y - mu_
        var = jnp.mean(yc * yc, axis=1, keepdims=True)
        hv = yc * jax.lax.rsqrt(var + 1e-5) * g_ref[...] + b_ref[...]

        logits = jnp.dot(hv, pw_ref[...], preferred_element_type=_f32) + pb_ref[...]
        mx = jnp.max(logits, axis=1, keepdims=True)
        ex = jnp.exp(logits - mx)
        s = ex / jnp.sum(ex, axis=1, keepdims=True)
        s_ref[...] = s
        bc = bcol_ref[...]
        lane = lax.broadcasted_iota(jnp.int32, (nb, 256), 1) // _K
        stile = jnp.concatenate([s] * _B, axis=1)
        wm_ = jnp.where(lane == bc, stile, 0.0)
        entcol = jnp.sum(s * jnp.log(s + 1e-8), axis=1, keepdims=True)
        ones = jnp.ones((nb, 1), _f32)
        w_ext = jnp.concatenate(
            [wm_, s, ones, jnp.zeros((nb, 7), _f32)], axis=1)
        r_ext = jnp.concatenate(
            [hv, p_ref[...], ones, entcol, jnp.zeros((nb, 4), _f32)],
            axis=1)
        acc = lax.dot_general(w_ext, r_ext, (((0,), (0,)), ((), ())),
                              preferred_element_type=_f32)

        @pl.when(pl.program_id(0) == 0)
        def _():
            acc_ref[...] = acc

        @pl.when(pl.program_id(0) != 0)
        def _():
            acc_ref[...] += acc

        @pl.when(pl.program_id(0) == nsteps - 1)
        def _():
            a = acc_ref[...]
            den = a[:256, 130:131] + 1e-8
            pooled = a[:256, :128] / den
            z = jnp.maximum(
                jnp.dot(pooled, o1w_ref[...], preferred_element_type=_f32)
                + o1b_ref[...], 0.0)
            lat_ref[...] = (jnp.dot(z, o2w_ref[...],
                                    preferred_element_type=_f32)
                            + o2b_ref[...]) * gn_ref[...]
            mu_ref[...] = a[:256, 128:130] / den
            usage = a[256:288, 130:131] * (1.0 / _N)
            lb = jnp.sum(usage * jnp.log(usage * _K + 1e-8), axis=0,
                         keepdims=True)
            ent = -a[288:289, 131:132] * (1.0 / _N)
            loss_ref[...] = ent + lb

    return pl.pallas_call(
        body,
        grid=(nsteps,),
        in_specs=[
            pl.BlockSpec((nb, 128), lambda i: (i, 0)),
            pl.BlockSpec((2, nb, _MW), lambda i: (0, i, 0)),
            pl.BlockSpec((128, 64), lambda i: (0, 0)),
            pl.BlockSpec((_MW, 64), lambda i: (0, 0)),
            pl.BlockSpec((1, 64), lambda i: (0, 0)),
            pl.BlockSpec((1, 64), lambda i: (0, 0)),
            pl.BlockSpec((64, 128), lambda i: (0, 0)),
            pl.BlockSpec((1, 128), lambda i: (0, 0)),
            pl.BlockSpec((1, 128), lambda i: (0, 0)),
            pl.BlockSpec((1, 128), lambda i: (0, 0)),
            pl.BlockSpec((nb, 1), lambda i: (i, 0)),
            pl.BlockSpec((nb, 2), lambda i: (i, 0)),
            pl.BlockSpec((128, _K), lambda i: (0, 0)),
            pl.BlockSpec((1, _K), lambda i: (0, 0)),
            pl.BlockSpec((128, 128), lambda i: (0, 0)),
            pl.BlockSpec((1, 128), lambda i: (0, 0)),
            pl.BlockSpec((128, _LAT), lambda i: (0, 0)),
            pl.BlockSpec((1, _LAT), lambda i: (0, 0)),
            pl.BlockSpec((1, _LAT), lambda i: (0, 0)),
        ],
        out_specs=[
            pl.BlockSpec((nb, _K), lambda i: (i, 0)),
            pl.BlockSpec((256, _LAT), lambda i: (0, 0)),
            pl.BlockSpec((256, 2), lambda i: (0, 0)),
            pl.BlockSpec((1, 1), lambda i: (0, 0)),
        ],
        out_shape=[
            jax.ShapeDtypeStruct((_N, _K), _f32),
            jax.ShapeDtypeStruct((256, _LAT), _f32),
            jax.ShapeDtypeStruct((256, 2), _f32),
            jax.ShapeDtypeStruct((1, 1), _f32),
        ],
        scratch_shapes=[pltpu.VMEM((296, 136), _f32)],
    )(feat, msum, w['wx'], w['wm'], w['wn'], w['h1b'], w['h2w'], w['h2b'],
      w['g'], w['b'], bcol, pos, poolw, poolb, o1w, o1b, o2w, o2b, gain)


def _layer_weights(p):
    e1w, e1b = p['e1']
    v1w, v1b = p['v1']
    wcat = jnp.concatenate(
        [e1w[:_F], v1w[:_F], e1w[_F:2 * _F], v1w[_F:2 * _F]], axis=1)
    bcat = jnp.concatenate(
        [e1b, v1b, jnp.zeros((2 * _HID,), _f32)]).reshape(1, 256)
    wg = jnp.concatenate([e1w[2 * _F:], v1w[2 * _F:]], axis=0)
    h1w, h1b = p['h1']
    wx = h1w[:_F]
    wm = jnp.concatenate([h1w[_F:_F + 64], jnp.zeros((_MW - 64, 64), _f32)],
                         axis=0)
    wn = h1w[_F + 64].reshape(1, 64)
    return dict(
        wcat=wcat, bcat=bcat, wg=wg,
        e2w=p['e2'][0], e2b=p['e2'][1].reshape(1, 64),
        e3w=p['e3'][0], e3b=p['e3'][1].reshape(1, 64),
        v2row=p['v2'][0].reshape(1, 64), v2b=p['v2'][1].reshape(1, 1),
        wx=wx, wm=wm, wn=wn, h1b=h1b.reshape(1, 64),
        h2w=p['h2'][0], h2b=p['h2'][1].reshape(1, 128),
    )


def kernel(x, edge_index, batch, p1, p2, ln1, ln2, pool, out1, out2,
           latent_gain):
    src = edge_index[0]
    dst = edge_index[1]
    pos = x[:, :2]
    zeros_tab = jnp.zeros((_N, _MW), _f32)
    bcol = batch.reshape(_N, 1)

    w1 = _layer_weights(p1)
    w1['g'] = ln1[0].reshape(1, 128)
    w1['b'] = ln1[1].reshape(1, 128)
    w2 = _layer_weights(p2)
    w2['g'] = ln2[0].reshape(1, 128)
    w2['b'] = ln2[1].reshape(1, 128)

    pd, ps = _proj(x, w1['wcat'], w1['bcat'])
    gd, gs, rx, ry, dd, dt = _gather(
        pd, ps, src, dst, geo_tabs=(x[:, 0], x[:, 1], x[:, 2], x[:, 3]))
    rx = rx.reshape(_E, 1)
    ry = ry.reshape(_E, 1)
    dd = dd.reshape(_E, 1)
    dt = dt.reshape(_E, 1)
    m1 = _edge_call(gd, gs, rx, ry, dd, dt, w1['wg'], w1['e2w'], w1['e2b'],
                    w1['e3w'], w1['e3b'], w1['v2row'], w1['v2b'])
    msum1 = _scatter(m1, dst, zeros_tab)
    h1, pd2, ps2 = _node_proj(x, msum1, w1, w2['wcat'], w2['bcat'])

    gd2, gs2 = _gather(pd2, ps2, src, dst)
    m2 = _edge_call(gd2, gs2, rx, ry, dd, dt, w2['wg'], w2['e2w'], w2['e2b'],
                    w2['e3w'], w2['e3b'], w2['v2row'], w2['v2b'])
    msum2 = _scatter(m2, dst, zeros_tab)
    s, lat, mu, loss = _node_pool(
        h1, msum2, w2, bcol, pos, pool[0], pool[1].reshape(1, _K), out1[0],
        out1[1].reshape(1, 128), out2[0], out2[1].reshape(1, _LAT),
        latent_gain.reshape(1, _LAT))
    return (lat.reshape(_B, _K, _LAT), s, loss[0, 0],
            mu.reshape(_B, _K, 2))


# speedup vs baseline: 1.0768x; 1.0768x over previous
import functools

import jax
import jax.numpy as jnp
from jax import lax
from jax.experimental import pallas as pl
from jax.experimental.pallas import tpu as pltpu
from jax.experimental.pallas import tpu_sc as plsc

_N = 10000
_E = 320000
_F = 128
_HID = 64
_OUT = 128
_K = 32
_LAT = 64
_B = 8

_NW = 32
_EPW = _E // _NW
_C = 80
_NSUB = 5
_STEP = _C * _NSUB
_NITER = _EPW // _STEP
_ROWS = _N // 16
_MW = 128

_f32 = jnp.float32

_MESH = plsc.VectorSubcoreMesh(core_axis_name="c", subcore_axis_name="s")


def _silu(x):
    return x / (1.0 + jnp.exp(-x))


def _geo(px, py, vx, vy, src, dst):
    scratch = (
        [pltpu.VMEM((_C,), jnp.int32)] * (2 * _NSUB)
        + [pltpu.VMEM((_C,), _f32)] * (8 * _NSUB)
        + [pltpu.VMEM((_C,), _f32)] * (4 * _NSUB)
        + [pltpu.SemaphoreType.DMA]
    )

    @functools.partial(
        pl.kernel,
        mesh=_MESH,
        out_type=[jax.ShapeDtypeStruct((_E,), _f32)] * 4,
        scratch_types=scratch,
    )
    def k(px_h, py_h, vx_h, vy_h, src_h, dst_h, rx_o, ry_o, dd_o, dt_o, *scr):
        idxd = scr[0:_NSUB]
        idxs = scr[_NSUB:2 * _NSUB]
        gb = scr[2 * _NSUB:10 * _NSUB]
        ob = scr[10 * _NSUB:14 * _NSUB]
        sem = scr[14 * _NSUB]
        wid = lax.axis_index("s") * 2 + lax.axis_index("c")
        base = wid * _EPW
        tabs = (px_h, py_h, vx_h, vy_h)
        gouts = (rx_o, ry_o, dd_o, dt_o)

        def step(i, _):
            offs = [pl.multiple_of(base + i * _STEP + j * _C, 8)
                    for j in range(_NSUB)]
            cps = []
            for j in range(_NSUB):
                cps.append(pltpu.async_copy(dst_h.at[pl.ds(offs[j], _C)], idxd[j], sem))
                cps.append(pltpu.async_copy(src_h.at[pl.ds(offs[j], _C)], idxs[j], sem))
            for cp in cps:
                cp.wait()
            cps = []
            for j in range(_NSUB):
                for t in range(4):
                    cps.append(pltpu.async_copy(
                        tabs[t].at[idxd[j]], gb[8 * j + t], sem))
                    cps.append(pltpu.async_copy(
                        tabs[t].at[idxs[j]], gb[8 * j + 4 + t], sem))
            for cp in cps:
                cp.wait()
            for j in range(_NSUB):
                for g in range(_C // 16):
                    sl = pl.ds(pl.multiple_of(g * 16, 8), 16)
                    rx = gb[8 * j + 4][sl] - gb[8 * j + 0][sl]
                    ry = gb[8 * j + 5][sl] - gb[8 * j + 1][sl]
                    wx = gb[8 * j + 6][sl] - gb[8 * j + 2][sl]
                    wy = gb[8 * j + 7][sl] - gb[8 * j + 3][sl]
                    ob[4 * j + 0][sl] = rx
                    ob[4 * j + 1][sl] = ry
                    ob[4 * j + 2][sl] = rx * rx + ry * ry
                    ob[4 * j + 3][sl] = wx * rx + wy * ry
            cps = []
            for j in range(_NSUB):
                for t in range(4):
                    cps.append(pltpu.async_copy(
                        ob[4 * j + t], gouts[t].at[pl.ds(offs[j], _C)], sem))
            for cp in cps:
                cp.wait()
            return 0

        lax.fori_loop(0, _NITER, step, 0)

    return k(px, py, vx, vy, src, dst)


def _gather(pd, ps, src, dst, geo_tabs=None):
    with_geo = geo_tabs is not None
    scratch = (
        [pltpu.VMEM((_C,), jnp.int32)] * (2 * _NSUB)
        + [pltpu.VMEM((_C, 128), _f32)] * (2 * _NSUB)
        + ([pltpu.VMEM((_C,), _f32)] * (12 * _NSUB) if with_geo else [])
        + [pltpu.SemaphoreType.DMA]
    )
    out_type = [
        jax.ShapeDtypeStruct((_E, 128), _f32),
        jax.ShapeDtypeStruct((_E, 128), _f32),
    ] + ([jax.ShapeDtypeStruct((_E,), _f32)] * 4 if with_geo else [])

    @functools.partial(
        pl.kernel, mesh=_MESH, out_type=out_type, scratch_types=scratch,
    )
    def k(*args):
        n_in = 8 if with_geo else 4
        n_out = 6 if with_geo else 2
        ins = args[:n_in]
        outs = args[n_in:n_in + n_out]
        scr = args[n_in + n_out:]
        if with_geo:
            pd_h, ps_h, src_h, dst_h, px_h, py_h, vx_h, vy_h = ins
            gd_o, gs_o, rx_o, ry_o, dd_o, dt_o = outs
            tabs = (px_h, py_h, vx_h, vy_h)
            gouts = (rx_o, ry_o, dd_o, dt_o)
        else:
            pd_h, ps_h, src_h, dst_h = ins
            gd_o, gs_o = outs
        idxd = scr[0:_NSUB]
        idxs = scr[_NSUB:2 * _NSUB]
        bufd = scr[2 * _NSUB:3 * _NSUB]
        bufs = scr[3 * _NSUB:4 * _NSUB]
        if with_geo:
            gb = scr[4 * _NSUB:12 * _NSUB]
            ob = scr[12 * _NSUB:16 * _NSUB]
            sem = scr[16 * _NSUB]
        else:
            sem = scr[4 * _NSUB]
        wid = lax.axis_index("s") * 2 + lax.axis_index("c")
        base = wid * _EPW

        def step(i, _):
            offs = [pl.multiple_of(base + i * _STEP + j * _C, 8)
                    for j in range(_NSUB)]
            cps = []
            for j in range(_NSUB):
                cps.append(pltpu.async_copy(dst_h.at[pl.ds(offs[j], _C)], idxd[j], sem))
                cps.append(pltpu.async_copy(src_h.at[pl.ds(offs[j], _C)], idxs[j], sem))
            for cp in cps:
                cp.wait()
            cps = []
            for j in range(_NSUB):
                cps.append(pltpu.async_copy(pd_h.at[idxd[j]], bufd[j], sem))
                cps.append(pltpu.async_copy(ps_h.at[idxs[j]], bufs[j], sem))
                if with_geo:
                    for t in range(4):
                        cps.append(pltpu.async_copy(
                            tabs[t].at[idxd[j]], gb[8 * j + t], sem))
                        cps.append(pltpu.async_copy(
                            tabs[t].at[idxs[j]], gb[8 * j + 4 + t], sem))
            for cp in cps:
                cp.wait()
            if with_geo:
                for j in range(_NSUB):
                    for g in range(_C // 16):
                        sl = pl.ds(pl.multiple_of(g * 16, 8), 16)
                        rx = gb[8 * j + 4][sl] - gb[8 * j + 0][sl]
                        ry = gb[8 * j + 5][sl] - gb[8 * j + 1][sl]
                        wx = gb[8 * j + 6][sl] - gb[8 * j + 2][sl]
                        wy = gb[8 * j + 7][sl] - gb[8 * j + 3][sl]
                        ob[4 * j + 0][sl] = rx
                        ob[4 * j + 1][sl] = ry
                        ob[4 * j + 2][sl] = rx * rx + ry * ry
                        ob[4 * j + 3][sl] = wx * rx + wy * ry
            cps = []
            for j in range(_NSUB):
                cps.append(pltpu.async_copy(bufd[j], gd_o.at[pl.ds(offs[j], _C)], sem))
                cps.append(pltpu.async_copy(bufs[j], gs_o.at[pl.ds(offs[j], _C)], sem))
                if with_geo:
                    for t in range(4):
                        cps.append(pltpu.async_copy(
                            ob[4 * j + t], gouts[t].at[pl.ds(offs[j], _C)], sem))
            for cp in cps:
                cp.wait()
            return 0

        lax.fori_loop(0, _NITER, step, 0)

    if with_geo:
        return k(pd, ps, src, dst, *geo_tabs)
    return k(pd, ps, src, dst)


_CS = 40
_SSTEP = _CS * _NSUB


def _scatter(m, dst, zeros):
    scratch = (
        [pltpu.VMEM((_CS,), jnp.int32)] * _NSUB
        + [pltpu.VMEM((_CS, _MW), _f32)] * _NSUB
        + [pltpu.VMEM_SHARED((_N, _MW), _f32), pltpu.SemaphoreType.DMA]
    )

    @functools.partial(
        pl.kernel,
        mesh=_MESH,
        out_type=jax.ShapeDtypeStruct((2, _N, _MW), _f32),
        scratch_types=scratch,
    )
    def k(m_h, dst_h, z_h, out_h, *scr):
        idx = scr[0:_NSUB]
        buf = scr[_NSUB:2 * _NSUB]
        table = scr[2 * _NSUB]
        sem = scr[2 * _NSUB + 1]
        cid = lax.axis_index("c")
        sid = lax.axis_index("s")
        row0 = pl.multiple_of(sid * 624, 8)

        @pl.when(sid < 15)
        def _():
            pltpu.sync_copy(z_h.at[pl.ds(row0, 624)],
                            table.at[pl.ds(row0, 624)])

        @pl.when(sid == 15)
        def _():
            pltpu.sync_copy(z_h.at[pl.ds(9360, 640)],
                            table.at[pl.ds(9360, 640)])

        plsc.subcore_barrier()
        base = cid * (_E // 2) + sid * _EPW

        def step(i, _):
            offs = [pl.multiple_of(base + i * _SSTEP + j * _CS, 8)
                    for j in range(_NSUB)]
            cps = []
            for j in range(_NSUB):
                cps.append(pltpu.async_copy(dst_h.at[pl.ds(offs[j], _CS)], idx[j], sem))
                cps.append(pltpu.async_copy(m_h.at[pl.ds(offs[j], _CS)], buf[j], sem))
            for cp in cps:
                cp.wait()
            cps = []
            for j in range(_NSUB):
                cps.append(pltpu.async_copy(buf[j], table.at[idx[j]], sem, add=True))
            for cp in cps:
                cp.wait()
            return 0

        lax.fori_loop(0, _EPW // _SSTEP, step, 0)
        plsc.subcore_barrier()

        @pl.when(sid < 15)
        def _():
            pltpu.sync_copy(table.at[pl.ds(row0, 624)],
                            out_h.at[cid, pl.ds(row0, 624)])

        @pl.when(sid == 15)
        def _():
            pltpu.sync_copy(table.at[pl.ds(9360, 640)],
                            out_h.at[cid, pl.ds(9360, 640)])

    return k(m, dst, zeros)


def _proj(feat, wcat, bcat):
    nb = 2000

    def body(f_ref, w_ref, b_ref, pd_ref, ps_ref):
        p = jnp.dot(f_ref[...], w_ref[...], preferred_element_type=_f32)
        p = p + b_ref[...]
        pd_ref[...] = p[:, :128]
        ps_ref[...] = p[:, 128:]

    return pl.pallas_call(
        body,
        grid=(_N // nb,),
        in_specs=[
            pl.BlockSpec((nb, 128), lambda i: (i, 0)),
            pl.BlockSpec((128, 256), lambda i: (0, 0)),
            pl.BlockSpec((1, 256), lambda i: (0, 0)),
        ],
        out_specs=[pl.BlockSpec((nb, 128), lambda i: (i, 0))] * 2,
        out_shape=[jax.ShapeDtypeStruct((_N, 128), _f32)] * 2,
    )(feat, wcat, bcat)


def _edge_call(gd, gs, rx, ry, dd, dt, wg, e2w, e2b, e3w, e3b, v2row, v2b):
    eb = 5000

    def body(gd_ref, gs_ref, rx_ref, ry_ref, dd_ref, dt_ref, wg_ref, e2w_ref,
             e2b_ref, e3w_ref, e3b_ref, v2_ref, v2b_ref, m_ref):
        gdv = gd_ref[...]
        gsv = gs_ref[...]
        dist = dd_ref[...]
        dot = dt_ref[...]
        wgv = wg_ref[...]
        th = gdv[:, :64] + gsv[:, :64] + dist * wgv[0:1] + dot * wgv[1:2]
        th = _silu(th)
        th = _silu(jnp.dot(th, e2w_ref[...], preferred_element_type=_f32)
                   + e2b_ref[...])
        mh = jnp.dot(th, e3w_ref[...], preferred_element_type=_f32) + e3b_ref[...]
        tv = gdv[:, 64:] + gsv[:, 64:] + dist * wgv[2:3] + dot * wgv[3:4]
        tv = _silu(tv)
        vw = jnp.sum(tv * v2_ref[...], axis=1, keepdims=True) + v2b_ref[...]
        mv = jnp.concatenate([vw * rx_ref[...], vw * ry_ref[...]], axis=1)
        m_ref[...] = jnp.concatenate(
            [mh, mv, jnp.zeros((eb, _MW - 66), _f32)], axis=1)

    return pl.pallas_call(
        body,
        grid=(_E // eb,),
        in_specs=[
            pl.BlockSpec((eb, 128), lambda i: (i, 0)),
            pl.BlockSpec((eb, 128), lambda i: (i, 0)),
            pl.BlockSpec((eb, 1), lambda i: (i, 0)),
            pl.BlockSpec((eb, 1), lambda i: (i, 0)),
            pl.BlockSpec((eb, 1), lambda i: (i, 0)),
            pl.BlockSpec((eb, 1), lambda i: (i, 0)),
            pl.BlockSpec((4, 64), lambda i: (0, 0)),
            pl.BlockSpec((64, 64), lambda i: (0, 0)),
            pl.BlockSpec((1, 64), lambda i: (0, 0)),
            pl.BlockSpec((64, 64), lambda i: (0, 0)),
            pl.BlockSpec((1, 64), lambda i: (0, 0)),
            pl.BlockSpec((1, 64), lambda i: (0, 0)),
            pl.BlockSpec((1, 1), lambda i: (0, 0)),
        ],
        out_specs=pl.BlockSpec((eb, _MW), lambda i: (i, 0)),
        out_shape=jax.ShapeDtypeStruct((_E, _MW), _f32),
    )(gd, gs, rx, ry, dd, dt, wg, e2w, e2b, e3w, e3b, v2row, v2b)


def _node_proj(feat, msum, w, wcat2, bcat2):
    nb = 2000

    def body(f_ref, ms_ref, wx_ref, wm_ref, wn_ref, h1b_ref, h2w_ref,
             h2b_ref, g_ref, b_ref, wc_ref, bc_ref, o_ref, pd_ref, ps_ref):
        f = f_ref[...]
        m = ms_ref[0] + ms_ref[1]
        mvx = m[:, 64:65]
        mvy = m[:, 65:66]
        mvn = jnp.sqrt(mvx * mvx + mvy * mvy + 1e-12)
        hh = (jnp.dot(f, wx_ref[...], preferred_element_type=_f32)
              + jnp.dot(m, wm_ref[...], preferred_element_type=_f32)
              + mvn * wn_ref[...] + h1b_ref[...])
        hh = _silu(hh)
        up = jnp.dot(hh, h2w_ref[...], preferred_element_type=_f32) + h2b_ref[...]
        y = jnp.maximum(f + up, 0.0)
        mu = jnp.mean(y, axis=1, keepdims=True)
        yc = y - mu
        var = jnp.mean(yc * yc, axis=1, keepdims=True)
        h = yc * jax.lax.rsqrt(var + 1e-5) * g_ref[...] + b_ref[...]
        o_ref[...] = h
        p = jnp.dot(h, wc_ref[...], preferred_element_type=_f32) + bc_ref[...]
        pd_ref[...] = p[:, :128]
        ps_ref[...] = p[:, 128:]

    return pl.pallas_call(
        body,
        grid=(_N // nb,),
        in_specs=[
            pl.BlockSpec((nb, 128), lambda i: (i, 0)),
            pl.BlockSpec((2, nb, _MW), lambda i: (0, i, 0)),
            pl.BlockSpec((128, 64), lambda i: (0, 0)),
            pl.BlockSpec((_MW, 64), lambda i: (0, 0)),
            pl.BlockSpec((1, 64), lambda i: (0, 0)),
            pl.BlockSpec((1, 64), lambda i: (0, 0)),
            pl.BlockSpec((64, 128), lambda i: (0, 0)),
            pl.BlockSpec((1, 128), lambda i: (0, 0)),
            pl.BlockSpec((1, 128), lambda i: (0, 0)),
            pl.BlockSpec((1, 128), lambda i: (0, 0)),
            pl.BlockSpec((128, 256), lambda i: (0, 0)),
            pl.BlockSpec((1, 256), lambda i: (0, 0)),
        ],
        out_specs=[
            pl.BlockSpec((nb, 128), lambda i: (i, 0)),
            pl.BlockSpec((nb, 128), lambda i: (i, 0)),
            pl.BlockSpec((nb, 128), lambda i: (i, 0)),
        ],
        out_shape=[jax.ShapeDtypeStruct((_N, 128), _f32)] * 3,
    )(feat, msum, w['wx'], w['wm'], w['wn'], w['h1b'], w['h2w'], w['h2b'],
      w['g'], w['b'], wcat2, bcat2)


def _node_pool(feat, msum, w, bcol, pos, poolw, poolb, o1w, o1b, o2w, o2b,
               gain):
    nb = 2000
    nsteps = _N // nb

    def body(f_ref, ms_ref, wx_ref, wm_ref, wn_ref, h1b_ref, h2w_ref,
             h2b_ref, g_ref, b_ref, bcol_ref, p_ref, pw_ref, pb_ref,
             o1w_ref, o1b_ref, o2w_ref, o2b_ref, gn_ref,
             s_ref, lat_ref, mu_ref, loss_ref, acc_ref):
        f = f_ref[...]
        m = ms_ref[0] + ms_ref[1]
        mvx = m[:, 64:65]
        mvy = m[:, 65:66]
        mvn = jnp.sqrt(mvx * mvx + mvy * mvy + 1e-12)
        hh = (jnp.dot(f, wx_ref[...], preferred_element_type=_f32)
              + jnp.dot(m, wm_ref[...], preferred_element_type=_f32)
              + mvn * wn_ref[...] + h1b_ref[...])
        hh = _silu(hh)
        up = jnp.dot(hh, h2w_ref[...], preferred_element_type=_f32) + h2b_ref[...]
        y = jnp.maximum(f + up, 0.0)
        mu_ = jnp.mean(y, axis=1, keepdims=True)
        yc = y - mu_
        var = jnp.mean(yc * yc, axis=1, keepdims=True)
        hv = yc * jax.lax.rsqrt(var + 1e-5) * g_ref[...] + b_ref[...]

        logits = jnp.dot(hv, pw_ref[...], preferred_element_type=_f32) + pb_ref[...]
        mx = jnp.max(logits, axis=1, keepdims=True)
        ex = jnp.exp(logits - mx)
        s = ex / jnp.sum(ex, axis=1, keepdims=True)
        s_ref[...] = s
        bc = bcol_ref[...]
        lane = lax.broadcasted_iota(jnp.int32, (nb, 256), 1) // _K
        stile = jnp.concatenate([s] * _B, axis=1)
        wm_ = jnp.where(lane == bc, stile, 0.0)
        entcol = jnp.sum(s * jnp.log(s + 1e-8), axis=1, keepdims=True)
        ones = jnp.ones((nb, 1), _f32)
        w_ext = jnp.concatenate(
            [wm_, s, ones, jnp.zeros((nb, 7), _f32)], axis=1)
        r_ext = jnp.concatenate(
            [hv, p_ref[...], ones, entcol, jnp.zeros((nb, 4), _f32)],
            axis=1)
        acc = lax.dot_general(w_ext, r_ext, (((0,), (0,)), ((), ())),
                              preferred_element_type=_f32)

        @pl.when(pl.program_id(0) == 0)
        def _():
            acc_ref[...] = acc

        @pl.when(pl.program_id(0) != 0)
        def _():
            acc_ref[...] += acc

        @pl.when(pl.program_id(0) == nsteps - 1)
        def _():
            a = acc_ref[...]
            den = a[:256, 130:131] + 1e-8
            pooled = a[:256, :128] / den
            z = jnp.maximum(
                jnp.dot(pooled, o1w_ref[...], preferred_element_type=_f32)
                + o1b_ref[...], 0.0)
            lat_ref[...] = (jnp.dot(z, o2w_ref[...],
                                    preferred_element_type=_f32)
                            + o2b_ref[...]) * gn_ref[...]
            mu_ref[...] = a[:256, 128:130] / den
            usage = a[256:288, 130:131] * (1.0 / _N)
            lb = jnp.sum(usage * jnp.log(usage * _K + 1e-8), axis=0,
                         keepdims=True)
            ent = -a[288:289, 131:132] * (1.0 / _N)
            loss_ref[...] = ent + lb

    return pl.pallas_call(
        body,
        grid=(nsteps,),
        in_specs=[
            pl.BlockSpec((nb, 128), lambda i: (i, 0)),
            pl.BlockSpec((2, nb, _MW), lambda i: (0, i, 0)),
            pl.BlockSpec((128, 64), lambda i: (0, 0)),
            pl.BlockSpec((_MW, 64), lambda i: (0, 0)),
            pl.BlockSpec((1, 64), lambda i: (0, 0)),
            pl.BlockSpec((1, 64), lambda i: (0, 0)),
            pl.BlockSpec((64, 128), lambda i: (0, 0)),
            pl.BlockSpec((1, 128), lambda i: (0, 0)),
            pl.BlockSpec((1, 128), lambda i: (0, 0)),
            pl.BlockSpec((1, 128), lambda i: (0, 0)),
            pl.BlockSpec((nb, 1), lambda i: (i, 0)),
            pl.BlockSpec((nb, 2), lambda i: (i, 0)),
            pl.BlockSpec((128, _K), lambda i: (0, 0)),
            pl.BlockSpec((1, _K), lambda i: (0, 0)),
            pl.BlockSpec((128, 128), lambda i: (0, 0)),
            pl.BlockSpec((1, 128), lambda i: (0, 0)),
            pl.BlockSpec((128, _LAT), lambda i: (0, 0)),
            pl.BlockSpec((1, _LAT), lambda i: (0, 0)),
            pl.BlockSpec((1, _LAT), lambda i: (0, 0)),
        ],
        out_specs=[
            pl.BlockSpec((nb, _K), lambda i: (i, 0)),
            pl.BlockSpec((256, _LAT), lambda i: (0, 0)),
            pl.BlockSpec((256, 2), lambda i: (0, 0)),
            pl.BlockSpec((1, 1), lambda i: (0, 0)),
        ],
        out_shape=[
            jax.ShapeDtypeStruct((_N, _K), _f32),
            jax.ShapeDtypeStruct((256, _LAT), _f32),
            jax.ShapeDtypeStruct((256, 2), _f32),
            jax.ShapeDtypeStruct((1, 1), _f32),
        ],
        scratch_shapes=[pltpu.VMEM((296, 136), _f32)],
    )(feat, msum, w['wx'], w['wm'], w['wn'], w['h1b'], w['h2w'], w['h2b'],
      w['g'], w['b'], bcol, pos, poolw, poolb, o1w, o1b, o2w, o2b, gain)


def _layer_weights(p):
    e1w, e1b = p['e1']
    v1w, v1b = p['v1']
    wcat = jnp.concatenate(
        [e1w[:_F], v1w[:_F], e1w[_F:2 * _F], v1w[_F:2 * _F]], axis=1)
    bcat = jnp.concatenate(
        [e1b, v1b, jnp.zeros((2 * _HID,), _f32)]).reshape(1, 256)
    wg = jnp.concatenate([e1w[2 * _F:], v1w[2 * _F:]], axis=0)
    h1w, h1b = p['h1']
    wx = h1w[:_F]
    wm = jnp.concatenate([h1w[_F:_F + 64], jnp.zeros((_MW - 64, 64), _f32)],
                         axis=0)
    wn = h1w[_F + 64].reshape(1, 64)
    return dict(
        wcat=wcat, bcat=bcat, wg=wg,
        e2w=p['e2'][0], e2b=p['e2'][1].reshape(1, 64),
        e3w=p['e3'][0], e3b=p['e3'][1].reshape(1, 64),
        v2row=p['v2'][0].reshape(1, 64), v2b=p['v2'][1].reshape(1, 1),
        wx=wx, wm=wm, wn=wn, h1b=h1b.reshape(1, 64),
        h2w=p['h2'][0], h2b=p['h2'][1].reshape(1, 128),
    )


def kernel(x, edge_index, batch, p1, p2, ln1, ln2, pool, out1, out2,
           latent_gain):
    src = edge_index[0]
    dst = edge_index[1]
    pos = x[:, :2]
    zeros_tab = jnp.zeros((_N, _MW), _f32)
    bcol = batch.reshape(_N, 1)

    w1 = _layer_weights(p1)
    w1['g'] = ln1[0].reshape(1, 128)
    w1['b'] = ln1[1].reshape(1, 128)
    w2 = _layer_weights(p2)
    w2['g'] = ln2[0].reshape(1, 128)
    w2['b'] = ln2[1].reshape(1, 128)

    rx, ry, dd, dt = _geo(x[:, 0], x[:, 1], x[:, 2], x[:, 3], src, dst)
    pd, ps = _proj(x, w1['wcat'], w1['bcat'])
    gd, gs = _gather(pd, ps, src, dst)
    rx = rx.reshape(_E, 1)
    ry = ry.reshape(_E, 1)
    dd = dd.reshape(_E, 1)
    dt = dt.reshape(_E, 1)
    m1 = _edge_call(gd, gs, rx, ry, dd, dt, w1['wg'], w1['e2w'], w1['e2b'],
                    w1['e3w'], w1['e3b'], w1['v2row'], w1['v2b'])
    msum1 = _scatter(m1, dst, zeros_tab)
    h1, pd2, ps2 = _node_proj(x, msum1, w1, w2['wcat'], w2['bcat'])

    gd2, gs2 = _gather(pd2, ps2, src, dst)
    m2 = _edge_call(gd2, gs2, rx, ry, dd, dt, w2['wg'], w2['e2w'], w2['e2b'],
                    w2['e3w'], w2['e3b'], w2['v2row'], w2['v2b'])
    msum2 = _scatter(m2, dst, zeros_tab)
    s, lat, mu, loss = _node_pool(
        h1, msum2, w2, bcol, pos, pool[0], pool[1].reshape(1, _K), out1[0],
        out1[1].reshape(1, 128), out2[0], out2[1].reshape(1, _LAT),
        latent_gain.reshape(1, _LAT))
    return (lat.reshape(_B, _K, _LAT), s, loss[0, 0],
            mu.reshape(_B, _K, 2))


# speedup vs baseline: 1.4810x; 1.3753x over previous
import functools

import jax
import jax.numpy as jnp
from jax import lax
from jax.experimental import pallas as pl
from jax.experimental.pallas import tpu as pltpu
from jax.experimental.pallas import tpu_sc as plsc

_N = 10000
_E = 320000
_F = 128
_HID = 64
_OUT = 128
_K = 32
_LAT = 64
_B = 8

_NW = 32
_EPW = _E // _NW
_C = 80
_NSUB = 5
_STEP = _C * _NSUB
_NITER = _EPW // _STEP
_ROWS = _N // 16
_MW = 128

_f32 = jnp.float32

_MESH = plsc.VectorSubcoreMesh(core_axis_name="c", subcore_axis_name="s")


def _silu(x):
    return x / (1.0 + jnp.exp(-x))


def _geo(px, py, vx, vy, src, dst):
    scratch = (
        [pltpu.VMEM((_C,), jnp.int32)] * (2 * _NSUB)
        + [pltpu.VMEM((_C,), _f32)] * (8 * _NSUB)
        + [pltpu.VMEM((_C,), _f32)] * (4 * _NSUB)
        + [pltpu.SemaphoreType.DMA]
    )

    @functools.partial(
        pl.kernel,
        mesh=_MESH,
        out_type=[jax.ShapeDtypeStruct((_E,), _f32)] * 4,
        scratch_types=scratch,
    )
    def k(px_h, py_h, vx_h, vy_h, src_h, dst_h, rx_o, ry_o, dd_o, dt_o, *scr):
        idxd = scr[0:_NSUB]
        idxs = scr[_NSUB:2 * _NSUB]
        gb = scr[2 * _NSUB:10 * _NSUB]
        ob = scr[10 * _NSUB:14 * _NSUB]
        sem = scr[14 * _NSUB]
        wid = lax.axis_index("s") * 2 + lax.axis_index("c")
        base = wid * _EPW
        tabs = (px_h, py_h, vx_h, vy_h)
        gouts = (rx_o, ry_o, dd_o, dt_o)

        def step(i, _):
            offs = [pl.multiple_of(base + i * _STEP + j * _C, 8)
                    for j in range(_NSUB)]
            cps = []
            for j in range(_NSUB):
                cps.append(pltpu.async_copy(dst_h.at[pl.ds(offs[j], _C)], idxd[j], sem))
                cps.append(pltpu.async_copy(src_h.at[pl.ds(offs[j], _C)], idxs[j], sem))
            for cp in cps:
                cp.wait()
            cps = []
            for j in range(_NSUB):
                for t in range(4):
                    cps.append(pltpu.async_copy(
                        tabs[t].at[idxd[j]], gb[8 * j + t], sem))
                    cps.append(pltpu.async_copy(
                        tabs[t].at[idxs[j]], gb[8 * j + 4 + t], sem))
            for cp in cps:
                cp.wait()
            for j in range(_NSUB):
                for g in range(_C // 16):
                    sl = pl.ds(pl.multiple_of(g * 16, 8), 16)
                    rx = gb[8 * j + 4][sl] - gb[8 * j + 0][sl]
                    ry = gb[8 * j + 5][sl] - gb[8 * j + 1][sl]
                    wx = gb[8 * j + 6][sl] - gb[8 * j + 2][sl]
                    wy = gb[8 * j + 7][sl] - gb[8 * j + 3][sl]
                    ob[4 * j + 0][sl] = rx
                    ob[4 * j + 1][sl] = ry
                    ob[4 * j + 2][sl] = rx * rx + ry * ry
                    ob[4 * j + 3][sl] = wx * rx + wy * ry
            cps = []
            for j in range(_NSUB):
                for t in range(4):
                    cps.append(pltpu.async_copy(
                        ob[4 * j + t], gouts[t].at[pl.ds(offs[j], _C)], sem))
            for cp in cps:
                cp.wait()
            return 0

        lax.fori_loop(0, _NITER, step, 0)

    return k(px, py, vx, vy, src, dst)


def _gather(pd, ps, src, dst, geo_tabs=None):
    with_geo = geo_tabs is not None
    scratch = (
        [pltpu.VMEM((_C,), jnp.int32)] * (2 * _NSUB)
        + [pltpu.VMEM((_C, 128), _f32)] * (2 * _NSUB)
        + ([pltpu.VMEM((_C,), _f32)] * (12 * _NSUB) if with_geo else [])
        + [pltpu.SemaphoreType.DMA]
    )
    out_type = [
        jax.ShapeDtypeStruct((_E, 128), _f32),
        jax.ShapeDtypeStruct((_E, 128), _f32),
    ] + ([jax.ShapeDtypeStruct((_E,), _f32)] * 4 if with_geo else [])

    @functools.partial(
        pl.kernel, mesh=_MESH, out_type=out_type, scratch_types=scratch,
    )
    def k(*args):
        n_in = 8 if with_geo else 4
        n_out = 6 if with_geo else 2
        ins = args[:n_in]
        outs = args[n_in:n_in + n_out]
        scr = args[n_in + n_out:]
        if with_geo:
            pd_h, ps_h, src_h, dst_h, px_h, py_h, vx_h, vy_h = ins
            gd_o, gs_o, rx_o, ry_o, dd_o, dt_o = outs
            tabs = (px_h, py_h, vx_h, vy_h)
            gouts = (rx_o, ry_o, dd_o, dt_o)
        else:
            pd_h, ps_h, src_h, dst_h = ins
            gd_o, gs_o = outs
        idxd = scr[0:_NSUB]
        idxs = scr[_NSUB:2 * _NSUB]
        bufd = scr[2 * _NSUB:3 * _NSUB]
        bufs = scr[3 * _NSUB:4 * _NSUB]
        if with_geo:
            gb = scr[4 * _NSUB:12 * _NSUB]
            ob = scr[12 * _NSUB:16 * _NSUB]
            sem = scr[16 * _NSUB]
        else:
            sem = scr[4 * _NSUB]
        wid = lax.axis_index("s") * 2 + lax.axis_index("c")
        base = wid * _EPW

        def step(i, _):
            offs = [pl.multiple_of(base + i * _STEP + j * _C, 8)
                    for j in range(_NSUB)]
            cps = []
            for j in range(_NSUB):
                cps.append(pltpu.async_copy(dst_h.at[pl.ds(offs[j], _C)], idxd[j], sem))
                cps.append(pltpu.async_copy(src_h.at[pl.ds(offs[j], _C)], idxs[j], sem))
            for cp in cps:
                cp.wait()
            cps = []
            for j in range(_NSUB):
                cps.append(pltpu.async_copy(pd_h.at[idxd[j]], bufd[j], sem))
                cps.append(pltpu.async_copy(ps_h.at[idxs[j]], bufs[j], sem))
                if with_geo:
                    for t in range(4):
                        cps.append(pltpu.async_copy(
                            tabs[t].at[idxd[j]], gb[8 * j + t], sem))
                        cps.append(pltpu.async_copy(
                            tabs[t].at[idxs[j]], gb[8 * j + 4 + t], sem))
            for cp in cps:
                cp.wait()
            if with_geo:
                for j in range(_NSUB):
                    for g in range(_C // 16):
                        sl = pl.ds(pl.multiple_of(g * 16, 8), 16)
                        rx = gb[8 * j + 4][sl] - gb[8 * j + 0][sl]
                        ry = gb[8 * j + 5][sl] - gb[8 * j + 1][sl]
                        wx = gb[8 * j + 6][sl] - gb[8 * j + 2][sl]
                        wy = gb[8 * j + 7][sl] - gb[8 * j + 3][sl]
                        ob[4 * j + 0][sl] = rx
                        ob[4 * j + 1][sl] = ry
                        ob[4 * j + 2][sl] = rx * rx + ry * ry
                        ob[4 * j + 3][sl] = wx * rx + wy * ry
            cps = []
            for j in range(_NSUB):
                cps.append(pltpu.async_copy(bufd[j], gd_o.at[pl.ds(offs[j], _C)], sem))
                cps.append(pltpu.async_copy(bufs[j], gs_o.at[pl.ds(offs[j], _C)], sem))
                if with_geo:
                    for t in range(4):
                        cps.append(pltpu.async_copy(
                            ob[4 * j + t], gouts[t].at[pl.ds(offs[j], _C)], sem))
            for cp in cps:
                cp.wait()
            return 0

        lax.fori_loop(0, _NITER, step, 0)

    if with_geo:
        return k(pd, ps, src, dst, *geo_tabs)
    return k(pd, ps, src, dst)


_CS = 40
_SSTEP = _CS * _NSUB


def _scatter(m, dst, zeros):
    scratch = (
        [pltpu.VMEM((_CS,), jnp.int32)] * _NSUB
        + [pltpu.VMEM((_CS, _MW), _f32)] * _NSUB
        + [pltpu.VMEM_SHARED((_N, _MW), _f32), pltpu.SemaphoreType.DMA]
    )

    @functools.partial(
        pl.kernel,
        mesh=_MESH,
        out_type=jax.ShapeDtypeStruct((2, _N, _MW), _f32),
        scratch_types=scratch,
    )
    def k(m_h, dst_h, z_h, out_h, *scr):
        idx = scr[0:_NSUB]
        buf = scr[_NSUB:2 * _NSUB]
        table = scr[2 * _NSUB]
        sem = scr[2 * _NSUB + 1]
        cid = lax.axis_index("c")
        sid = lax.axis_index("s")
        row0 = pl.multiple_of(sid * 624, 8)

        @pl.when(sid < 15)
        def _():
            pltpu.sync_copy(z_h.at[pl.ds(row0, 624)],
                            table.at[pl.ds(row0, 624)])

        @pl.when(sid == 15)
        def _():
            pltpu.sync_copy(z_h.at[pl.ds(9360, 640)],
                            table.at[pl.ds(9360, 640)])

        plsc.subcore_barrier()
        base = cid * (_E // 2) + sid * _EPW

        def step(i, _):
            offs = [pl.multiple_of(base + i * _SSTEP + j * _CS, 8)
                    for j in range(_NSUB)]
            cps = []
            for j in range(_NSUB):
                cps.append(pltpu.async_copy(dst_h.at[pl.ds(offs[j], _CS)], idx[j], sem))
                cps.append(pltpu.async_copy(m_h.at[pl.ds(offs[j], _CS)], buf[j], sem))
            for cp in cps:
                cp.wait()
            cps = []
            for j in range(_NSUB):
                cps.append(pltpu.async_copy(buf[j], table.at[idx[j]], sem, add=True))
            for cp in cps:
                cp.wait()
            return 0

        lax.fori_loop(0, _EPW // _SSTEP, step, 0)
        plsc.subcore_barrier()

        @pl.when(sid < 15)
        def _():
            pltpu.sync_copy(table.at[pl.ds(row0, 624)],
                            out_h.at[cid, pl.ds(row0, 624)])

        @pl.when(sid == 15)
        def _():
            pltpu.sync_copy(table.at[pl.ds(9360, 640)],
                            out_h.at[cid, pl.ds(9360, 640)])

    return k(m, dst, zeros)


def _proj(feat, wcat, bcat):
    nb = 2000

    def body(f_ref, w_ref, b_ref, pd_ref, ps_ref):
        p = jnp.dot(f_ref[...], w_ref[...], preferred_element_type=_f32)
        p = p + b_ref[...]
        pd_ref[...] = p[:, :128]
        ps_ref[...] = p[:, 128:]

    return pl.pallas_call(
        body,
        grid=(_N // nb,),
        in_specs=[
            pl.BlockSpec((nb, 128), lambda i: (i, 0)),
            pl.BlockSpec((128, 256), lambda i: (0, 0)),
            pl.BlockSpec((1, 256), lambda i: (0, 0)),
        ],
        out_specs=[pl.BlockSpec((nb, 128), lambda i: (i, 0))] * 2,
        out_shape=[jax.ShapeDtypeStruct((_N, 128), _f32)] * 2,
    )(feat, wcat, bcat)


def _edge_call(gd, gs, geo, wg, e2w, e2b, e3w, e3b, v2col, v2b):
    eb = 5000

    def body(gd_ref, gs_ref, g_ref, wg_ref, e2w_ref, e2b_ref, e3w_ref,
             e3b_ref, v2_ref, v2b_ref, m_ref):
        g = g_ref[...]
        su = (gd_ref[...] + gs_ref[...]
              + jnp.dot(g[:, 2:4], wg_ref[...], preferred_element_type=_f32))
        th = _silu(su[:, :64])
        th = _silu(jnp.dot(th, e2w_ref[...], preferred_element_type=_f32)
                   + e2b_ref[...])
        mh = jnp.dot(th, e3w_ref[...], preferred_element_type=_f32) + e3b_ref[...]
        tv = _silu(su[:, 64:])
        vw = jnp.dot(tv, v2_ref[...], preferred_element_type=_f32) + v2b_ref[...]
        mv = vw * g[:, 0:2]
        m_ref[...] = jnp.concatenate(
            [mh, mv, jnp.zeros((eb, _MW - 66), _f32)], axis=1)

    return pl.pallas_call(
        body,
        grid=(_E // eb,),
        in_specs=[
            pl.BlockSpec((eb, 128), lambda i: (i, 0)),
            pl.BlockSpec((eb, 128), lambda i: (i, 0)),
            pl.BlockSpec((eb, 4), lambda i: (i, 0)),
            pl.BlockSpec((2, 128), lambda i: (0, 0)),
            pl.BlockSpec((64, 64), lambda i: (0, 0)),
            pl.BlockSpec((1, 64), lambda i: (0, 0)),
            pl.BlockSpec((64, 64), lambda i: (0, 0)),
            pl.BlockSpec((1, 64), lambda i: (0, 0)),
            pl.BlockSpec((64, 1), lambda i: (0, 0)),
            pl.BlockSpec((1, 1), lambda i: (0, 0)),
        ],
        out_specs=pl.BlockSpec((eb, _MW), lambda i: (i, 0)),
        out_shape=jax.ShapeDtypeStruct((_E, _MW), _f32),
    )(gd, gs, geo, wg, e2w, e2b, e3w, e3b, v2col, v2b)


def _node_proj(feat, msum, w, wcat2, bcat2):
    nb = 2000

    def body(f_ref, ms_ref, wx_ref, wm_ref, wn_ref, h1b_ref, h2w_ref,
             h2b_ref, g_ref, b_ref, wc_ref, bc_ref, o_ref, pd_ref, ps_ref):
        f = f_ref[...]
        m = ms_ref[0] + ms_ref[1]
        mvx = m[:, 64:65]
        mvy = m[:, 65:66]
        mvn = jnp.sqrt(mvx * mvx + mvy * mvy + 1e-12)
        hh = (jnp.dot(f, wx_ref[...], preferred_element_type=_f32)
              + jnp.dot(m, wm_ref[...], preferred_element_type=_f32)
              + mvn * wn_ref[...] + h1b_ref[...])
        hh = _silu(hh)
        up = jnp.dot(hh, h2w_ref[...], preferred_element_type=_f32) + h2b_ref[...]
        y = jnp.maximum(f + up, 0.0)
        mu = jnp.mean(y, axis=1, keepdims=True)
        yc = y - mu
        var = jnp.mean(yc * yc, axis=1, keepdims=True)
        h = yc * jax.lax.rsqrt(var + 1e-5) * g_ref[...] + b_ref[...]
        o_ref[...] = h
        p = jnp.dot(h, wc_ref[...], preferred_element_type=_f32) + bc_ref[...]
        pd_ref[...] = p[:, :128]
        ps_ref[...] = p[:, 128:]

    return pl.pallas_call(
        body,
        grid=(_N // nb,),
        in_specs=[
            pl.BlockSpec((nb, 128), lambda i: (i, 0)),
            pl.BlockSpec((2, nb, _MW), lambda i: (0, i, 0)),
            pl.BlockSpec((128, 64), lambda i: (0, 0)),
            pl.BlockSpec((_MW, 64), lambda i: (0, 0)),
            pl.BlockSpec((1, 64), lambda i: (0, 0)),
            pl.BlockSpec((1, 64), lambda i: (0, 0)),
            pl.BlockSpec((64, 128), lambda i: (0, 0)),
            pl.BlockSpec((1, 128), lambda i: (0, 0)),
            pl.BlockSpec((1, 128), lambda i: (0, 0)),
            pl.BlockSpec((1, 128), lambda i: (0, 0)),
            pl.BlockSpec((128, 256), lambda i: (0, 0)),
            pl.BlockSpec((1, 256), lambda i: (0, 0)),
        ],
        out_specs=[
            pl.BlockSpec((nb, 128), lambda i: (i, 0)),
            pl.BlockSpec((nb, 128), lambda i: (i, 0)),
            pl.BlockSpec((nb, 128), lambda i: (i, 0)),
        ],
        out_shape=[jax.ShapeDtypeStruct((_N, 128), _f32)] * 3,
    )(feat, msum, w['wx'], w['wm'], w['wn'], w['h1b'], w['h2w'], w['h2b'],
      w['g'], w['b'], wcat2, bcat2)


def _node_pool(feat, msum, w, bcol, pos, poolw, poolb, o1w, o1b, o2w, o2b,
               gain):
    nb = 2000
    nsteps = _N // nb

    def body(f_ref, ms_ref, wx_ref, wm_ref, wn_ref, h1b_ref, h2w_ref,
             h2b_ref, g_ref, b_ref, bcol_ref, p_ref, pw_ref, pb_ref,
             o1w_ref, o1b_ref, o2w_ref, o2b_ref, gn_ref,
             s_ref, lat_ref, mu_ref, loss_ref, acc_ref):
        f = f_ref[...]
        m = ms_ref[0] + ms_ref[1]
        mvx = m[:, 64:65]
        mvy = m[:, 65:66]
        mvn = jnp.sqrt(mvx * mvx + mvy * mvy + 1e-12)
        hh = (jnp.dot(f, wx_ref[...], preferred_element_type=_f32)
              + jnp.dot(m, wm_ref[...], preferred_element_type=_f32)
              + mvn * wn_ref[...] + h1b_ref[...])
        hh = _silu(hh)
        up = jnp.dot(hh, h2w_ref[...], preferred_element_type=_f32) + h2b_ref[...]
        y = jnp.maximum(f + up, 0.0)
        mu_ = jnp.mean(y, axis=1, keepdims=True)
        yc = y - mu_
        var = jnp.mean(yc * yc, axis=1, keepdims=True)
        hv = yc * jax.lax.rsqrt(var + 1e-5) * g_ref[...] + b_ref[...]

        logits = jnp.dot(hv, pw_ref[...], preferred_element_type=_f32) + pb_ref[...]
        mx = jnp.max(logits, axis=1, keepdims=True)
        ex = jnp.exp(logits - mx)
        s = ex / jnp.sum(ex, axis=1, keepdims=True)
        s_ref[...] = s
        bc = bcol_ref[...]
        lane = lax.broadcasted_iota(jnp.int32, (nb, 256), 1) // _K
        stile = jnp.concatenate([s] * _B, axis=1)
        wm_ = jnp.where(lane == bc, stile, 0.0)
        entcol = jnp.sum(s * jnp.log(s + 1e-8), axis=1, keepdims=True)
        ones = jnp.ones((nb, 1), _f32)
        w_ext = jnp.concatenate(
            [wm_, s, ones, jnp.zeros((nb, 7), _f32)], axis=1)
        r_ext = jnp.concatenate(
            [hv, p_ref[...], ones, entcol, jnp.zeros((nb, 4), _f32)],
            axis=1)
        acc = lax.dot_general(w_ext, r_ext, (((0,), (0,)), ((), ())),
                              preferred_element_type=_f32)

        @pl.when(pl.program_id(0) == 0)
        def _():
            acc_ref[...] = acc

        @pl.when(pl.program_id(0) != 0)
        def _():
            acc_ref[...] += acc

        @pl.when(pl.program_id(0) == nsteps - 1)
        def _():
            a = acc_ref[...]
            den = a[:256, 130:131] + 1e-8
            pooled = a[:256, :128] / den
            z = jnp.maximum(
                jnp.dot(pooled, o1w_ref[...], preferred_element_type=_f32)
                + o1b_ref[...], 0.0)
            lat_ref[...] = (jnp.dot(z, o2w_ref[...],
                                    preferred_element_type=_f32)
                            + o2b_ref[...]) * gn_ref[...]
            mu_ref[...] = a[:256, 128:130] / den
            usage = a[256:288, 130:131] * (1.0 / _N)
            lb = jnp.sum(usage * jnp.log(usage * _K + 1e-8), axis=0,
                         keepdims=True)
            ent = -a[288:289, 131:132] * (1.0 / _N)
            loss_ref[...] = ent + lb

    return pl.pallas_call(
        body,
        grid=(nsteps,),
        in_specs=[
            pl.BlockSpec((nb, 128), lambda i: (i, 0)),
            pl.BlockSpec((2, nb, _MW), lambda i: (0, i, 0)),
            pl.BlockSpec((128, 64), lambda i: (0, 0)),
            pl.BlockSpec((_MW, 64), lambda i: (0, 0)),
            pl.BlockSpec((1, 64), lambda i: (0, 0)),
            pl.BlockSpec((1, 64), lambda i: (0, 0)),
            pl.BlockSpec((64, 128), lambda i: (0, 0)),
            pl.BlockSpec((1, 128), lambda i: (0, 0)),
            pl.BlockSpec((1, 128), lambda i: (0, 0)),
            pl.BlockSpec((1, 128), lambda i: (0, 0)),
            pl.BlockSpec((nb, 1), lambda i: (i, 0)),
            pl.BlockSpec((nb, 2), lambda i: (i, 0)),
            pl.BlockSpec((128, _K), lambda i: (0, 0)),
            pl.BlockSpec((1, _K), lambda i: (0, 0)),
            pl.BlockSpec((128, 128), lambda i: (0, 0)),
            pl.BlockSpec((1, 128), lambda i: (0, 0)),
            pl.BlockSpec((128, _LAT), lambda i: (0, 0)),
            pl.BlockSpec((1, _LAT), lambda i: (0, 0)),
            pl.BlockSpec((1, _LAT), lambda i: (0, 0)),
        ],
        out_specs=[
            pl.BlockSpec((nb, _K), lambda i: (i, 0)),
            pl.BlockSpec((256, _LAT), lambda i: (0, 0)),
            pl.BlockSpec((256, 2), lambda i: (0, 0)),
            pl.BlockSpec((1, 1), lambda i: (0, 0)),
        ],
        out_shape=[
            jax.ShapeDtypeStruct((_N, _K), _f32),
            jax.ShapeDtypeStruct((256, _LAT), _f32),
            jax.ShapeDtypeStruct((256, 2), _f32),
            jax.ShapeDtypeStruct((1, 1), _f32),
        ],
        scratch_shapes=[pltpu.VMEM((296, 136), _f32)],
    )(feat, msum, w['wx'], w['wm'], w['wn'], w['h1b'], w['h2w'], w['h2b'],
      w['g'], w['b'], bcol, pos, poolw, poolb, o1w, o1b, o2w, o2b, gain)


def _layer_weights(p):
    e1w, e1b = p['e1']
    v1w, v1b = p['v1']
    wcat = jnp.concatenate(
        [e1w[:_F], v1w[:_F], e1w[_F:2 * _F], v1w[_F:2 * _F]], axis=1)
    bcat = jnp.concatenate(
        [e1b, v1b, jnp.zeros((2 * _HID,), _f32)]).reshape(1, 256)
    wg = jnp.concatenate([e1w[2 * _F:], v1w[2 * _F:]], axis=1)
    h1w, h1b = p['h1']
    wx = h1w[:_F]
    wm = jnp.concatenate([h1w[_F:_F + 64], jnp.zeros((_MW - 64, 64), _f32)],
                         axis=0)
    wn = h1w[_F + 64].reshape(1, 64)
    return dict(
        wcat=wcat, bcat=bcat, wg=wg,
        e2w=p['e2'][0], e2b=p['e2'][1].reshape(1, 64),
        e3w=p['e3'][0], e3b=p['e3'][1].reshape(1, 64),
        v2col=p['v2'][0], v2b=p['v2'][1].reshape(1, 1),
        wx=wx, wm=wm, wn=wn, h1b=h1b.reshape(1, 64),
        h2w=p['h2'][0], h2b=p['h2'][1].reshape(1, 128),
    )


def kernel(x, edge_index, batch, p1, p2, ln1, ln2, pool, out1, out2,
           latent_gain):
    src = edge_index[0]
    dst = edge_index[1]
    pos = x[:, :2]
    zeros_tab = jnp.zeros((_N, _MW), _f32)
    bcol = batch.reshape(_N, 1)

    w1 = _layer_weights(p1)
    w1['g'] = ln1[0].reshape(1, 128)
    w1['b'] = ln1[1].reshape(1, 128)
    w2 = _layer_weights(p2)
    w2['g'] = ln2[0].reshape(1, 128)
    w2['b'] = ln2[1].reshape(1, 128)

    rx, ry, dd, dt = _geo(x[:, 0], x[:, 1], x[:, 2], x[:, 3], src, dst)
    pd, ps = _proj(x, w1['wcat'], w1['bcat'])
    gd, gs = _gather(pd, ps, src, dst)
    geo4 = jnp.concatenate(
        [rx.reshape(_E, 1), ry.reshape(_E, 1), dd.reshape(_E, 1),
         dt.reshape(_E, 1)], axis=1)
    m1 = _edge_call(gd, gs, geo4, w1['wg'], w1['e2w'], w1['e2b'],
                    w1['e3w'], w1['e3b'], w1['v2col'], w1['v2b'])
    msum1 = _scatter(m1, dst, zeros_tab)
    h1, pd2, ps2 = _node_proj(x, msum1, w1, w2['wcat'], w2['bcat'])

    gd2, gs2 = _gather(pd2, ps2, src, dst)
    m2 = _edge_call(gd2, gs2, geo4, w2['wg'], w2['e2w'], w2['e2b'],
                    w2['e3w'], w2['e3b'], w2['v2col'], w2['v2b'])
    msum2 = _scatter(m2, dst, zeros_tab)
    s, lat, mu, loss = _node_pool(
        h1, msum2, w2, bcol, pos, pool[0], pool[1].reshape(1, _K), out1[0],
        out1[1].reshape(1, 128), out2[0], out2[1].reshape(1, _LAT),
        latent_gain.reshape(1, _LAT))
    return (lat.reshape(_B, _K, _LAT), s, loss[0, 0],
            mu.reshape(_B, _K, 2))


# speedup vs baseline: 1.5016x; 1.0139x over previous
import functools

import jax
import jax.numpy as jnp
from jax import lax
from jax.experimental import pallas as pl
from jax.experimental.pallas import tpu as pltpu
from jax.experimental.pallas import tpu_sc as plsc

_N = 10000
_E = 320000
_F = 128
_HID = 64
_OUT = 128
_K = 32
_LAT = 64
_B = 8

_NW = 32
_EPW = _E // _NW
_C = 80
_NSUB = 5
_STEP = _C * _NSUB
_NITER = _EPW // _STEP
_ROWS = _N // 16
_MW = 128

_f32 = jnp.float32

_MESH = plsc.VectorSubcoreMesh(core_axis_name="c", subcore_axis_name="s")


def _silu(x):
    return x / (1.0 + jnp.exp(-x))


def _geo(px, py, vx, vy, src, dst):
    scratch = (
        [pltpu.VMEM((_C,), jnp.int32)] * (2 * _NSUB)
        + [pltpu.VMEM((_C,), _f32)] * (8 * _NSUB)
        + [pltpu.VMEM((_C,), _f32)] * (4 * _NSUB)
        + [pltpu.SemaphoreType.DMA]
    )

    @functools.partial(
        pl.kernel,
        mesh=_MESH,
        out_type=[jax.ShapeDtypeStruct((_E,), _f32)] * 4,
        scratch_types=scratch,
    )
    def k(px_h, py_h, vx_h, vy_h, src_h, dst_h, rx_o, ry_o, dd_o, dt_o, *scr):
        idxd = scr[0:_NSUB]
        idxs = scr[_NSUB:2 * _NSUB]
        gb = scr[2 * _NSUB:10 * _NSUB]
        ob = scr[10 * _NSUB:14 * _NSUB]
        sem = scr[14 * _NSUB]
        wid = lax.axis_index("s") * 2 + lax.axis_index("c")
        base = wid * _EPW
        tabs = (px_h, py_h, vx_h, vy_h)
        gouts = (rx_o, ry_o, dd_o, dt_o)

        def step(i, _):
            offs = [pl.multiple_of(base + i * _STEP + j * _C, 8)
                    for j in range(_NSUB)]
            cps = []
            for j in range(_NSUB):
                cps.append(pltpu.async_copy(dst_h.at[pl.ds(offs[j], _C)], idxd[j], sem))
                cps.append(pltpu.async_copy(src_h.at[pl.ds(offs[j], _C)], idxs[j], sem))
            for cp in cps:
                cp.wait()
            cps = []
            for j in range(_NSUB):
                for t in range(4):
                    cps.append(pltpu.async_copy(
                        tabs[t].at[idxd[j]], gb[8 * j + t], sem))
                    cps.append(pltpu.async_copy(
                        tabs[t].at[idxs[j]], gb[8 * j + 4 + t], sem))
            for cp in cps:
                cp.wait()
            for j in range(_NSUB):
                for g in range(_C // 16):
                    sl = pl.ds(pl.multiple_of(g * 16, 8), 16)
                    rx = gb[8 * j + 4][sl] - gb[8 * j + 0][sl]
                    ry = gb[8 * j + 5][sl] - gb[8 * j + 1][sl]
                    wx = gb[8 * j + 6][sl] - gb[8 * j + 2][sl]
                    wy = gb[8 * j + 7][sl] - gb[8 * j + 3][sl]
                    ob[4 * j + 0][sl] = rx
                    ob[4 * j + 1][sl] = ry
                    ob[4 * j + 2][sl] = rx * rx + ry * ry
                    ob[4 * j + 3][sl] = wx * rx + wy * ry
            cps = []
            for j in range(_NSUB):
                for t in range(4):
                    cps.append(pltpu.async_copy(
                        ob[4 * j + t], gouts[t].at[pl.ds(offs[j], _C)], sem))
            for cp in cps:
                cp.wait()
            return 0

        lax.fori_loop(0, _NITER, step, 0)

    return k(px, py, vx, vy, src, dst)


def _gather(pd, ps, src, dst, geo_tabs=None):
    del geo_tabs
    cg = 40
    nst = _EPW // (cg * _NSUB)
    scratch = (
        [pltpu.VMEM((cg,), jnp.int32)] * (4 * _NSUB)
        + [pltpu.VMEM((cg, 128), _f32)] * (4 * _NSUB)
        + [pltpu.SemaphoreType.DMA] * 6
    )

    @functools.partial(
        pl.kernel,
        mesh=_MESH,
        out_type=[
            jax.ShapeDtypeStruct((_E, 128), _f32),
            jax.ShapeDtypeStruct((_E, 128), _f32),
        ],
        scratch_types=scratch,
    )
    def k(pd_h, ps_h, src_h, dst_h, gd_o, gs_o, *scr):
        idxd = [scr[0:_NSUB], scr[_NSUB:2 * _NSUB]]
        idxs = [scr[2 * _NSUB:3 * _NSUB], scr[3 * _NSUB:4 * _NSUB]]
        bufd = [scr[4 * _NSUB:5 * _NSUB], scr[5 * _NSUB:6 * _NSUB]]
        bufs = [scr[6 * _NSUB:7 * _NSUB], scr[7 * _NSUB:8 * _NSUB]]
        semi = [scr[8 * _NSUB], scr[8 * _NSUB + 1]]
        semg = [scr[8 * _NSUB + 2], scr[8 * _NSUB + 3]]
        semo = [scr[8 * _NSUB + 4], scr[8 * _NSUB + 5]]
        wid = lax.axis_index("s") * 2 + lax.axis_index("c")
        base = wid * _EPW

        def offs_of(k_, j):
            return pl.multiple_of(
                base + lax.rem(k_ * (cg * _NSUB), _EPW) + j * cg, 8)

        def fire_idx(p, k_):
            for j in range(_NSUB):
                o = offs_of(k_, j)
                pltpu.async_copy(dst_h.at[pl.ds(o, cg)], idxd[p][j], semi[p])
                pltpu.async_copy(src_h.at[pl.ds(o, cg)], idxs[p][j], semi[p])

        def wait_idx(p):
            for j in range(_NSUB):
                pltpu.make_async_copy(dst_h.at[pl.ds(0, cg)], idxd[p][j],
                                      semi[p]).wait()
                pltpu.make_async_copy(src_h.at[pl.ds(0, cg)], idxs[p][j],
                                      semi[p]).wait()

        def drain_out(p):
            for j in range(_NSUB):
                pltpu.make_async_copy(pd_h.at[pl.ds(0, cg)], bufd[p][j],
                                      semo[p]).wait()
                pltpu.make_async_copy(pd_h.at[pl.ds(0, cg)], bufs[p][j],
                                      semo[p]).wait()

        def step(k_, p):
            wait_idx(p)

            @pl.when(k_ >= 2)
            def _():
                drain_out(p)
            cps = []
            for j in range(_NSUB):
                cps.append(pltpu.async_copy(pd_h.at[idxd[p][j]], bufd[p][j],
                                            semg[p]))
                cps.append(pltpu.async_copy(ps_h.at[idxs[p][j]], bufs[p][j],
                                            semg[p]))
            fire_idx(1 - p, k_ + 1)
            for cp in cps:
                cp.wait()
            for j in range(_NSUB):
                o = offs_of(k_, j)
                pltpu.async_copy(bufd[p][j], gd_o.at[pl.ds(o, cg)], semo[p])
                pltpu.async_copy(bufs[p][j], gs_o.at[pl.ds(o, cg)], semo[p])

        fire_idx(0, 0)

        def body(i, _):
            step(2 * i, 0)
            step(2 * i + 1, 1)
            return 0

        lax.fori_loop(0, nst // 2, body, 0)
        drain_out(0)
        drain_out(1)
        wait_idx(0)

    return k(pd, ps, src, dst)


_CS = 40
_SSTEP = _CS * _NSUB


def _scatter(m, dst, zeros):
    scratch = (
        [pltpu.VMEM((_CS,), jnp.int32)] * _NSUB
        + [pltpu.VMEM((_CS, _MW), _f32)] * _NSUB
        + [pltpu.VMEM_SHARED((_N, _MW), _f32), pltpu.SemaphoreType.DMA]
    )

    @functools.partial(
        pl.kernel,
        mesh=_MESH,
        out_type=jax.ShapeDtypeStruct((2, _N, _MW), _f32),
        scratch_types=scratch,
    )
    def k(m_h, dst_h, z_h, out_h, *scr):
        idx = scr[0:_NSUB]
        buf = scr[_NSUB:2 * _NSUB]
        table = scr[2 * _NSUB]
        sem = scr[2 * _NSUB + 1]
        cid = lax.axis_index("c")
        sid = lax.axis_index("s")
        row0 = pl.multiple_of(sid * 624, 8)

        @pl.when(sid < 15)
        def _():
            pltpu.sync_copy(z_h.at[pl.ds(row0, 624)],
                            table.at[pl.ds(row0, 624)])

        @pl.when(sid == 15)
        def _():
            pltpu.sync_copy(z_h.at[pl.ds(9360, 640)],
                            table.at[pl.ds(9360, 640)])

        plsc.subcore_barrier()
        base = cid * (_E // 2) + sid * _EPW

        def step(i, _):
            offs = [pl.multiple_of(base + i * _SSTEP + j * _CS, 8)
                    for j in range(_NSUB)]
            cps = []
            for j in range(_NSUB):
                cps.append(pltpu.async_copy(dst_h.at[pl.ds(offs[j], _CS)], idx[j], sem))
                cps.append(pltpu.async_copy(m_h.at[pl.ds(offs[j], _CS)], buf[j], sem))
            for cp in cps:
                cp.wait()
            cps = []
            for j in range(_NSUB):
                cps.append(pltpu.async_copy(buf[j], table.at[idx[j]], sem, add=True))
            for cp in cps:
                cp.wait()
            return 0

        lax.fori_loop(0, _EPW // _SSTEP, step, 0)
        plsc.subcore_barrier()

        @pl.when(sid < 15)
        def _():
            pltpu.sync_copy(table.at[pl.ds(row0, 624)],
                            out_h.at[cid, pl.ds(row0, 624)])

        @pl.when(sid == 15)
        def _():
            pltpu.sync_copy(table.at[pl.ds(9360, 640)],
                            out_h.at[cid, pl.ds(9360, 640)])

    return k(m, dst, zeros)


def _proj(feat, wcat, bcat):
    nb = 2000

    def body(f_ref, w_ref, b_ref, pd_ref, ps_ref):
        p = jnp.dot(f_ref[...], w_ref[...], preferred_element_type=_f32)
        p = p + b_ref[...]
        pd_ref[...] = p[:, :128]
        ps_ref[...] = p[:, 128:]

    return pl.pallas_call(
        body,
        grid=(_N // nb,),
        in_specs=[
            pl.BlockSpec((nb, 128), lambda i: (i, 0)),
            pl.BlockSpec((128, 256), lambda i: (0, 0)),
            pl.BlockSpec((1, 256), lambda i: (0, 0)),
        ],
        out_specs=[pl.BlockSpec((nb, 128), lambda i: (i, 0))] * 2,
        out_shape=[jax.ShapeDtypeStruct((_N, 128), _f32)] * 2,
    )(feat, wcat, bcat)


def _edge_call(gd, gs, geo, wg, e2w, e2b, e3w, e3b, v2col, v2b):
    eb = 5000

    def body(gd_ref, gs_ref, g_ref, wg_ref, e2w_ref, e2b_ref, e3w_ref,
             e3b_ref, v2_ref, v2b_ref, m_ref):
        g = g_ref[...]
        su = (gd_ref[...] + gs_ref[...]
              + jnp.dot(g[:, 2:4], wg_ref[...], preferred_element_type=_f32))
        th = _silu(su[:, :64])
        th = _silu(jnp.dot(th, e2w_ref[...], preferred_element_type=_f32)
                   + e2b_ref[...])
        mh = jnp.dot(th, e3w_ref[...], preferred_element_type=_f32) + e3b_ref[...]
        tv = _silu(su[:, 64:])
        vw = jnp.dot(tv, v2_ref[...], preferred_element_type=_f32) + v2b_ref[...]
        mv = vw * g[:, 0:2]
        m_ref[...] = jnp.concatenate(
            [mh, mv, jnp.zeros((eb, _MW - 66), _f32)], axis=1)

    return pl.pallas_call(
        body,
        grid=(_E // eb,),
        in_specs=[
            pl.BlockSpec((eb, 128), lambda i: (i, 0)),
            pl.BlockSpec((eb, 128), lambda i: (i, 0)),
            pl.BlockSpec((eb, 4), lambda i: (i, 0)),
            pl.BlockSpec((2, 128), lambda i: (0, 0)),
            pl.BlockSpec((64, 64), lambda i: (0, 0)),
            pl.BlockSpec((1, 64), lambda i: (0, 0)),
            pl.BlockSpec((64, 64), lambda i: (0, 0)),
            pl.BlockSpec((1, 64), lambda i: (0, 0)),
            pl.BlockSpec((64, 1), lambda i: (0, 0)),
            pl.BlockSpec((1, 1), lambda i: (0, 0)),
        ],
        out_specs=pl.BlockSpec((eb, _MW), lambda i: (i, 0)),
        out_shape=jax.ShapeDtypeStruct((_E, _MW), _f32),
    )(gd, gs, geo, wg, e2w, e2b, e3w, e3b, v2col, v2b)


def _node_proj(feat, msum, w, wcat2, bcat2):
    nb = 2000

    def body(f_ref, ms_ref, wx_ref, wm_ref, wn_ref, h1b_ref, h2w_ref,
             h2b_ref, g_ref, b_ref, wc_ref, bc_ref, o_ref, pd_ref, ps_ref):
        f = f_ref[...]
        m = ms_ref[0] + ms_ref[1]
        mvx = m[:, 64:65]
        mvy = m[:, 65:66]
        mvn = jnp.sqrt(mvx * mvx + mvy * mvy + 1e-12)
        hh = (jnp.dot(f, wx_ref[...], preferred_element_type=_f32)
              + jnp.dot(m, wm_ref[...], preferred_element_type=_f32)
              + mvn * wn_ref[...] + h1b_ref[...])
        hh = _silu(hh)
        up = jnp.dot(hh, h2w_ref[...], preferred_element_type=_f32) + h2b_ref[...]
        y = jnp.maximum(f + up, 0.0)
        mu = jnp.mean(y, axis=1, keepdims=True)
        yc = y - mu
        var = jnp.mean(yc * yc, axis=1, keepdims=True)
        h = yc * jax.lax.rsqrt(var + 1e-5) * g_ref[...] + b_ref[...]
        o_ref[...] = h
        p = jnp.dot(h, wc_ref[...], preferred_element_type=_f32) + bc_ref[...]
        pd_ref[...] = p[:, :128]
        ps_ref[...] = p[:, 128:]

    return pl.pallas_call(
        body,
        grid=(_N // nb,),
        in_specs=[
            pl.BlockSpec((nb, 128), lambda i: (i, 0)),
            pl.BlockSpec((2, nb, _MW), lambda i: (0, i, 0)),
            pl.BlockSpec((128, 64), lambda i: (0, 0)),
            pl.BlockSpec((_MW, 64), lambda i: (0, 0)),
            pl.BlockSpec((1, 64), lambda i: (0, 0)),
            pl.BlockSpec((1, 64), lambda i: (0, 0)),
            pl.BlockSpec((64, 128), lambda i: (0, 0)),
            pl.BlockSpec((1, 128), lambda i: (0, 0)),
            pl.BlockSpec((1, 128), lambda i: (0, 0)),
            pl.BlockSpec((1, 128), lambda i: (0, 0)),
            pl.BlockSpec((128, 256), lambda i: (0, 0)),
            pl.BlockSpec((1, 256), lambda i: (0, 0)),
        ],
        out_specs=[
            pl.BlockSpec((nb, 128), lambda i: (i, 0)),
            pl.BlockSpec((nb, 128), lambda i: (i, 0)),
            pl.BlockSpec((nb, 128), lambda i: (i, 0)),
        ],
        out_shape=[jax.ShapeDtypeStruct((_N, 128), _f32)] * 3,
    )(feat, msum, w['wx'], w['wm'], w['wn'], w['h1b'], w['h2w'], w['h2b'],
      w['g'], w['b'], wcat2, bcat2)


def _node_pool(feat, msum, w, bcol, pos, poolw, poolb, o1w, o1b, o2w, o2b,
               gain):
    nb = 2000
    nsteps = _N // nb

    def body(f_ref, ms_ref, wx_ref, wm_ref, wn_ref, h1b_ref, h2w_ref,
             h2b_ref, g_ref, b_ref, bcol_ref, p_ref, pw_ref, pb_ref,
             o1w_ref, o1b_ref, o2w_ref, o2b_ref, gn_ref,
             s_ref, lat_ref, mu_ref, loss_ref, acc_ref):
        f = f_ref[...]
        m = ms_ref[0] + ms_ref[1]
        mvx = m[:, 64:65]
        mvy = m[:, 65:66]
        mvn = jnp.sqrt(mvx * mvx + mvy * mvy + 1e-12)
        hh = (jnp.dot(f, wx_ref[...], preferred_element_type=_f32)
              + jnp.dot(m, wm_ref[...], preferred_element_type=_f32)
              + mvn * wn_ref[...] + h1b_ref[...])
        hh = _silu(hh)
        up = jnp.dot(hh, h2w_ref[...], preferred_element_type=_f32) + h2b_ref[...]
        y = jnp.maximum(f + up, 0.0)
        mu_ = jnp.mean(y, axis=1, keepdims=True)
        yc = y - mu_
        var = jnp.mean(yc * yc, axis=1, keepdims=True)
        hv = yc * jax.lax.rsqrt(var + 1e-5) * g_ref[...] + b_ref[...]

        logits = jnp.dot(hv, pw_ref[...], preferred_element_type=_f32) + pb_ref[...]
        mx = jnp.max(logits, axis=1, keepdims=True)
        ex = jnp.exp(logits - mx)
        s = ex / jnp.sum(ex, axis=1, keepdims=True)
        s_ref[...] = s
        bc = bcol_ref[...]
        lane = lax.broadcasted_iota(jnp.int32, (nb, 256), 1) // _K
        stile = jnp.concatenate([s] * _B, axis=1)
        wm_ = jnp.where(lane == bc, stile, 0.0)
        entcol = jnp.sum(s * jnp.log(s + 1e-8), axis=1, keepdims=True)
        ones = jnp.ones((nb, 1), _f32)
        w_ext = jnp.concatenate(
            [wm_, s, ones, jnp.zeros((nb, 7), _f32)], axis=1)
        r_ext = jnp.concatenate(
            [hv, p_ref[...], ones, entcol, jnp.zeros((nb, 4), _f32)],
            axis=1)
        acc = lax.dot_general(w_ext, r_ext, (((0,), (0,)), ((), ())),
                              preferred_element_type=_f32)

        @pl.when(pl.program_id(0) == 0)
        def _():
            acc_ref[...] = acc

        @pl.when(pl.program_id(0) != 0)
        def _():
            acc_ref[...] += acc

        @pl.when(pl.program_id(0) == nsteps - 1)
        def _():
            a = acc_ref[...]
            den = a[:256, 130:131] + 1e-8
            pooled = a[:256, :128] / den
            z = jnp.maximum(
                jnp.dot(pooled, o1w_ref[...], preferred_element_type=_f32)
                + o1b_ref[...], 0.0)
            lat_ref[...] = (jnp.dot(z, o2w_ref[...],
                                    preferred_element_type=_f32)
                            + o2b_ref[...]) * gn_ref[...]
            mu_ref[...] = a[:256, 128:130] / den
            usage = a[256:288, 130:131] * (1.0 / _N)
            lb = jnp.sum(usage * jnp.log(usage * _K + 1e-8), axis=0,
                         keepdims=True)
            ent = -a[288:289, 131:132] * (1.0 / _N)
            loss_ref[...] = ent + lb

    return pl.pallas_call(
        body,
        grid=(nsteps,),
        in_specs=[
            pl.BlockSpec((nb, 128), lambda i: (i, 0)),
            pl.BlockSpec((2, nb, _MW), lambda i: (0, i, 0)),
            pl.BlockSpec((128, 64), lambda i: (0, 0)),
            pl.BlockSpec((_MW, 64), lambda i: (0, 0)),
            pl.BlockSpec((1, 64), lambda i: (0, 0)),
            pl.BlockSpec((1, 64), lambda i: (0, 0)),
            pl.BlockSpec((64, 128), lambda i: (0, 0)),
            pl.BlockSpec((1, 128), lambda i: (0, 0)),
            pl.BlockSpec((1, 128), lambda i: (0, 0)),
            pl.BlockSpec((1, 128), lambda i: (0, 0)),
            pl.BlockSpec((nb, 1), lambda i: (i, 0)),
            pl.BlockSpec((nb, 2), lambda i: (i, 0)),
            pl.BlockSpec((128, _K), lambda i: (0, 0)),
            pl.BlockSpec((1, _K), lambda i: (0, 0)),
            pl.BlockSpec((128, 128), lambda i: (0, 0)),
            pl.BlockSpec((1, 128), lambda i: (0, 0)),
            pl.BlockSpec((128, _LAT), lambda i: (0, 0)),
            pl.BlockSpec((1, _LAT), lambda i: (0, 0)),
            pl.BlockSpec((1, _LAT), lambda i: (0, 0)),
        ],
        out_specs=[
            pl.BlockSpec((nb, _K), lambda i: (i, 0)),
            pl.BlockSpec((256, _LAT), lambda i: (0, 0)),
            pl.BlockSpec((256, 2), lambda i: (0, 0)),
            pl.BlockSpec((1, 1), lambda i: (0, 0)),
        ],
        out_shape=[
            jax.ShapeDtypeStruct((_N, _K), _f32),
            jax.ShapeDtypeStruct((256, _LAT), _f32),
            jax.ShapeDtypeStruct((256, 2), _f32),
            jax.ShapeDtypeStruct((1, 1), _f32),
        ],
        scratch_shapes=[pltpu.VMEM((296, 136), _f32)],
    )(feat, msum, w['wx'], w['wm'], w['wn'], w['h1b'], w['h2w'], w['h2b'],
      w['g'], w['b'], bcol, pos, poolw, poolb, o1w, o1b, o2w, o2b, gain)


def _layer_weights(p):
    e1w, e1b = p['e1']
    v1w, v1b = p['v1']
    wcat = jnp.concatenate(
        [e1w[:_F], v1w[:_F], e1w[_F:2 * _F], v1w[_F:2 * _F]], axis=1)
    bcat = jnp.concatenate(
        [e1b, v1b, jnp.zeros((2 * _HID,), _f32)]).reshape(1, 256)
    wg = jnp.concatenate([e1w[2 * _F:], v1w[2 * _F:]], axis=1)
    h1w, h1b = p['h1']
    wx = h1w[:_F]
    wm = jnp.concatenate([h1w[_F:_F + 64], jnp.zeros((_MW - 64, 64), _f32)],
                         axis=0)
    wn = h1w[_F + 64].reshape(1, 64)
    return dict(
        wcat=wcat, bcat=bcat, wg=wg,
        e2w=p['e2'][0], e2b=p['e2'][1].reshape(1, 64),
        e3w=p['e3'][0], e3b=p['e3'][1].reshape(1, 64),
        v2col=p['v2'][0], v2b=p['v2'][1].reshape(1, 1),
        wx=wx, wm=wm, wn=wn, h1b=h1b.reshape(1, 64),
        h2w=p['h2'][0], h2b=p['h2'][1].reshape(1, 128),
    )


def kernel(x, edge_index, batch, p1, p2, ln1, ln2, pool, out1, out2,
           latent_gain):
    src = edge_index[0]
    dst = edge_index[1]
    pos = x[:, :2]
    zeros_tab = jnp.zeros((_N, _MW), _f32)
    bcol = batch.reshape(_N, 1)

    w1 = _layer_weights(p1)
    w1['g'] = ln1[0].reshape(1, 128)
    w1['b'] = ln1[1].reshape(1, 128)
    w2 = _layer_weights(p2)
    w2['g'] = ln2[0].reshape(1, 128)
    w2['b'] = ln2[1].reshape(1, 128)

    rx, ry, dd, dt = _geo(x[:, 0], x[:, 1], x[:, 2], x[:, 3], src, dst)
    pd, ps = _proj(x, w1['wcat'], w1['bcat'])
    gd, gs = _gather(pd, ps, src, dst)
    geo4 = jnp.concatenate(
        [rx.reshape(_E, 1), ry.reshape(_E, 1), dd.reshape(_E, 1),
         dt.reshape(_E, 1)], axis=1)
    m1 = _edge_call(gd, gs, geo4, w1['wg'], w1['e2w'], w1['e2b'],
                    w1['e3w'], w1['e3b'], w1['v2col'], w1['v2b'])
    msum1 = _scatter(m1, dst, zeros_tab)
    h1, pd2, ps2 = _node_proj(x, msum1, w1, w2['wcat'], w2['bcat'])

    gd2, gs2 = _gather(pd2, ps2, src, dst)
    m2 = _edge_call(gd2, gs2, geo4, w2['wg'], w2['e2w'], w2['e2b'],
                    w2['e3w'], w2['e3b'], w2['v2col'], w2['v2b'])
    msum2 = _scatter(m2, dst, zeros_tab)
    s, lat, mu, loss = _node_pool(
        h1, msum2, w2, bcol, pos, pool[0], pool[1].reshape(1, _K), out1[0],
        out1[1].reshape(1, 128), out2[0], out2[1].reshape(1, _LAT),
        latent_gain.reshape(1, _LAT))
    return (lat.reshape(_B, _K, _LAT), s, loss[0, 0],
            mu.reshape(_B, _K, 2))


# speedup vs baseline: 1.6817x; 1.1199x over previous
import functools

import jax
import jax.numpy as jnp
from jax import lax
from jax.experimental import pallas as pl
from jax.experimental.pallas import tpu as pltpu
from jax.experimental.pallas import tpu_sc as plsc

_N = 10000
_E = 320000
_F = 128
_HID = 64
_OUT = 128
_K = 32
_LAT = 64
_B = 8

_NW = 32
_EPW = _E // _NW
_C = 80
_NSUB = 5
_STEP = _C * _NSUB
_NITER = _EPW // _STEP
_ROWS = _N // 16
_MW = 128

_f32 = jnp.float32

_MESH = plsc.VectorSubcoreMesh(core_axis_name="c", subcore_axis_name="s")


def _silu(x):
    return x / (1.0 + jnp.exp(-x))


def _geo(px, py, vx, vy, src, dst):
    scratch = (
        [pltpu.VMEM((_C,), jnp.int32)] * (2 * _NSUB)
        + [pltpu.VMEM((_C,), _f32)] * (8 * _NSUB)
        + [pltpu.VMEM((_C,), _f32)] * (4 * _NSUB)
        + [pltpu.SemaphoreType.DMA]
    )

    @functools.partial(
        pl.kernel,
        mesh=_MESH,
        out_type=[jax.ShapeDtypeStruct((_E,), _f32)] * 4,
        scratch_types=scratch,
    )
    def k(px_h, py_h, vx_h, vy_h, src_h, dst_h, rx_o, ry_o, dd_o, dt_o, *scr):
        idxd = scr[0:_NSUB]
        idxs = scr[_NSUB:2 * _NSUB]
        gb = scr[2 * _NSUB:10 * _NSUB]
        ob = scr[10 * _NSUB:14 * _NSUB]
        sem = scr[14 * _NSUB]
        wid = lax.axis_index("s") * 2 + lax.axis_index("c")
        base = wid * _EPW
        tabs = (px_h, py_h, vx_h, vy_h)
        gouts = (rx_o, ry_o, dd_o, dt_o)

        def step(i, _):
            offs = [pl.multiple_of(base + i * _STEP + j * _C, 8)
                    for j in range(_NSUB)]
            cps = []
            for j in range(_NSUB):
                cps.append(pltpu.async_copy(dst_h.at[pl.ds(offs[j], _C)], idxd[j], sem))
                cps.append(pltpu.async_copy(src_h.at[pl.ds(offs[j], _C)], idxs[j], sem))
            for cp in cps:
                cp.wait()
            cps = []
            for j in range(_NSUB):
                for t in range(4):
                    cps.append(pltpu.async_copy(
                        tabs[t].at[idxd[j]], gb[8 * j + t], sem))
                    cps.append(pltpu.async_copy(
                        tabs[t].at[idxs[j]], gb[8 * j + 4 + t], sem))
            for cp in cps:
                cp.wait()
            for j in range(_NSUB):
                for g in range(_C // 16):
                    sl = pl.ds(pl.multiple_of(g * 16, 8), 16)
                    rx = gb[8 * j + 4][sl] - gb[8 * j + 0][sl]
                    ry = gb[8 * j + 5][sl] - gb[8 * j + 1][sl]
                    wx = gb[8 * j + 6][sl] - gb[8 * j + 2][sl]
                    wy = gb[8 * j + 7][sl] - gb[8 * j + 3][sl]
                    ob[4 * j + 0][sl] = rx
                    ob[4 * j + 1][sl] = ry
                    ob[4 * j + 2][sl] = rx * rx + ry * ry
                    ob[4 * j + 3][sl] = wx * rx + wy * ry
            cps = []
            for j in range(_NSUB):
                for t in range(4):
                    cps.append(pltpu.async_copy(
                        ob[4 * j + t], gouts[t].at[pl.ds(offs[j], _C)], sem))
            for cp in cps:
                cp.wait()
            return 0

        lax.fori_loop(0, _NITER, step, 0)

    return k(px, py, vx, vy, src, dst)


def _gather(pd, ps, src, dst, geo_tabs=None):
    del geo_tabs
    cg = 40
    nst = _EPW // (cg * _NSUB)
    scratch = (
        [pltpu.VMEM((cg,), jnp.int32)] * (4 * _NSUB)
        + [pltpu.VMEM((cg, 128), _f32)] * (4 * _NSUB)
        + [pltpu.SemaphoreType.DMA] * 6
    )

    @functools.partial(
        pl.kernel,
        mesh=_MESH,
        out_type=jax.ShapeDtypeStruct((_E, 128), _f32),
        scratch_types=scratch,
    )
    def k(pd_h, ps_h, src_h, dst_h, su_o, *scr):
        idxd = [scr[0:_NSUB], scr[_NSUB:2 * _NSUB]]
        idxs = [scr[2 * _NSUB:3 * _NSUB], scr[3 * _NSUB:4 * _NSUB]]
        bufd = [scr[4 * _NSUB:5 * _NSUB], scr[5 * _NSUB:6 * _NSUB]]
        bufs = [scr[6 * _NSUB:7 * _NSUB], scr[7 * _NSUB:8 * _NSUB]]
        semi = [scr[8 * _NSUB], scr[8 * _NSUB + 1]]
        semg = [scr[8 * _NSUB + 2], scr[8 * _NSUB + 3]]
        semo = [scr[8 * _NSUB + 4], scr[8 * _NSUB + 5]]
        wid = lax.axis_index("s") * 2 + lax.axis_index("c")
        base = wid * _EPW

        def offs_of(k_, j):
            if isinstance(k_, int):
                km = (k_ * cg * _NSUB) % _EPW
            else:
                km = lax.rem(k_ * (cg * _NSUB), _EPW)
            return pl.multiple_of(base + km + j * cg, 8)

        def fire_idx(p, k_):
            for j in range(_NSUB):
                o = offs_of(k_, j)
                pltpu.async_copy(dst_h.at[pl.ds(o, cg)], idxd[p][j], semi[p])
                pltpu.async_copy(src_h.at[pl.ds(o, cg)], idxs[p][j], semi[p])

        def wait_idx(p):
            for j in range(_NSUB):
                pltpu.make_async_copy(dst_h.at[pl.ds(0, cg)], idxd[p][j],
                                      semi[p]).wait()
                pltpu.make_async_copy(src_h.at[pl.ds(0, cg)], idxs[p][j],
                                      semi[p]).wait()

        def drain_out(p):
            for j in range(_NSUB):
                pltpu.make_async_copy(pd_h.at[pl.ds(0, cg)], bufd[p][j],
                                      semo[p]).wait()

        def consume(q, k_):
            for j in range(_NSUB):
                pltpu.make_async_copy(pd_h.at[pl.ds(0, cg)], bufd[q][j],
                                      semg[q]).wait()
                pltpu.make_async_copy(pd_h.at[pl.ds(0, cg)], bufs[q][j],
                                      semg[q]).wait()

            def addrow(r, _):
                for j in range(_NSUB):
                    for g in range(8):
                        sl = pl.ds(g * 16, 16)
                        bufd[q][j][r, sl] = bufd[q][j][r, sl] + bufs[q][j][r, sl]
                return 0

            lax.fori_loop(0, cg, addrow, 0)
            for j in range(_NSUB):
                o = offs_of(k_, j)
                pltpu.async_copy(bufd[q][j], su_o.at[pl.ds(o, cg)], semo[q])

        def step(k_, p, pred):
            wait_idx(p)

            @pl.when(pred)
            def _():
                drain_out(p)
            for j in range(_NSUB):
                pltpu.async_copy(pd_h.at[idxd[p][j]], bufd[p][j], semg[p])
                pltpu.async_copy(ps_h.at[idxs[p][j]], bufs[p][j], semg[p])
            fire_idx(1 - p, k_ + 1)
            if isinstance(k_, int):
                if k_ >= 1:
                    consume(1 - p, k_ - 1)
            else:
                @pl.when(k_ >= 1)
                def _():
                    consume(1 - p, k_ - 1)

        fire_idx(0, 0)

        def body(i, _):
            step(2 * i, 0, i >= 1)
            step(2 * i + 1, 1, i >= 1)
            return 0

        lax.fori_loop(0, nst // 2, body, 0)
        consume(1, nst - 1)
        drain_out(0)
        drain_out(1)
        wait_idx(0)

    return k(pd, ps, src, dst)


_CS = 40
_SSTEP = _CS * _NSUB


def _scatter(m, dst, zeros):
    scratch = (
        [pltpu.VMEM((_CS,), jnp.int32)] * _NSUB
        + [pltpu.VMEM((_CS, _MW), _f32)] * _NSUB
        + [pltpu.VMEM_SHARED((_N, _MW), _f32), pltpu.SemaphoreType.DMA]
    )

    @functools.partial(
        pl.kernel,
        mesh=_MESH,
        out_type=jax.ShapeDtypeStruct((2, _N, _MW), _f32),
        scratch_types=scratch,
    )
    def k(m_h, dst_h, z_h, out_h, *scr):
        idx = scr[0:_NSUB]
        buf = scr[_NSUB:2 * _NSUB]
        table = scr[2 * _NSUB]
        sem = scr[2 * _NSUB + 1]
        cid = lax.axis_index("c")
        sid = lax.axis_index("s")
        row0 = pl.multiple_of(sid * 624, 8)

        @pl.when(sid < 15)
        def _():
            pltpu.sync_copy(z_h.at[pl.ds(row0, 624)],
                            table.at[pl.ds(row0, 624)])

        @pl.when(sid == 15)
        def _():
            pltpu.sync_copy(z_h.at[pl.ds(9360, 640)],
                            table.at[pl.ds(9360, 640)])

        plsc.subcore_barrier()
        base = cid * (_E // 2) + sid * _EPW

        def step(i, _):
            offs = [pl.multiple_of(base + i * _SSTEP + j * _CS, 8)
                    for j in range(_NSUB)]
            cps = []
            for j in range(_NSUB):
                cps.append(pltpu.async_copy(dst_h.at[pl.ds(offs[j], _CS)], idx[j], sem))
                cps.append(pltpu.async_copy(m_h.at[pl.ds(offs[j], _CS)], buf[j], sem))
            for cp in cps:
                cp.wait()
            cps = []
            for j in range(_NSUB):
                cps.append(pltpu.async_copy(buf[j], table.at[idx[j]], sem, add=True))
            for cp in cps:
                cp.wait()
            return 0

        lax.fori_loop(0, _EPW // _SSTEP, step, 0)
        plsc.subcore_barrier()

        @pl.when(sid < 15)
        def _():
            pltpu.sync_copy(table.at[pl.ds(row0, 624)],
                            out_h.at[cid, pl.ds(row0, 624)])

        @pl.when(sid == 15)
        def _():
            pltpu.sync_copy(table.at[pl.ds(9360, 640)],
                            out_h.at[cid, pl.ds(9360, 640)])

    return k(m, dst, zeros)


def _proj(feat, wcat, bcat):
    nb = 2000

    def body(f_ref, w_ref, b_ref, pd_ref, ps_ref):
        p = jnp.dot(f_ref[...], w_ref[...], preferred_element_type=_f32)
        p = p + b_ref[...]
        pd_ref[...] = p[:, :128]
        ps_ref[...] = p[:, 128:]

    return pl.pallas_call(
        body,
        grid=(_N // nb,),
        in_specs=[
            pl.BlockSpec((nb, 128), lambda i: (i, 0)),
            pl.BlockSpec((128, 256), lambda i: (0, 0)),
            pl.BlockSpec((1, 256), lambda i: (0, 0)),
        ],
        out_specs=[pl.BlockSpec((nb, 128), lambda i: (i, 0))] * 2,
        out_shape=[jax.ShapeDtypeStruct((_N, 128), _f32)] * 2,
    )(feat, wcat, bcat)


def _edge_call(su, geo, wg, e2w, e2b, e3w, e3b, v2col, v2b):
    eb = 5000

    def body(su_ref, g_ref, wg_ref, e2w_ref, e2b_ref, e3w_ref,
             e3b_ref, v2_ref, v2b_ref, m_ref):
        g = g_ref[...]
        su = (su_ref[...]
              + jnp.dot(g[:, 2:4], wg_ref[...], preferred_element_type=_f32))
        th = _silu(su[:, :64])
        th = _silu(jnp.dot(th, e2w_ref[...], preferred_element_type=_f32)
                   + e2b_ref[...])
        mh = jnp.dot(th, e3w_ref[...], preferred_element_type=_f32) + e3b_ref[...]
        tv = _silu(su[:, 64:])
        vw = jnp.dot(tv, v2_ref[...], preferred_element_type=_f32) + v2b_ref[...]
        mv = vw * g[:, 0:2]
        m_ref[...] = jnp.concatenate(
            [mh, mv, jnp.zeros((eb, _MW - 66), _f32)], axis=1)

    return pl.pallas_call(
        body,
        grid=(_E // eb,),
        in_specs=[
            pl.BlockSpec((eb, 128), lambda i: (i, 0)),
            pl.BlockSpec((eb, 4), lambda i: (i, 0)),
            pl.BlockSpec((2, 128), lambda i: (0, 0)),
            pl.BlockSpec((64, 64), lambda i: (0, 0)),
            pl.BlockSpec((1, 64), lambda i: (0, 0)),
            pl.BlockSpec((64, 64), lambda i: (0, 0)),
            pl.BlockSpec((1, 64), lambda i: (0, 0)),
            pl.BlockSpec((64, 1), lambda i: (0, 0)),
            pl.BlockSpec((1, 1), lambda i: (0, 0)),
        ],
        out_specs=pl.BlockSpec((eb, _MW), lambda i: (i, 0)),
        out_shape=jax.ShapeDtypeStruct((_E, _MW), _f32),
    )(su, geo, wg, e2w, e2b, e3w, e3b, v2col, v2b)


def _node_proj(feat, msum, w, wcat2, bcat2):
    nb = 2000

    def body(f_ref, ms_ref, wx_ref, wm_ref, wn_ref, h1b_ref, h2w_ref,
             h2b_ref, g_ref, b_ref, wc_ref, bc_ref, o_ref, pd_ref, ps_ref):
        f = f_ref[...]
        m = ms_ref[0] + ms_ref[1]
        mvx = m[:, 64:65]
        mvy = m[:, 65:66]
        mvn = jnp.sqrt(mvx * mvx + mvy * mvy + 1e-12)
        hh = (jnp.dot(f, wx_ref[...], preferred_element_type=_f32)
              + jnp.dot(m, wm_ref[...], preferred_element_type=_f32)
              + mvn * wn_ref[...] + h1b_ref[...])
        hh = _silu(hh)
        up = jnp.dot(hh, h2w_ref[...], preferred_element_type=_f32) + h2b_ref[...]
        y = jnp.maximum(f + up, 0.0)
        mu = jnp.mean(y, axis=1, keepdims=True)
        yc = y - mu
        var = jnp.mean(yc * yc, axis=1, keepdims=True)
        h = yc * jax.lax.rsqrt(var + 1e-5) * g_ref[...] + b_ref[...]
        o_ref[...] = h
        p = jnp.dot(h, wc_ref[...], preferred_element_type=_f32) + bc_ref[...]
        pd_ref[...] = p[:, :128]
        ps_ref[...] = p[:, 128:]

    return pl.pallas_call(
        body,
        grid=(_N // nb,),
        in_specs=[
            pl.BlockSpec((nb, 128), lambda i: (i, 0)),
            pl.BlockSpec((2, nb, _MW), lambda i: (0, i, 0)),
            pl.BlockSpec((128, 64), lambda i: (0, 0)),
            pl.BlockSpec((_MW, 64), lambda i: (0, 0)),
            pl.BlockSpec((1, 64), lambda i: (0, 0)),
            pl.BlockSpec((1, 64), lambda i: (0, 0)),
            pl.BlockSpec((64, 128), lambda i: (0, 0)),
            pl.BlockSpec((1, 128), lambda i: (0, 0)),
            pl.BlockSpec((1, 128), lambda i: (0, 0)),
            pl.BlockSpec((1, 128), lambda i: (0, 0)),
            pl.BlockSpec((128, 256), lambda i: (0, 0)),
            pl.BlockSpec((1, 256), lambda i: (0, 0)),
        ],
        out_specs=[
            pl.BlockSpec((nb, 128), lambda i: (i, 0)),
            pl.BlockSpec((nb, 128), lambda i: (i, 0)),
            pl.BlockSpec((nb, 128), lambda i: (i, 0)),
        ],
        out_shape=[jax.ShapeDtypeStruct((_N, 128), _f32)] * 3,
    )(feat, msum, w['wx'], w['wm'], w['wn'], w['h1b'], w['h2w'], w['h2b'],
      w['g'], w['b'], wcat2, bcat2)


def _node_pool(feat, msum, w, bcol, pos, poolw, poolb, o1w, o1b, o2w, o2b,
               gain):
    nb = 2000
    nsteps = _N // nb

    def body(f_ref, ms_ref, wx_ref, wm_ref, wn_ref, h1b_ref, h2w_ref,
             h2b_ref, g_ref, b_ref, bcol_ref, p_ref, pw_ref, pb_ref,
             o1w_ref, o1b_ref, o2w_ref, o2b_ref, gn_ref,
             s_ref, lat_ref, mu_ref, loss_ref, acc_ref):
        f = f_ref[...]
        m = ms_ref[0] + ms_ref[1]
        mvx = m[:, 64:65]
        mvy = m[:, 65:66]
        mvn = jnp.sqrt(mvx * mvx + mvy * mvy + 1e-12)
        hh = (jnp.dot(f, wx_ref[...], preferred_element_type=_f32)
              + jnp.dot(m, wm_ref[...], preferred_element_type=_f32)
              + mvn * wn_ref[...] + h1b_ref[...])
        hh = _silu(hh)
        up = jnp.dot(hh, h2w_ref[...], preferred_element_type=_f32) + h2b_ref[...]
        y = jnp.maximum(f + up, 0.0)
        mu_ = jnp.mean(y, axis=1, keepdims=True)
        yc = y - mu_
        var = jnp.mean(yc * yc, axis=1, keepdims=True)
        hv = yc * jax.lax.rsqrt(var + 1e-5) * g_ref[...] + b_ref[...]

        logits = jnp.dot(hv, pw_ref[...], preferred_element_type=_f32) + pb_ref[...]
        mx = jnp.max(logits, axis=1, keepdims=True)
        ex = jnp.exp(logits - mx)
        s = ex / jnp.sum(ex, axis=1, keepdims=True)
        s_ref[...] = s
        bc = bcol_ref[...]
        lane = lax.broadcasted_iota(jnp.int32, (nb, 256), 1) // _K
        stile = jnp.concatenate([s] * _B, axis=1)
        wm_ = jnp.where(lane == bc, stile, 0.0)
        entcol = jnp.sum(s * jnp.log(s + 1e-8), axis=1, keepdims=True)
        ones = jnp.ones((nb, 1), _f32)
        w_ext = jnp.concatenate(
            [wm_, s, ones, jnp.zeros((nb, 7), _f32)], axis=1)
        r_ext = jnp.concatenate(
            [hv, p_ref[...], ones, entcol, jnp.zeros((nb, 4), _f32)],
            axis=1)
        acc = lax.dot_general(w_ext, r_ext, (((0,), (0,)), ((), ())),
                              preferred_element_type=_f32)

        @pl.when(pl.program_id(0) == 0)
        def _():
            acc_ref[...] = acc

        @pl.when(pl.program_id(0) != 0)
        def _():
            acc_ref[...] += acc

        @pl.when(pl.program_id(0) == nsteps - 1)
        def _():
            a = acc_ref[...]
            den = a[:256, 130:131] + 1e-8
            pooled = a[:256, :128] / den
            z = jnp.maximum(
                jnp.dot(pooled, o1w_ref[...], preferred_element_type=_f32)
                + o1b_ref[...], 0.0)
            lat_ref[...] = (jnp.dot(z, o2w_ref[...],
                                    preferred_element_type=_f32)
                            + o2b_ref[...]) * gn_ref[...]
            mu_ref[...] = a[:256, 128:130] / den
            usage = a[256:288, 130:131] * (1.0 / _N)
            lb = jnp.sum(usage * jnp.log(usage * _K + 1e-8), axis=0,
                         keepdims=True)
            ent = -a[288:289, 131:132] * (1.0 / _N)
            loss_ref[...] = ent + lb

    return pl.pallas_call(
        body,
        grid=(nsteps,),
        in_specs=[
            pl.BlockSpec((nb, 128), lambda i: (i, 0)),
            pl.BlockSpec((2, nb, _MW), lambda i: (0, i, 0)),
            pl.BlockSpec((128, 64), lambda i: (0, 0)),
            pl.BlockSpec((_MW, 64), lambda i: (0, 0)),
            pl.BlockSpec((1, 64), lambda i: (0, 0)),
            pl.BlockSpec((1, 64), lambda i: (0, 0)),
            pl.BlockSpec((64, 128), lambda i: (0, 0)),
            pl.BlockSpec((1, 128), lambda i: (0, 0)),
            pl.BlockSpec((1, 128), lambda i: (0, 0)),
            pl.BlockSpec((1, 128), lambda i: (0, 0)),
            pl.BlockSpec((nb, 1), lambda i: (i, 0)),
            pl.BlockSpec((nb, 2), lambda i: (i, 0)),
            pl.BlockSpec((128, _K), lambda i: (0, 0)),
            pl.BlockSpec((1, _K), lambda i: (0, 0)),
            pl.BlockSpec((128, 128), lambda i: (0, 0)),
            pl.BlockSpec((1, 128), lambda i: (0, 0)),
            pl.BlockSpec((128, _LAT), lambda i: (0, 0)),
            pl.BlockSpec((1, _LAT), lambda i: (0, 0)),
            pl.BlockSpec((1, _LAT), lambda i: (0, 0)),
        ],
        out_specs=[
            pl.BlockSpec((nb, _K), lambda i: (i, 0)),
            pl.BlockSpec((256, _LAT), lambda i: (0, 0)),
            pl.BlockSpec((256, 2), lambda i: (0, 0)),
            pl.BlockSpec((1, 1), lambda i: (0, 0)),
        ],
        out_shape=[
            jax.ShapeDtypeStruct((_N, _K), _f32),
            jax.ShapeDtypeStruct((256, _LAT), _f32),
            jax.ShapeDtypeStruct((256, 2), _f32),
            jax.ShapeDtypeStruct((1, 1), _f32),
        ],
        scratch_shapes=[pltpu.VMEM((296, 136), _f32)],
    )(feat, msum, w['wx'], w['wm'], w['wn'], w['h1b'], w['h2w'], w['h2b'],
      w['g'], w['b'], bcol, pos, poolw, poolb, o1w, o1b, o2w, o2b, gain)


def _layer_weights(p):
    e1w, e1b = p['e1']
    v1w, v1b = p['v1']
    wcat = jnp.concatenate(
        [e1w[:_F], v1w[:_F], e1w[_F:2 * _F], v1w[_F:2 * _F]], axis=1)
    bcat = jnp.concatenate(
        [e1b, v1b, jnp.zeros((2 * _HID,), _f32)]).reshape(1, 256)
    wg = jnp.concatenate([e1w[2 * _F:], v1w[2 * _F:]], axis=1)
    h1w, h1b = p['h1']
    wx = h1w[:_F]
    wm = jnp.concatenate([h1w[_F:_F + 64], jnp.zeros((_MW - 64, 64), _f32)],
                         axis=0)
    wn = h1w[_F + 64].reshape(1, 64)
    return dict(
        wcat=wcat, bcat=bcat, wg=wg,
        e2w=p['e2'][0], e2b=p['e2'][1].reshape(1, 64),
        e3w=p['e3'][0], e3b=p['e3'][1].reshape(1, 64),
        v2col=p['v2'][0], v2b=p['v2'][1].reshape(1, 1),
        wx=wx, wm=wm, wn=wn, h1b=h1b.reshape(1, 64),
        h2w=p['h2'][0], h2b=p['h2'][1].reshape(1, 128),
    )


def kernel(x, edge_index, batch, p1, p2, ln1, ln2, pool, out1, out2,
           latent_gain):
    src = edge_index[0]
    dst = edge_index[1]
    pos = x[:, :2]
    zeros_tab = jnp.zeros((_N, _MW), _f32)
    bcol = batch.reshape(_N, 1)

    w1 = _layer_weights(p1)
    w1['g'] = ln1[0].reshape(1, 128)
    w1['b'] = ln1[1].reshape(1, 128)
    w2 = _layer_weights(p2)
    w2['g'] = ln2[0].reshape(1, 128)
    w2['b'] = ln2[1].reshape(1, 128)

    rx, ry, dd, dt = _geo(x[:, 0], x[:, 1], x[:, 2], x[:, 3], src, dst)
    pd, ps = _proj(x, w1['wcat'], w1['bcat'])
    su1 = _gather(pd, ps, src, dst)
    geo4 = jnp.concatenate(
        [rx.reshape(_E, 1), ry.reshape(_E, 1), dd.reshape(_E, 1),
         dt.reshape(_E, 1)], axis=1)
    m1 = _edge_call(su1, geo4, w1['wg'], w1['e2w'], w1['e2b'],
                    w1['e3w'], w1['e3b'], w1['v2col'], w1['v2b'])
    msum1 = _scatter(m1, dst, zeros_tab)
    h1, pd2, ps2 = _node_proj(x, msum1, w1, w2['wcat'], w2['bcat'])

    su2 = _gather(pd2, ps2, src, dst)
    m2 = _edge_call(su2, geo4, w2['wg'], w2['e2w'], w2['e2b'],
                    w2['e3w'], w2['e3b'], w2['v2col'], w2['v2b'])
    msum2 = _scatter(m2, dst, zeros_tab)
    s, lat, mu, loss = _node_pool(
        h1, msum2, w2, bcol, pos, pool[0], pool[1].reshape(1, _K), out1[0],
        out1[1].reshape(1, 128), out2[0], out2[1].reshape(1, _LAT),
        latent_gain.reshape(1, _LAT))
    return (lat.reshape(_B, _K, _LAT), s, loss[0, 0],
            mu.reshape(_B, _K, 2))


# speedup vs baseline: 1.6818x; 1.0000x over previous
import functools

import jax
import jax.numpy as jnp
from jax import lax
from jax.experimental import pallas as pl
from jax.experimental.pallas import tpu as pltpu
from jax.experimental.pallas import tpu_sc as plsc

_N = 10000
_E = 320000
_F = 128
_HID = 64
_OUT = 128
_K = 32
_LAT = 64
_B = 8

_NW = 32
_EPW = _E // _NW
_C = 80
_NSUB = 5
_STEP = _C * _NSUB
_NITER = _EPW // _STEP
_ROWS = _N // 16
_MW = 128

_f32 = jnp.float32

_MESH = plsc.VectorSubcoreMesh(core_axis_name="c", subcore_axis_name="s")


def _silu(x):
    return x / (1.0 + jnp.exp(-x))


def _geo(px, py, vx, vy, src, dst):
    scratch = (
        [pltpu.VMEM((_C,), jnp.int32)] * (2 * _NSUB)
        + [pltpu.VMEM((_C,), _f32)] * (8 * _NSUB)
        + [pltpu.VMEM((_C,), _f32)] * (4 * _NSUB)
        + [pltpu.SemaphoreType.DMA]
    )

    @functools.partial(
        pl.kernel,
        mesh=_MESH,
        out_type=[jax.ShapeDtypeStruct((_E,), _f32)] * 4,
        scratch_types=scratch,
    )
    def k(px_h, py_h, vx_h, vy_h, src_h, dst_h, rx_o, ry_o, dd_o, dt_o, *scr):
        idxd = scr[0:_NSUB]
        idxs = scr[_NSUB:2 * _NSUB]
        gb = scr[2 * _NSUB:10 * _NSUB]
        ob = scr[10 * _NSUB:14 * _NSUB]
        sem = scr[14 * _NSUB]
        wid = lax.axis_index("s") * 2 + lax.axis_index("c")
        base = wid * _EPW
        tabs = (px_h, py_h, vx_h, vy_h)
        gouts = (rx_o, ry_o, dd_o, dt_o)

        def step(i, _):
            offs = [pl.multiple_of(base + i * _STEP + j * _C, 8)
                    for j in range(_NSUB)]
            cps = []
            for j in range(_NSUB):
                cps.append(pltpu.async_copy(dst_h.at[pl.ds(offs[j], _C)], idxd[j], sem))
                cps.append(pltpu.async_copy(src_h.at[pl.ds(offs[j], _C)], idxs[j], sem))
            for cp in cps:
                cp.wait()
            cps = []
            for j in range(_NSUB):
                for t in range(4):
                    cps.append(pltpu.async_copy(
                        tabs[t].at[idxd[j]], gb[8 * j + t], sem))
                    cps.append(pltpu.async_copy(
                        tabs[t].at[idxs[j]], gb[8 * j + 4 + t], sem))
            for cp in cps:
                cp.wait()
            for j in range(_NSUB):
                for g in range(_C // 16):
                    sl = pl.ds(pl.multiple_of(g * 16, 8), 16)
                    rx = gb[8 * j + 4][sl] - gb[8 * j + 0][sl]
                    ry = gb[8 * j + 5][sl] - gb[8 * j + 1][sl]
                    wx = gb[8 * j + 6][sl] - gb[8 * j + 2][sl]
                    wy = gb[8 * j + 7][sl] - gb[8 * j + 3][sl]
                    ob[4 * j + 0][sl] = rx
                    ob[4 * j + 1][sl] = ry
                    ob[4 * j + 2][sl] = rx * rx + ry * ry
                    ob[4 * j + 3][sl] = wx * rx + wy * ry
            cps = []
            for j in range(_NSUB):
                for t in range(4):
                    cps.append(pltpu.async_copy(
                        ob[4 * j + t], gouts[t].at[pl.ds(offs[j], _C)], sem))
            for cp in cps:
                cp.wait()
            return 0

        lax.fori_loop(0, _NITER, step, 0)

    return k(px, py, vx, vy, src, dst)


def _gather(pd, ps, src, dst, geo_tabs=None):
    del geo_tabs
    cg = 40
    nst = _EPW // (cg * _NSUB)
    scratch = (
        [pltpu.VMEM((cg,), jnp.int32)] * (4 * _NSUB)
        + [pltpu.VMEM((cg, 128), _f32)] * (4 * _NSUB)
        + [pltpu.SemaphoreType.DMA] * 6
    )

    @functools.partial(
        pl.kernel,
        mesh=_MESH,
        out_type=jax.ShapeDtypeStruct((_E, 128), _f32),
        scratch_types=scratch,
    )
    def k(pd_h, ps_h, src_h, dst_h, su_o, *scr):
        idxd = [scr[0:_NSUB], scr[_NSUB:2 * _NSUB]]
        idxs = [scr[2 * _NSUB:3 * _NSUB], scr[3 * _NSUB:4 * _NSUB]]
        bufd = [scr[4 * _NSUB:5 * _NSUB], scr[5 * _NSUB:6 * _NSUB]]
        bufs = [scr[6 * _NSUB:7 * _NSUB], scr[7 * _NSUB:8 * _NSUB]]
        semi = [scr[8 * _NSUB], scr[8 * _NSUB + 1]]
        semg = [scr[8 * _NSUB + 2], scr[8 * _NSUB + 3]]
        semo = [scr[8 * _NSUB + 4], scr[8 * _NSUB + 5]]
        wid = lax.axis_index("s") * 2 + lax.axis_index("c")
        base = wid * _EPW

        def offs_of(k_, j):
            if isinstance(k_, int):
                km = (k_ * cg * _NSUB) % _EPW
            else:
                km = lax.rem(k_ * (cg * _NSUB), _EPW)
            return pl.multiple_of(base + km + j * cg, 8)

        def fire_idx(p, k_):
            for j in range(_NSUB):
                o = offs_of(k_, j)
                pltpu.async_copy(dst_h.at[pl.ds(o, cg)], idxd[p][j], semi[p])
                pltpu.async_copy(src_h.at[pl.ds(o, cg)], idxs[p][j], semi[p])

        def wait_idx(p):
            for j in range(_NSUB):
                pltpu.make_async_copy(dst_h.at[pl.ds(0, cg)], idxd[p][j],
                                      semi[p]).wait()
                pltpu.make_async_copy(src_h.at[pl.ds(0, cg)], idxs[p][j],
                                      semi[p]).wait()

        def drain_out(p):
            for j in range(_NSUB):
                pltpu.make_async_copy(pd_h.at[pl.ds(0, cg)], bufd[p][j],
                                      semo[p]).wait()

        def consume(q, k_):
            for j in range(_NSUB):
                pltpu.make_async_copy(pd_h.at[pl.ds(0, cg)], bufd[q][j],
                                      semg[q]).wait()
                pltpu.make_async_copy(pd_h.at[pl.ds(0, cg)], bufs[q][j],
                                      semg[q]).wait()

            def addrow(r, _):
                for j in range(_NSUB):
                    for g in range(8):
                        sl = pl.ds(g * 16, 16)
                        bufd[q][j][r, sl] = bufd[q][j][r, sl] + bufs[q][j][r, sl]
                return 0

            lax.fori_loop(0, cg, addrow, 0)
            for j in range(_NSUB):
                o = offs_of(k_, j)
                pltpu.async_copy(bufd[q][j], su_o.at[pl.ds(o, cg)], semo[q])

        def step(k_, p, pred):
            wait_idx(p)

            @pl.when(pred)
            def _():
                drain_out(p)
            for j in range(_NSUB):
                pltpu.async_copy(pd_h.at[idxd[p][j]], bufd[p][j], semg[p])
                pltpu.async_copy(ps_h.at[idxs[p][j]], bufs[p][j], semg[p])
            if isinstance(k_, int):
                if k_ >= 1:
                    consume(1 - p, k_ - 1)
            else:
                @pl.when(k_ >= 1)
                def _():
                    consume(1 - p, k_ - 1)
            fire_idx(1 - p, k_ + 1)

        fire_idx(0, 0)

        def body(i, _):
            step(2 * i, 0, i >= 1)
            step(2 * i + 1, 1, i >= 1)
            return 0

        lax.fori_loop(0, nst // 2, body, 0)
        consume(1, nst - 1)
        drain_out(0)
        drain_out(1)
        wait_idx(0)

    return k(pd, ps, src, dst)


_CS = 40
_SSTEP = _CS * _NSUB


def _scatter(m, dst, zeros):
    scratch = (
        [pltpu.VMEM((_CS,), jnp.int32)] * _NSUB
        + [pltpu.VMEM((_CS, _MW), _f32)] * _NSUB
        + [pltpu.VMEM_SHARED((_N, _MW), _f32), pltpu.SemaphoreType.DMA]
    )

    @functools.partial(
        pl.kernel,
        mesh=_MESH,
        out_type=jax.ShapeDtypeStruct((2, _N, _MW), _f32),
        scratch_types=scratch,
    )
    def k(m_h, dst_h, z_h, out_h, *scr):
        idx = scr[0:_NSUB]
        buf = scr[_NSUB:2 * _NSUB]
        table = scr[2 * _NSUB]
        sem = scr[2 * _NSUB + 1]
        cid = lax.axis_index("c")
        sid = lax.axis_index("s")
        row0 = pl.multiple_of(sid * 624, 8)

        @pl.when(sid < 15)
        def _():
            pltpu.sync_copy(z_h.at[pl.ds(row0, 624)],
                            table.at[pl.ds(row0, 624)])

        @pl.when(sid == 15)
        def _():
            pltpu.sync_copy(z_h.at[pl.ds(9360, 640)],
                            table.at[pl.ds(9360, 640)])

        plsc.subcore_barrier()
        base = cid * (_E // 2) + sid * _EPW

        def step(i, _):
            offs = [pl.multiple_of(base + i * _SSTEP + j * _CS, 8)
                    for j in range(_NSUB)]
            cps = []
            for j in range(_NSUB):
                cps.append(pltpu.async_copy(dst_h.at[pl.ds(offs[j], _CS)], idx[j], sem))
                cps.append(pltpu.async_copy(m_h.at[pl.ds(offs[j], _CS)], buf[j], sem))
            for cp in cps:
                cp.wait()
            cps = []
            for j in range(_NSUB):
                cps.append(pltpu.async_copy(buf[j], table.at[idx[j]], sem, add=True))
            for cp in cps:
                cp.wait()
            return 0

        lax.fori_loop(0, _EPW // _SSTEP, step, 0)
        plsc.subcore_barrier()

        @pl.when(sid < 15)
        def _():
            pltpu.sync_copy(table.at[pl.ds(row0, 624)],
                            out_h.at[cid, pl.ds(row0, 624)])

        @pl.when(sid == 15)
        def _():
            pltpu.sync_copy(table.at[pl.ds(9360, 640)],
                            out_h.at[cid, pl.ds(9360, 640)])

    return k(m, dst, zeros)


def _proj(feat, wcat, bcat):
    nb = 2000

    def body(f_ref, w_ref, b_ref, pd_ref, ps_ref):
        p = jnp.dot(f_ref[...], w_ref[...], preferred_element_type=_f32)
        p = p + b_ref[...]
        pd_ref[...] = p[:, :128]
        ps_ref[...] = p[:, 128:]

    return pl.pallas_call(
        body,
        grid=(_N // nb,),
        in_specs=[
            pl.BlockSpec((nb, 128), lambda i: (i, 0)),
            pl.BlockSpec((128, 256), lambda i: (0, 0)),
            pl.BlockSpec((1, 256), lambda i: (0, 0)),
        ],
        out_specs=[pl.BlockSpec((nb, 128), lambda i: (i, 0))] * 2,
        out_shape=[jax.ShapeDtypeStruct((_N, 128), _f32)] * 2,
    )(feat, wcat, bcat)


def _edge_call(su, geo, wg, e2w, e2b, e3w, e3b, v2col, v2b):
    eb = 5000

    def body(su_ref, g_ref, wg_ref, e2w_ref, e2b_ref, e3w_ref,
             e3b_ref, v2_ref, v2b_ref, m_ref):
        g = g_ref[...]
        su = (su_ref[...]
              + jnp.dot(g[:, 2:4], wg_ref[...], preferred_element_type=_f32))
        th = _silu(su[:, :64])
        th = _silu(jnp.dot(th, e2w_ref[...], preferred_element_type=_f32)
                   + e2b_ref[...])
        mh = jnp.dot(th, e3w_ref[...], preferred_element_type=_f32) + e3b_ref[...]
        tv = _silu(su[:, 64:])
        vw = jnp.dot(tv, v2_ref[...], preferred_element_type=_f32) + v2b_ref[...]
        mv = vw * g[:, 0:2]
        m_ref[...] = jnp.concatenate(
            [mh, mv, jnp.zeros((eb, _MW - 66), _f32)], axis=1)

    return pl.pallas_call(
        body,
        grid=(_E // eb,),
        in_specs=[
            pl.BlockSpec((eb, 128), lambda i: (i, 0)),
            pl.BlockSpec((eb, 4), lambda i: (i, 0)),
            pl.BlockSpec((2, 128), lambda i: (0, 0)),
            pl.BlockSpec((64, 64), lambda i: (0, 0)),
            pl.BlockSpec((1, 64), lambda i: (0, 0)),
            pl.BlockSpec((64, 64), lambda i: (0, 0)),
            pl.BlockSpec((1, 64), lambda i: (0, 0)),
            pl.BlockSpec((64, 1), lambda i: (0, 0)),
            pl.BlockSpec((1, 1), lambda i: (0, 0)),
        ],
        out_specs=pl.BlockSpec((eb, _MW), lambda i: (i, 0)),
        out_shape=jax.ShapeDtypeStruct((_E, _MW), _f32),
    )(su, geo, wg, e2w, e2b, e3w, e3b, v2col, v2b)


def _node_proj(feat, msum, w, wcat2, bcat2):
    nb = 2000

    def body(f_ref, ms_ref, wx_ref, wm_ref, wn_ref, h1b_ref, h2w_ref,
             h2b_ref, g_ref, b_ref, wc_ref, bc_ref, o_ref, pd_ref, ps_ref):
        f = f_ref[...]
        m = ms_ref[0] + ms_ref[1]
        mvx = m[:, 64:65]
        mvy = m[:, 65:66]
        mvn = jnp.sqrt(mvx * mvx + mvy * mvy + 1e-12)
        hh = (jnp.dot(f, wx_ref[...], preferred_element_type=_f32)
              + jnp.dot(m, wm_ref[...], preferred_element_type=_f32)
              + mvn * wn_ref[...] + h1b_ref[...])
        hh = _silu(hh)
        up = jnp.dot(hh, h2w_ref[...], preferred_element_type=_f32) + h2b_ref[...]
        y = jnp.maximum(f + up, 0.0)
        mu = jnp.mean(y, axis=1, keepdims=True)
        yc = y - mu
        var = jnp.mean(yc * yc, axis=1, keepdims=True)
        h = yc * jax.lax.rsqrt(var + 1e-5) * g_ref[...] + b_ref[...]
        o_ref[...] = h
        p = jnp.dot(h, wc_ref[...], preferred_element_type=_f32) + bc_ref[...]
        pd_ref[...] = p[:, :128]
        ps_ref[...] = p[:, 128:]

    return pl.pallas_call(
        body,
        grid=(_N // nb,),
        in_specs=[
            pl.BlockSpec((nb, 128), lambda i: (i, 0)),
            pl.BlockSpec((2, nb, _MW), lambda i: (0, i, 0)),
            pl.BlockSpec((128, 64), lambda i: (0, 0)),
            pl.BlockSpec((_MW, 64), lambda i: (0, 0)),
            pl.BlockSpec((1, 64), lambda i: (0, 0)),
            pl.BlockSpec((1, 64), lambda i: (0, 0)),
            pl.BlockSpec((64, 128), lambda i: (0, 0)),
            pl.BlockSpec((1, 128), lambda i: (0, 0)),
            pl.BlockSpec((1, 128), lambda i: (0, 0)),
            pl.BlockSpec((1, 128), lambda i: (0, 0)),
            pl.BlockSpec((128, 256), lambda i: (0, 0)),
            pl.BlockSpec((1, 256), lambda i: (0, 0)),
        ],
        out_specs=[
            pl.BlockSpec((nb, 128), lambda i: (i, 0)),
            pl.BlockSpec((nb, 128), lambda i: (i, 0)),
            pl.BlockSpec((nb, 128), lambda i: (i, 0)),
        ],
        out_shape=[jax.ShapeDtypeStruct((_N, 128), _f32)] * 3,
    )(feat, msum, w['wx'], w['wm'], w['wn'], w['h1b'], w['h2w'], w['h2b'],
      w['g'], w['b'], wcat2, bcat2)


def _node_pool(feat, msum, w, bcol, pos, poolw, poolb, o1w, o1b, o2w, o2b,
               gain):
    nb = 2000
    nsteps = _N // nb

    def body(f_ref, ms_ref, wx_ref, wm_ref, wn_ref, h1b_ref, h2w_ref,
             h2b_ref, g_ref, b_ref, bcol_ref, p_ref, pw_ref, pb_ref,
             o1w_ref, o1b_ref, o2w_ref, o2b_ref, gn_ref,
             s_ref, lat_ref, mu_ref, loss_ref, acc_ref):
        f = f_ref[...]
        m = ms_ref[0] + ms_ref[1]
        mvx = m[:, 64:65]
        mvy = m[:, 65:66]
        mvn = jnp.sqrt(mvx * mvx + mvy * mvy + 1e-12)
        hh = (jnp.dot(f, wx_ref[...], preferred_element_type=_f32)
              + jnp.dot(m, wm_ref[...], preferred_element_type=_f32)
              + mvn * wn_ref[...] + h1b_ref[...])
        hh = _silu(hh)
        up = jnp.dot(hh, h2w_ref[...], preferred_element_type=_f32) + h2b_ref[...]
        y = jnp.maximum(f + up, 0.0)
        mu_ = jnp.mean(y, axis=1, keepdims=True)
        yc = y - mu_
        var = jnp.mean(yc * yc, axis=1, keepdims=True)
        hv = yc * jax.lax.rsqrt(var + 1e-5) * g_ref[...] + b_ref[...]

        logits = jnp.dot(hv, pw_ref[...], preferred_element_type=_f32) + pb_ref[...]
        mx = jnp.max(logits, axis=1, keepdims=True)
        ex = jnp.exp(logits - mx)
        s = ex / jnp.sum(ex, axis=1, keepdims=True)
        s_ref[...] = s
        bc = bcol_ref[...]
        lane = lax.broadcasted_iota(jnp.int32, (nb, 256), 1) // _K
        stile = jnp.concatenate([s] * _B, axis=1)
        wm_ = jnp.where(lane == bc, stile, 0.0)
        entcol = jnp.sum(s * jnp.log(s + 1e-8), axis=1, keepdims=True)
        ones = jnp.ones((nb, 1), _f32)
        w_ext = jnp.concatenate(
            [wm_, s, ones, jnp.zeros((nb, 7), _f32)], axis=1)
        r_ext = jnp.concatenate(
            [hv, p_ref[...], ones, entcol, jnp.zeros((nb, 4), _f32)],
            axis=1)
        acc = lax.dot_general(w_ext, r_ext, (((0,), (0,)), ((), ())),
                              preferred_element_type=_f32)

        @pl.when(pl.program_id(0) == 0)
        def _():
            acc_ref[...] = acc

        @pl.when(pl.program_id(0) != 0)
        def _():
            acc_ref[...] += acc

        @pl.when(pl.program_id(0) == nsteps - 1)
        def _():
            a = acc_ref[...]
            den = a[:256, 130:131] + 1e-8
            pooled = a[:256, :128] / den
            z = jnp.maximum(
                jnp.dot(pooled, o1w_ref[...], preferred_element_type=_f32)
                + o1b_ref[...], 0.0)
            lat_ref[...] = (jnp.dot(z, o2w_ref[...],
                                    preferred_element_type=_f32)
                            + o2b_ref[...]) * gn_ref[...]
            mu_ref[...] = a[:256, 128:130] / den
            usage = a[256:288, 130:131] * (1.0 / _N)
            lb = jnp.sum(usage * jnp.log(usage * _K + 1e-8), axis=0,
                         keepdims=True)
            ent = -a[288:289, 131:132] * (1.0 / _N)
            loss_ref[...] = ent + lb

    return pl.pallas_call(
        body,
        grid=(nsteps,),
        in_specs=[
            pl.BlockSpec((nb, 128), lambda i: (i, 0)),
            pl.BlockSpec((2, nb, _MW), lambda i: (0, i, 0)),
            pl.BlockSpec((128, 64), lambda i: (0, 0)),
            pl.BlockSpec((_MW, 64), lambda i: (0, 0)),
            pl.BlockSpec((1, 64), lambda i: (0, 0)),
            pl.BlockSpec((1, 64), lambda i: (0, 0)),
            pl.BlockSpec((64, 128), lambda i: (0, 0)),
            pl.BlockSpec((1, 128), lambda i: (0, 0)),
            pl.BlockSpec((1, 128), lambda i: (0, 0)),
            pl.BlockSpec((1, 128), lambda i: (0, 0)),
            pl.BlockSpec((nb, 1), lambda i: (i, 0)),
            pl.BlockSpec((nb, 2), lambda i: (i, 0)),
            pl.BlockSpec((128, _K), lambda i: (0, 0)),
            pl.BlockSpec((1, _K), lambda i: (0, 0)),
            pl.BlockSpec((128, 128), lambda i: (0, 0)),
            pl.BlockSpec((1, 128), lambda i: (0, 0)),
            pl.BlockSpec((128, _LAT), lambda i: (0, 0)),
            pl.BlockSpec((1, _LAT), lambda i: (0, 0)),
            pl.BlockSpec((1, _LAT), lambda i: (0, 0)),
        ],
        out_specs=[
            pl.BlockSpec((nb, _K), lambda i: (i, 0)),
            pl.BlockSpec((256, _LAT), lambda i: (0, 0)),
            pl.BlockSpec((256, 2), lambda i: (0, 0)),
            pl.BlockSpec((1, 1), lambda i: (0, 0)),
        ],
        out_shape=[
            jax.ShapeDtypeStruct((_N, _K), _f32),
            jax.ShapeDtypeStruct((256, _LAT), _f32),
            jax.ShapeDtypeStruct((256, 2), _f32),
            jax.ShapeDtypeStruct((1, 1), _f32),
        ],
        scratch_shapes=[pltpu.VMEM((296, 136), _f32)],
    )(feat, msum, w['wx'], w['wm'], w['wn'], w['h1b'], w['h2w'], w['h2b'],
      w['g'], w['b'], bcol, pos, poolw, poolb, o1w, o1b, o2w, o2b, gain)


def _layer_weights(p):
    e1w, e1b = p['e1']
    v1w, v1b = p['v1']
    wcat = jnp.concatenate(
        [e1w[:_F], v1w[:_F], e1w[_F:2 * _F], v1w[_F:2 * _F]], axis=1)
    bcat = jnp.concatenate(
        [e1b, v1b, jnp.zeros((2 * _HID,), _f32)]).reshape(1, 256)
    wg = jnp.concatenate([e1w[2 * _F:], v1w[2 * _F:]], axis=1)
    h1w, h1b = p['h1']
    wx = h1w[:_F]
    wm = jnp.concatenate([h1w[_F:_F + 64], jnp.zeros((_MW - 64, 64), _f32)],
                         axis=0)
    wn = h1w[_F + 64].reshape(1, 64)
    return dict(
        wcat=wcat, bcat=bcat, wg=wg,
        e2w=p['e2'][0], e2b=p['e2'][1].reshape(1, 64),
        e3w=p['e3'][0], e3b=p['e3'][1].reshape(1, 64),
        v2col=p['v2'][0], v2b=p['v2'][1].reshape(1, 1),
        wx=wx, wm=wm, wn=wn, h1b=h1b.reshape(1, 64),
        h2w=p['h2'][0], h2b=p['h2'][1].reshape(1, 128),
    )


def kernel(x, edge_index, batch, p1, p2, ln1, ln2, pool, out1, out2,
           latent_gain):
    src = edge_index[0]
    dst = edge_index[1]
    pos = x[:, :2]
    zeros_tab = jnp.zeros((_N, _MW), _f32)
    bcol = batch.reshape(_N, 1)

    w1 = _layer_weights(p1)
    w1['g'] = ln1[0].reshape(1, 128)
    w1['b'] = ln1[1].reshape(1, 128)
    w2 = _layer_weights(p2)
    w2['g'] = ln2[0].reshape(1, 128)
    w2['b'] = ln2[1].reshape(1, 128)

    rx, ry, dd, dt = _geo(x[:, 0], x[:, 1], x[:, 2], x[:, 3], src, dst)
    pd, ps = _proj(x, w1['wcat'], w1['bcat'])
    su1 = _gather(pd, ps, src, dst)
    geo4 = jnp.concatenate(
        [rx.reshape(_E, 1), ry.reshape(_E, 1), dd.reshape(_E, 1),
         dt.reshape(_E, 1)], axis=1)
    m1 = _edge_call(su1, geo4, w1['wg'], w1['e2w'], w1['e2b'],
                    w1['e3w'], w1['e3b'], w1['v2col'], w1['v2b'])
    msum1 = _scatter(m1, dst, zeros_tab)
    h1, pd2, ps2 = _node_proj(x, msum1, w1, w2['wcat'], w2['bcat'])

    su2 = _gather(pd2, ps2, src, dst)
    m2 = _edge_call(su2, geo4, w2['wg'], w2['e2w'], w2['e2b'],
                    w2['e3w'], w2['e3b'], w2['v2col'], w2['v2b'])
    msum2 = _scatter(m2, dst, zeros_tab)
    s, lat, mu, loss = _node_pool(
        h1, msum2, w2, bcol, pos, pool[0], pool[1].reshape(1, _K), out1[0],
        out1[1].reshape(1, 128), out2[0], out2[1].reshape(1, _LAT),
        latent_gain.reshape(1, _LAT))
    return (lat.reshape(_B, _K, _LAT), s, loss[0, 0],
            mu.reshape(_B, _K, 2))


# speedup vs baseline: 1.7831x; 1.0603x over previous
import functools

import jax
import jax.numpy as jnp
from jax import lax
from jax.experimental import pallas as pl
from jax.experimental.pallas import tpu as pltpu
from jax.experimental.pallas import tpu_sc as plsc

_N = 10000
_E = 320000
_F = 128
_HID = 64
_OUT = 128
_K = 32
_LAT = 64
_B = 8

_NW = 32
_EPW = _E // _NW
_C = 80
_NSUB = 5
_STEP = _C * _NSUB
_NITER = _EPW // _STEP
_ROWS = _N // 16
_MW = 128

_f32 = jnp.float32

_MESH = plsc.VectorSubcoreMesh(core_axis_name="c", subcore_axis_name="s")


def _silu(x):
    return x / (1.0 + jnp.exp(-x))


def _geo(px, py, vx, vy, src, dst):
    scratch = (
        [pltpu.VMEM((_C,), jnp.int32)] * (2 * _NSUB)
        + [pltpu.VMEM((_C,), _f32)] * (8 * _NSUB)
        + [pltpu.VMEM((_C,), _f32)] * (4 * _NSUB)
        + [pltpu.SemaphoreType.DMA]
    )

    @functools.partial(
        pl.kernel,
        mesh=_MESH,
        out_type=[jax.ShapeDtypeStruct((_E,), _f32)] * 4,
        scratch_types=scratch,
    )
    def k(px_h, py_h, vx_h, vy_h, src_h, dst_h, rx_o, ry_o, dd_o, dt_o, *scr):
        idxd = scr[0:_NSUB]
        idxs = scr[_NSUB:2 * _NSUB]
        gb = scr[2 * _NSUB:10 * _NSUB]
        ob = scr[10 * _NSUB:14 * _NSUB]
        sem = scr[14 * _NSUB]
        wid = lax.axis_index("s") * 2 + lax.axis_index("c")
        base = wid * _EPW
        tabs = (px_h, py_h, vx_h, vy_h)
        gouts = (rx_o, ry_o, dd_o, dt_o)

        def step(i, _):
            offs = [pl.multiple_of(base + i * _STEP + j * _C, 8)
                    for j in range(_NSUB)]
            cps = []
            for j in range(_NSUB):
                cps.append(pltpu.async_copy(dst_h.at[pl.ds(offs[j], _C)], idxd[j], sem))
                cps.append(pltpu.async_copy(src_h.at[pl.ds(offs[j], _C)], idxs[j], sem))
            for cp in cps:
                cp.wait()
            cps = []
            for j in range(_NSUB):
                for t in range(4):
                    cps.append(pltpu.async_copy(
                        tabs[t].at[idxd[j]], gb[8 * j + t], sem))
                    cps.append(pltpu.async_copy(
                        tabs[t].at[idxs[j]], gb[8 * j + 4 + t], sem))
            for cp in cps:
                cp.wait()
            for j in range(_NSUB):
                for g in range(_C // 16):
                    sl = pl.ds(pl.multiple_of(g * 16, 8), 16)
                    rx = gb[8 * j + 4][sl] - gb[8 * j + 0][sl]
                    ry = gb[8 * j + 5][sl] - gb[8 * j + 1][sl]
                    wx = gb[8 * j + 6][sl] - gb[8 * j + 2][sl]
                    wy = gb[8 * j + 7][sl] - gb[8 * j + 3][sl]
                    ob[4 * j + 0][sl] = rx
                    ob[4 * j + 1][sl] = ry
                    ob[4 * j + 2][sl] = rx * rx + ry * ry
                    ob[4 * j + 3][sl] = wx * rx + wy * ry
            cps = []
            for j in range(_NSUB):
                for t in range(4):
                    cps.append(pltpu.async_copy(
                        ob[4 * j + t], gouts[t].at[pl.ds(offs[j], _C)], sem))
            for cp in cps:
                cp.wait()
            return 0

        lax.fori_loop(0, _NITER, step, 0)

    return k(px, py, vx, vy, src, dst)


def _gather(pd, ps, src, dst, geo_tabs=None):
    del geo_tabs
    cg = 40
    nst = _EPW // (cg * _NSUB)
    scratch = (
        [pltpu.VMEM((cg,), jnp.int32)] * (4 * _NSUB)
        + [pltpu.VMEM((cg, 128), _f32)] * (4 * _NSUB)
        + [pltpu.SemaphoreType.DMA] * 6
    )

    @functools.partial(
        pl.kernel,
        mesh=_MESH,
        out_type=jax.ShapeDtypeStruct((_E, 128), _f32),
        scratch_types=scratch,
    )
    def k(pd_h, ps_h, src_h, dst_h, su_o, *scr):
        idxd = [scr[0:_NSUB], scr[_NSUB:2 * _NSUB]]
        idxs = [scr[2 * _NSUB:3 * _NSUB], scr[3 * _NSUB:4 * _NSUB]]
        bufd = [scr[4 * _NSUB:5 * _NSUB], scr[5 * _NSUB:6 * _NSUB]]
        bufs = [scr[6 * _NSUB:7 * _NSUB], scr[7 * _NSUB:8 * _NSUB]]
        semi = [scr[8 * _NSUB], scr[8 * _NSUB + 1]]
        semg = [scr[8 * _NSUB + 2], scr[8 * _NSUB + 3]]
        semo = [scr[8 * _NSUB + 4], scr[8 * _NSUB + 5]]
        wid = lax.axis_index("s") * 2 + lax.axis_index("c")
        base = wid * _EPW

        def offs_of(k_, j):
            if isinstance(k_, int):
                km = (k_ * cg * _NSUB) % _EPW
            else:
                km = lax.rem(k_ * (cg * _NSUB), _EPW)
            return pl.multiple_of(base + km + j * cg, 8)

        def fire_idx(p, k_):
            for j in range(_NSUB):
                o = offs_of(k_, j)
                pltpu.async_copy(dst_h.at[pl.ds(o, cg)], idxd[p][j], semi[p])
                pltpu.async_copy(src_h.at[pl.ds(o, cg)], idxs[p][j], semi[p])

        def wait_idx(p):
            for j in range(_NSUB):
                pltpu.make_async_copy(dst_h.at[pl.ds(0, cg)], idxd[p][j],
                                      semi[p]).wait()
                pltpu.make_async_copy(src_h.at[pl.ds(0, cg)], idxs[p][j],
                                      semi[p]).wait()

        def drain_out(p):
            for j in range(_NSUB):
                pltpu.make_async_copy(pd_h.at[pl.ds(0, cg)], bufd[p][j],
                                      semo[p]).wait()

        def consume(q, k_):
            for j in range(_NSUB):
                pltpu.make_async_copy(pd_h.at[pl.ds(0, cg)], bufd[q][j],
                                      semg[q]).wait()
                pltpu.make_async_copy(pd_h.at[pl.ds(0, cg)], bufs[q][j],
                                      semg[q]).wait()

            def addrow(r, _):
                for j in range(_NSUB):
                    for g in range(8):
                        sl = pl.ds(g * 16, 16)
                        bufd[q][j][r, sl] = bufd[q][j][r, sl] + bufs[q][j][r, sl]
                return 0

            lax.fori_loop(0, cg, addrow, 0)
            for j in range(_NSUB):
                o = offs_of(k_, j)
                pltpu.async_copy(bufd[q][j], su_o.at[pl.ds(o, cg)], semo[q])

        def step(k_, p, pred):
            wait_idx(p)

            @pl.when(pred)
            def _():
                drain_out(p)
            for j in range(_NSUB):
                pltpu.async_copy(pd_h.at[idxd[p][j]], bufd[p][j], semg[p])
                pltpu.async_copy(ps_h.at[idxs[p][j]], bufs[p][j], semg[p])
            if isinstance(k_, int):
                if k_ >= 1:
                    consume(1 - p, k_ - 1)
            else:
                @pl.when(k_ >= 1)
                def _():
                    consume(1 - p, k_ - 1)
            fire_idx(1 - p, k_ + 1)

        fire_idx(0, 0)

        def body(i, _):
            step(2 * i, 0, i >= 1)
            step(2 * i + 1, 1, i >= 1)
            return 0

        lax.fori_loop(0, nst // 2, body, 0)
        consume(1, nst - 1)
        drain_out(0)
        drain_out(1)
        wait_idx(0)

    return k(pd, ps, src, dst)


_CS = 40
_SSTEP = _CS * _NSUB


def _scatter(m, dst, zeros):
    scratch = (
        [pltpu.VMEM((_CS,), jnp.int32)] * _NSUB
        + [pltpu.VMEM((_CS, _MW), _f32)] * _NSUB
        + [pltpu.VMEM_SHARED((_N, _MW), _f32), pltpu.SemaphoreType.DMA]
        + [pltpu.SemaphoreType.DMA] * (2 * _NSUB)
    )

    @functools.partial(
        pl.kernel,
        mesh=_MESH,
        out_type=jax.ShapeDtypeStruct((2, _N, _MW), _f32),
        scratch_types=scratch,
    )
    def k(m_h, dst_h, z_h, out_h, *scr):
        idx = scr[0:_NSUB]
        buf = scr[_NSUB:2 * _NSUB]
        table = scr[2 * _NSUB]
        sem = scr[2 * _NSUB + 1]
        seml = scr[2 * _NSUB + 2:2 * _NSUB + 2 + _NSUB]
        sema = scr[2 * _NSUB + 2 + _NSUB:2 * _NSUB + 2 + 2 * _NSUB]
        cid = lax.axis_index("c")
        sid = lax.axis_index("s")
        row0 = pl.multiple_of(sid * 624, 8)

        @pl.when(sid < 15)
        def _():
            pltpu.sync_copy(z_h.at[pl.ds(row0, 624)],
                            table.at[pl.ds(row0, 624)])

        @pl.when(sid == 15)
        def _():
            pltpu.sync_copy(z_h.at[pl.ds(9360, 640)],
                            table.at[pl.ds(9360, 640)])

        plsc.subcore_barrier()
        base = cid * (_E // 2) + sid * _EPW

        def step(i, _):
            offs = [pl.multiple_of(base + i * _SSTEP + j * _CS, 8)
                    for j in range(_NSUB)]
            cps = []
            for j in range(_NSUB):
                @pl.when(i > 0)
                def _(j=j):
                    pltpu.make_async_copy(m_h.at[pl.ds(0, _CS)], buf[j],
                                          sema[j]).wait()
                cps.append(pltpu.async_copy(dst_h.at[pl.ds(offs[j], _CS)],
                                            idx[j], seml[j]))
                cps.append(pltpu.async_copy(m_h.at[pl.ds(offs[j], _CS)],
                                            buf[j], seml[j]))
            for j in range(_NSUB):
                cps[2 * j].wait()
                cps[2 * j + 1].wait()
                pltpu.async_copy(buf[j], table.at[idx[j]], sema[j], add=True)
            return 0

        lax.fori_loop(0, _EPW // _SSTEP, step, 0)
        for j in range(_NSUB):
            pltpu.make_async_copy(m_h.at[pl.ds(0, _CS)], buf[j],
                                  sema[j]).wait()
        plsc.subcore_barrier()

        @pl.when(sid < 15)
        def _():
            pltpu.sync_copy(table.at[pl.ds(row0, 624)],
                            out_h.at[cid, pl.ds(row0, 624)])

        @pl.when(sid == 15)
        def _():
            pltpu.sync_copy(table.at[pl.ds(9360, 640)],
                            out_h.at[cid, pl.ds(9360, 640)])

    return k(m, dst, zeros)


def _proj(feat, wcat, bcat):
    nb = 2000

    def body(f_ref, w_ref, b_ref, pd_ref, ps_ref):
        p = jnp.dot(f_ref[...], w_ref[...], preferred_element_type=_f32)
        p = p + b_ref[...]
        pd_ref[...] = p[:, :128]
        ps_ref[...] = p[:, 128:]

    return pl.pallas_call(
        body,
        grid=(_N // nb,),
        in_specs=[
            pl.BlockSpec((nb, 128), lambda i: (i, 0)),
            pl.BlockSpec((128, 256), lambda i: (0, 0)),
            pl.BlockSpec((1, 256), lambda i: (0, 0)),
        ],
        out_specs=[pl.BlockSpec((nb, 128), lambda i: (i, 0))] * 2,
        out_shape=[jax.ShapeDtypeStruct((_N, 128), _f32)] * 2,
    )(feat, wcat, bcat)


def _edge_call(su, geo, wg, e2w, e2b, e3w, e3b, v2col, v2b):
    eb = 5000

    def body(su_ref, g_ref, wg_ref, e2w_ref, e2b_ref, e3w_ref,
             e3b_ref, v2_ref, v2b_ref, m_ref):
        g = g_ref[...]
        su = (su_ref[...]
              + jnp.dot(g[:, 2:4], wg_ref[...], preferred_element_type=_f32))
        th = _silu(su[:, :64])
        th = _silu(jnp.dot(th, e2w_ref[...], preferred_element_type=_f32)
                   + e2b_ref[...])
        mh = jnp.dot(th, e3w_ref[...], preferred_element_type=_f32) + e3b_ref[...]
        tv = _silu(su[:, 64:])
        vw = jnp.dot(tv, v2_ref[...], preferred_element_type=_f32) + v2b_ref[...]
        mv = vw * g[:, 0:2]
        m_ref[...] = jnp.concatenate(
            [mh, mv, jnp.zeros((eb, _MW - 66), _f32)], axis=1)

    return pl.pallas_call(
        body,
        grid=(_E // eb,),
        in_specs=[
            pl.BlockSpec((eb, 128), lambda i: (i, 0)),
            pl.BlockSpec((eb, 4), lambda i: (i, 0)),
            pl.BlockSpec((2, 128), lambda i: (0, 0)),
            pl.BlockSpec((64, 64), lambda i: (0, 0)),
            pl.BlockSpec((1, 64), lambda i: (0, 0)),
            pl.BlockSpec((64, 64), lambda i: (0, 0)),
            pl.BlockSpec((1, 64), lambda i: (0, 0)),
            pl.BlockSpec((64, 1), lambda i: (0, 0)),
            pl.BlockSpec((1, 1), lambda i: (0, 0)),
        ],
        out_specs=pl.BlockSpec((eb, _MW), lambda i: (i, 0)),
        out_shape=jax.ShapeDtypeStruct((_E, _MW), _f32),
    )(su, geo, wg, e2w, e2b, e3w, e3b, v2col, v2b)


def _node_proj(feat, msum, w, wcat2, bcat2):
    nb = 2000

    def body(f_ref, ms_ref, wx_ref, wm_ref, wn_ref, h1b_ref, h2w_ref,
             h2b_ref, g_ref, b_ref, wc_ref, bc_ref, o_ref, pd_ref, ps_ref):
        f = f_ref[...]
        m = ms_ref[0] + ms_ref[1]
        mvx = m[:, 64:65]
        mvy = m[:, 65:66]
        mvn = jnp.sqrt(mvx * mvx + mvy * mvy + 1e-12)
        hh = (jnp.dot(f, wx_ref[...], preferred_element_type=_f32)
              + jnp.dot(m, wm_ref[...], preferred_element_type=_f32)
              + mvn * wn_ref[...] + h1b_ref[...])
        hh = _silu(hh)
        up = jnp.dot(hh, h2w_ref[...], preferred_element_type=_f32) + h2b_ref[...]
        y = jnp.maximum(f + up, 0.0)
        mu = jnp.mean(y, axis=1, keepdims=True)
        yc = y - mu
        var = jnp.mean(yc * yc, axis=1, keepdims=True)
        h = yc * jax.lax.rsqrt(var + 1e-5) * g_ref[...] + b_ref[...]
        o_ref[...] = h
        p = jnp.dot(h, wc_ref[...], preferred_element_type=_f32) + bc_ref[...]
        pd_ref[...] = p[:, :128]
        ps_ref[...] = p[:, 128:]

    return pl.pallas_call(
        body,
        grid=(_N // nb,),
        in_specs=[
            pl.BlockSpec((nb, 128), lambda i: (i, 0)),
            pl.BlockSpec((2, nb, _MW), lambda i: (0, i, 0)),
            pl.BlockSpec((128, 64), lambda i: (0, 0)),
            pl.BlockSpec((_MW, 64), lambda i: (0, 0)),
            pl.BlockSpec((1, 64), lambda i: (0, 0)),
            pl.BlockSpec((1, 64), lambda i: (0, 0)),
            pl.BlockSpec((64, 128), lambda i: (0, 0)),
            pl.BlockSpec((1, 128), lambda i: (0, 0)),
            pl.BlockSpec((1, 128), lambda i: (0, 0)),
            pl.BlockSpec((1, 128), lambda i: (0, 0)),
            pl.BlockSpec((128, 256), lambda i: (0, 0)),
            pl.BlockSpec((1, 256), lambda i: (0, 0)),
        ],
        out_specs=[
            pl.BlockSpec((nb, 128), lambda i: (i, 0)),
            pl.BlockSpec((nb, 128), lambda i: (i, 0)),
            pl.BlockSpec((nb, 128), lambda i: (i, 0)),
        ],
        out_shape=[jax.ShapeDtypeStruct((_N, 128), _f32)] * 3,
    )(feat, msum, w['wx'], w['wm'], w['wn'], w['h1b'], w['h2w'], w['h2b'],
      w['g'], w['b'], wcat2, bcat2)


def _node_pool(feat, msum, w, bcol, pos, poolw, poolb, o1w, o1b, o2w, o2b,
               gain):
    nb = 2000
    nsteps = _N // nb

    def body(f_ref, ms_ref, wx_ref, wm_ref, wn_ref, h1b_ref, h2w_ref,
             h2b_ref, g_ref, b_ref, bcol_ref, p_ref, pw_ref, pb_ref,
             o1w_ref, o1b_ref, o2w_ref, o2b_ref, gn_ref,
             s_ref, lat_ref, mu_ref, loss_ref, acc_ref):
        f = f_ref[...]
        m = ms_ref[0] + ms_ref[1]
        mvx = m[:, 64:65]
        mvy = m[:, 65:66]
        mvn = jnp.sqrt(mvx * mvx + mvy * mvy + 1e-12)
        hh = (jnp.dot(f, wx_ref[...], preferred_element_type=_f32)
              + jnp.dot(m, wm_ref[...], preferred_element_type=_f32)
              + mvn * wn_ref[...] + h1b_ref[...])
        hh = _silu(hh)
        up = jnp.dot(hh, h2w_ref[...], preferred_element_type=_f32) + h2b_ref[...]
        y = jnp.maximum(f + up, 0.0)
        mu_ = jnp.mean(y, axis=1, keepdims=True)
        yc = y - mu_
        var = jnp.mean(yc * yc, axis=1, keepdims=True)
        hv = yc * jax.lax.rsqrt(var + 1e-5) * g_ref[...] + b_ref[...]

        logits = jnp.dot(hv, pw_ref[...], preferred_element_type=_f32) + pb_ref[...]
        mx = jnp.max(logits, axis=1, keepdims=True)
        ex = jnp.exp(logits - mx)
        s = ex / jnp.sum(ex, axis=1, keepdims=True)
        s_ref[...] = s
        bc = bcol_ref[...]
        lane = lax.broadcasted_iota(jnp.int32, (nb, 256), 1) // _K
        stile = jnp.concatenate([s] * _B, axis=1)
        wm_ = jnp.where(lane == bc, stile, 0.0)
        entcol = jnp.sum(s * jnp.log(s + 1e-8), axis=1, keepdims=True)
        ones = jnp.ones((nb, 1), _f32)
        w_ext = jnp.concatenate(
            [wm_, s, ones, jnp.zeros((nb, 7), _f32)], axis=1)
        r_ext = jnp.concatenate(
            [hv, p_ref[...], ones, entcol, jnp.zeros((nb, 4), _f32)],
            axis=1)
        acc = lax.dot_general(w_ext, r_ext, (((0,), (0,)), ((), ())),
                              preferred_element_type=_f32)

        @pl.when(pl.program_id(0) == 0)
        def _():
            acc_ref[...] = acc

        @pl.when(pl.program_id(0) != 0)
        def _():
            acc_ref[...] += acc

        @pl.when(pl.program_id(0) == nsteps - 1)
        def _():
            a = acc_ref[...]
            den = a[:256, 130:131] + 1e-8
            pooled = a[:256, :128] / den
            z = jnp.maximum(
                jnp.dot(pooled, o1w_ref[...], preferred_element_type=_f32)
                + o1b_ref[...], 0.0)
            lat_ref[...] = (jnp.dot(z, o2w_ref[...],
                                    preferred_element_type=_f32)
                            + o2b_ref[...]) * gn_ref[...]
            mu_ref[...] = a[:256, 128:130] / den
            usage = a[256:288, 130:131] * (1.0 / _N)
            lb = jnp.sum(usage * jnp.log(usage * _K + 1e-8), axis=0,
                         keepdims=True)
            ent = -a[288:289, 131:132] * (1.0 / _N)
            loss_ref[...] = ent + lb

    return pl.pallas_call(
        body,
        grid=(nsteps,),
        in_specs=[
            pl.BlockSpec((nb, 128), lambda i: (i, 0)),
            pl.BlockSpec((2, nb, _MW), lambda i: (0, i, 0)),
            pl.BlockSpec((128, 64), lambda i: (0, 0)),
            pl.BlockSpec((_MW, 64), lambda i: (0, 0)),
            pl.BlockSpec((1, 64), lambda i: (0, 0)),
            pl.BlockSpec((1, 64), lambda i: (0, 0)),
            pl.BlockSpec((64, 128), lambda i: (0, 0)),
            pl.BlockSpec((1, 128), lambda i: (0, 0)),
            pl.BlockSpec((1, 128), lambda i: (0, 0)),
            pl.BlockSpec((1, 128), lambda i: (0, 0)),
            pl.BlockSpec((nb, 1), lambda i: (i, 0)),
            pl.BlockSpec((nb, 2), lambda i: (i, 0)),
            pl.BlockSpec((128, _K), lambda i: (0, 0)),
            pl.BlockSpec((1, _K), lambda i: (0, 0)),
            pl.BlockSpec((128, 128), lambda i: (0, 0)),
            pl.BlockSpec((1, 128), lambda i: (0, 0)),
            pl.BlockSpec((128, _LAT), lambda i: (0, 0)),
            pl.BlockSpec((1, _LAT), lambda i: (0, 0)),
            pl.BlockSpec((1, _LAT), lambda i: (0, 0)),
        ],
        out_specs=[
            pl.BlockSpec((nb, _K), lambda i: (i, 0)),
            pl.BlockSpec((256, _LAT), lambda i: (0, 0)),
            pl.BlockSpec((256, 2), lambda i: (0, 0)),
            pl.BlockSpec((1, 1), lambda i: (0, 0)),
        ],
        out_shape=[
            jax.ShapeDtypeStruct((_N, _K), _f32),
            jax.ShapeDtypeStruct((256, _LAT), _f32),
            jax.ShapeDtypeStruct((256, 2), _f32),
            jax.ShapeDtypeStruct((1, 1), _f32),
        ],
        scratch_shapes=[pltpu.VMEM((296, 136), _f32)],
    )(feat, msum, w['wx'], w['wm'], w['wn'], w['h1b'], w['h2w'], w['h2b'],
      w['g'], w['b'], bcol, pos, poolw, poolb, o1w, o1b, o2w, o2b, gain)


def _layer_weights(p):
    e1w, e1b = p['e1']
    v1w, v1b = p['v1']
    wcat = jnp.concatenate(
        [e1w[:_F], v1w[:_F], e1w[_F:2 * _F], v1w[_F:2 * _F]], axis=1)
    bcat = jnp.concatenate(
        [e1b, v1b, jnp.zeros((2 * _HID,), _f32)]).reshape(1, 256)
    wg = jnp.concatenate([e1w[2 * _F:], v1w[2 * _F:]], axis=1)
    h1w, h1b = p['h1']
    wx = h1w[:_F]
    wm = jnp.concatenate([h1w[_F:_F + 64], jnp.zeros((_MW - 64, 64), _f32)],
                         axis=0)
    wn = h1w[_F + 64].reshape(1, 64)
    return dict(
        wcat=wcat, bcat=bcat, wg=wg,
        e2w=p['e2'][0], e2b=p['e2'][1].reshape(1, 64),
        e3w=p['e3'][0], e3b=p['e3'][1].reshape(1, 64),
        v2col=p['v2'][0], v2b=p['v2'][1].reshape(1, 1),
        wx=wx, wm=wm, wn=wn, h1b=h1b.reshape(1, 64),
        h2w=p['h2'][0], h2b=p['h2'][1].reshape(1, 128),
    )


def kernel(x, edge_index, batch, p1, p2, ln1, ln2, pool, out1, out2,
           latent_gain):
    src = edge_index[0]
    dst = edge_index[1]
    pos = x[:, :2]
    zeros_tab = jnp.zeros((_N, _MW), _f32)
    bcol = batch.reshape(_N, 1)

    w1 = _layer_weights(p1)
    w1['g'] = ln1[0].reshape(1, 128)
    w1['b'] = ln1[1].reshape(1, 128)
    w2 = _layer_weights(p2)
    w2['g'] = ln2[0].reshape(1, 128)
    w2['b'] = ln2[1].reshape(1, 128)

    rx, ry, dd, dt = _geo(x[:, 0], x[:, 1], x[:, 2], x[:, 3], src, dst)
    pd, ps = _proj(x, w1['wcat'], w1['bcat'])
    su1 = _gather(pd, ps, src, dst)
    geo4 = jnp.concatenate(
        [rx.reshape(_E, 1), ry.reshape(_E, 1), dd.reshape(_E, 1),
         dt.reshape(_E, 1)], axis=1)
    m1 = _edge_call(su1, geo4, w1['wg'], w1['e2w'], w1['e2b'],
                    w1['e3w'], w1['e3b'], w1['v2col'], w1['v2b'])
    msum1 = _scatter(m1, dst, zeros_tab)
    h1, pd2, ps2 = _node_proj(x, msum1, w1, w2['wcat'], w2['bcat'])

    su2 = _gather(pd2, ps2, src, dst)
    m2 = _edge_call(su2, geo4, w2['wg'], w2['e2w'], w2['e2b'],
                    w2['e3w'], w2['e3b'], w2['v2col'], w2['v2b'])
    msum2 = _scatter(m2, dst, zeros_tab)
    s, lat, mu, loss = _node_pool(
        h1, msum2, w2, bcol, pos, pool[0], pool[1].reshape(1, _K), out1[0],
        out1[1].reshape(1, 128), out2[0], out2[1].reshape(1, _LAT),
        latent_gain.reshape(1, _LAT))
    return (lat.reshape(_B, _K, _LAT), s, loss[0, 0],
            mu.reshape(_B, _K, 2))


# speedup vs baseline: 1.7904x; 1.0041x over previous
import functools

import jax
import jax.numpy as jnp
from jax import lax
from jax.experimental import pallas as pl
from jax.experimental.pallas import tpu as pltpu
from jax.experimental.pallas import tpu_sc as plsc

_N = 10000
_E = 320000
_F = 128
_HID = 64
_OUT = 128
_K = 32
_LAT = 64
_B = 8

_NW = 32
_EPW = _E // _NW
_C = 80
_NSUB = 5
_STEP = _C * _NSUB
_NITER = _EPW // _STEP
_ROWS = _N // 16
_MW = 128

_f32 = jnp.float32

_MESH = plsc.VectorSubcoreMesh(core_axis_name="c", subcore_axis_name="s")


def _silu(x):
    return x / (1.0 + jnp.exp(-x))


def _geo(px, py, vx, vy, src, dst):
    scratch = (
        [pltpu.VMEM((_C,), jnp.int32)] * (2 * _NSUB)
        + [pltpu.VMEM((_C,), _f32)] * (8 * _NSUB)
        + [pltpu.VMEM((_C,), _f32)] * (4 * _NSUB)
        + [pltpu.SemaphoreType.DMA] * (3 * _NSUB)
    )

    @functools.partial(
        pl.kernel,
        mesh=_MESH,
        out_type=[jax.ShapeDtypeStruct((_E,), _f32)] * 4,
        scratch_types=scratch,
    )
    def k(px_h, py_h, vx_h, vy_h, src_h, dst_h, rx_o, ry_o, dd_o, dt_o, *scr):
        idxd = scr[0:_NSUB]
        idxs = scr[_NSUB:2 * _NSUB]
        gb = scr[2 * _NSUB:10 * _NSUB]
        ob = scr[10 * _NSUB:14 * _NSUB]
        semi = scr[14 * _NSUB:15 * _NSUB]
        semg = scr[15 * _NSUB:16 * _NSUB]
        semo = scr[16 * _NSUB:17 * _NSUB]
        wid = lax.axis_index("s") * 2 + lax.axis_index("c")
        base = wid * _EPW
        tabs = (px_h, py_h, vx_h, vy_h)
        gouts = (rx_o, ry_o, dd_o, dt_o)

        def step(i, _):
            offs = [pl.multiple_of(base + i * _STEP + j * _C, 8)
                    for j in range(_NSUB)]
            lps = []
            for j in range(_NSUB):
                @pl.when(i > 0)
                def _(j=j):
                    for t in range(4):
                        pltpu.make_async_copy(rx_o.at[pl.ds(0, _C)],
                                              ob[4 * j + t], semo[j]).wait()
                lps.append(pltpu.async_copy(dst_h.at[pl.ds(offs[j], _C)],
                                            idxd[j], semi[j]))
                lps.append(pltpu.async_copy(src_h.at[pl.ds(offs[j], _C)],
                                            idxs[j], semi[j]))
            gps = []
            for j in range(_NSUB):
                lps[2 * j].wait()
                lps[2 * j + 1].wait()
                for t in range(4):
                    gps.append(pltpu.async_copy(
                        tabs[t].at[idxd[j]], gb[8 * j + t], semg[j]))
                    gps.append(pltpu.async_copy(
                        tabs[t].at[idxs[j]], gb[8 * j + 4 + t], semg[j]))
            for j in range(_NSUB):
                for cp in gps[8 * j:8 * j + 8]:
                    cp.wait()
                for g in range(_C // 16):
                    sl = pl.ds(pl.multiple_of(g * 16, 8), 16)
                    rx = gb[8 * j + 4][sl] - gb[8 * j + 0][sl]
                    ry = gb[8 * j + 5][sl] - gb[8 * j + 1][sl]
                    wx = gb[8 * j + 6][sl] - gb[8 * j + 2][sl]
                    wy = gb[8 * j + 7][sl] - gb[8 * j + 3][sl]
                    ob[4 * j + 0][sl] = rx
                    ob[4 * j + 1][sl] = ry
                    ob[4 * j + 2][sl] = rx * rx + ry * ry
                    ob[4 * j + 3][sl] = wx * rx + wy * ry
                for t in range(4):
                    pltpu.async_copy(ob[4 * j + t],
                                     gouts[t].at[pl.ds(offs[j], _C)], semo[j])
            return 0

        lax.fori_loop(0, _NITER, step, 0)
        for j in range(_NSUB):
            for t in range(4):
                pltpu.make_async_copy(rx_o.at[pl.ds(0, _C)], ob[4 * j + t],
                                      semo[j]).wait()

    return k(px, py, vx, vy, src, dst)


def _gather(pd, ps, src, dst, geo_tabs=None):
    del geo_tabs
    cg = 40
    nst = _EPW // (cg * _NSUB)
    scratch = (
        [pltpu.VMEM((cg,), jnp.int32)] * (4 * _NSUB)
        + [pltpu.VMEM((cg, 128), _f32)] * (4 * _NSUB)
        + [pltpu.SemaphoreType.DMA] * 6
    )

    @functools.partial(
        pl.kernel,
        mesh=_MESH,
        out_type=jax.ShapeDtypeStruct((_E, 128), _f32),
        scratch_types=scratch,
    )
    def k(pd_h, ps_h, src_h, dst_h, su_o, *scr):
        idxd = [scr[0:_NSUB], scr[_NSUB:2 * _NSUB]]
        idxs = [scr[2 * _NSUB:3 * _NSUB], scr[3 * _NSUB:4 * _NSUB]]
        bufd = [scr[4 * _NSUB:5 * _NSUB], scr[5 * _NSUB:6 * _NSUB]]
        bufs = [scr[6 * _NSUB:7 * _NSUB], scr[7 * _NSUB:8 * _NSUB]]
        semi = [scr[8 * _NSUB], scr[8 * _NSUB + 1]]
        semg = [scr[8 * _NSUB + 2], scr[8 * _NSUB + 3]]
        semo = [scr[8 * _NSUB + 4], scr[8 * _NSUB + 5]]
        wid = lax.axis_index("s") * 2 + lax.axis_index("c")
        base = wid * _EPW

        def offs_of(k_, j):
            if isinstance(k_, int):
                km = (k_ * cg * _NSUB) % _EPW
            else:
                km = lax.rem(k_ * (cg * _NSUB), _EPW)
            return pl.multiple_of(base + km + j * cg, 8)

        def fire_idx(p, k_):
            for j in range(_NSUB):
                o = offs_of(k_, j)
                pltpu.async_copy(dst_h.at[pl.ds(o, cg)], idxd[p][j], semi[p])
                pltpu.async_copy(src_h.at[pl.ds(o, cg)], idxs[p][j], semi[p])

        def wait_idx(p):
            for j in range(_NSUB):
                pltpu.make_async_copy(dst_h.at[pl.ds(0, cg)], idxd[p][j],
                                      semi[p]).wait()
                pltpu.make_async_copy(src_h.at[pl.ds(0, cg)], idxs[p][j],
                                      semi[p]).wait()

        def drain_out(p):
            for j in range(_NSUB):
                pltpu.make_async_copy(pd_h.at[pl.ds(0, cg)], bufd[p][j],
                                      semo[p]).wait()

        def consume(q, k_):
            for j in range(_NSUB):
                pltpu.make_async_copy(pd_h.at[pl.ds(0, cg)], bufd[q][j],
                                      semg[q]).wait()
                pltpu.make_async_copy(pd_h.at[pl.ds(0, cg)], bufs[q][j],
                                      semg[q]).wait()

            def addrow(r, _):
                for j in range(_NSUB):
                    for g in range(8):
                        sl = pl.ds(g * 16, 16)
                        bufd[q][j][r, sl] = bufd[q][j][r, sl] + bufs[q][j][r, sl]
                return 0

            lax.fori_loop(0, cg, addrow, 0)
            for j in range(_NSUB):
                o = offs_of(k_, j)
                pltpu.async_copy(bufd[q][j], su_o.at[pl.ds(o, cg)], semo[q])

        def step(k_, p, pred):
            wait_idx(p)

            @pl.when(pred)
            def _():
                drain_out(p)
            for j in range(_NSUB):
                pltpu.async_copy(pd_h.at[idxd[p][j]], bufd[p][j], semg[p])
                pltpu.async_copy(ps_h.at[idxs[p][j]], bufs[p][j], semg[p])
            if isinstance(k_, int):
                if k_ >= 1:
                    consume(1 - p, k_ - 1)
            else:
                @pl.when(k_ >= 1)
                def _():
                    consume(1 - p, k_ - 1)
            fire_idx(1 - p, k_ + 1)

        fire_idx(0, 0)

        def body(i, _):
            step(2 * i, 0, i >= 1)
            step(2 * i + 1, 1, i >= 1)
            return 0

        lax.fori_loop(0, nst // 2, body, 0)
        consume(1, nst - 1)
        drain_out(0)
        drain_out(1)
        wait_idx(0)

    return k(pd, ps, src, dst)


_CS = 40
_SSTEP = _CS * _NSUB


def _scatter(m, dst, zeros):
    scratch = (
        [pltpu.VMEM((_CS,), jnp.int32)] * _NSUB
        + [pltpu.VMEM((_CS, _MW), _f32)] * _NSUB
        + [pltpu.VMEM_SHARED((_N, _MW), _f32), pltpu.SemaphoreType.DMA]
        + [pltpu.SemaphoreType.DMA] * (2 * _NSUB)
    )

    @functools.partial(
        pl.kernel,
        mesh=_MESH,
        out_type=jax.ShapeDtypeStruct((2, _N, _MW), _f32),
        scratch_types=scratch,
    )
    def k(m_h, dst_h, z_h, out_h, *scr):
        idx = scr[0:_NSUB]
        buf = scr[_NSUB:2 * _NSUB]
        table = scr[2 * _NSUB]
        sem = scr[2 * _NSUB + 1]
        seml = scr[2 * _NSUB + 2:2 * _NSUB + 2 + _NSUB]
        sema = scr[2 * _NSUB + 2 + _NSUB:2 * _NSUB + 2 + 2 * _NSUB]
        cid = lax.axis_index("c")
        sid = lax.axis_index("s")
        row0 = pl.multiple_of(sid * 624, 8)

        @pl.when(sid < 15)
        def _():
            pltpu.sync_copy(z_h.at[pl.ds(row0, 624)],
                            table.at[pl.ds(row0, 624)])

        @pl.when(sid == 15)
        def _():
            pltpu.sync_copy(z_h.at[pl.ds(9360, 640)],
                            table.at[pl.ds(9360, 640)])

        plsc.subcore_barrier()
        base = cid * (_E // 2) + sid * _EPW

        def step(i, _):
            offs = [pl.multiple_of(base + i * _SSTEP + j * _CS, 8)
                    for j in range(_NSUB)]
            cps = []
            for j in range(_NSUB):
                @pl.when(i > 0)
                def _(j=j):
                    pltpu.make_async_copy(m_h.at[pl.ds(0, _CS)], buf[j],
                                          sema[j]).wait()
                cps.append(pltpu.async_copy(dst_h.at[pl.ds(offs[j], _CS)],
                                            idx[j], seml[j]))
                cps.append(pltpu.async_copy(m_h.at[pl.ds(offs[j], _CS)],
                                            buf[j], seml[j]))
            for j in range(_NSUB):
                cps[2 * j].wait()
                cps[2 * j + 1].wait()
                pltpu.async_copy(buf[j], table.at[idx[j]], sema[j], add=True)
            return 0

        lax.fori_loop(0, _EPW // _SSTEP, step, 0)
        for j in range(_NSUB):
            pltpu.make_async_copy(m_h.at[pl.ds(0, _CS)], buf[j],
                                  sema[j]).wait()
        plsc.subcore_barrier()

        @pl.when(sid < 15)
        def _():
            pltpu.sync_copy(table.at[pl.ds(row0, 624)],
                            out_h.at[cid, pl.ds(row0, 624)])

        @pl.when(sid == 15)
        def _():
            pltpu.sync_copy(table.at[pl.ds(9360, 640)],
                            out_h.at[cid, pl.ds(9360, 640)])

    return k(m, dst, zeros)


def _proj(feat, wcat, bcat):
    nb = 2000

    def body(f_ref, w_ref, b_ref, pd_ref, ps_ref):
        p = jnp.dot(f_ref[...], w_ref[...], preferred_element_type=_f32)
        p = p + b_ref[...]
        pd_ref[...] = p[:, :128]
        ps_ref[...] = p[:, 128:]

    return pl.pallas_call(
        body,
        grid=(_N // nb,),
        in_specs=[
            pl.BlockSpec((nb, 128), lambda i: (i, 0)),
            pl.BlockSpec((128, 256), lambda i: (0, 0)),
            pl.BlockSpec((1, 256), lambda i: (0, 0)),
        ],
        out_specs=[pl.BlockSpec((nb, 128), lambda i: (i, 0))] * 2,
        out_shape=[jax.ShapeDtypeStruct((_N, 128), _f32)] * 2,
    )(feat, wcat, bcat)


def _edge_call(su, geo, wg, e2w, e2b, e3w, e3b, v2col, v2b):
    eb = 5000

    def body(su_ref, g_ref, wg_ref, e2w_ref, e2b_ref, e3w_ref,
             e3b_ref, v2_ref, v2b_ref, m_ref):
        g = g_ref[...]
        su = (su_ref[...]
              + jnp.dot(g[:, 2:4], wg_ref[...], preferred_element_type=_f32))
        th = _silu(su[:, :64])
        th = _silu(jnp.dot(th, e2w_ref[...], preferred_element_type=_f32)
                   + e2b_ref[...])
        mh = jnp.dot(th, e3w_ref[...], preferred_element_type=_f32) + e3b_ref[...]
        tv = _silu(su[:, 64:])
        vw = jnp.dot(tv, v2_ref[...], preferred_element_type=_f32) + v2b_ref[...]
        mv = vw * g[:, 0:2]
        m_ref[...] = jnp.concatenate(
            [mh, mv, jnp.zeros((eb, _MW - 66), _f32)], axis=1)

    return pl.pallas_call(
        body,
        grid=(_E // eb,),
        in_specs=[
            pl.BlockSpec((eb, 128), lambda i: (i, 0)),
            pl.BlockSpec((eb, 4), lambda i: (i, 0)),
            pl.BlockSpec((2, 128), lambda i: (0, 0)),
            pl.BlockSpec((64, 64), lambda i: (0, 0)),
            pl.BlockSpec((1, 64), lambda i: (0, 0)),
            pl.BlockSpec((64, 64), lambda i: (0, 0)),
            pl.BlockSpec((1, 64), lambda i: (0, 0)),
            pl.BlockSpec((64, 1), lambda i: (0, 0)),
            pl.BlockSpec((1, 1), lambda i: (0, 0)),
        ],
        out_specs=pl.BlockSpec((eb, _MW), lambda i: (i, 0)),
        out_shape=jax.ShapeDtypeStruct((_E, _MW), _f32),
    )(su, geo, wg, e2w, e2b, e3w, e3b, v2col, v2b)


def _node_proj(feat, msum, w, wcat2, bcat2):
    nb = 2000

    def body(f_ref, ms_ref, wx_ref, wm_ref, wn_ref, h1b_ref, h2w_ref,
             h2b_ref, g_ref, b_ref, wc_ref, bc_ref, o_ref, pd_ref, ps_ref):
        f = f_ref[...]
        m = ms_ref[0] + ms_ref[1]
        mvx = m[:, 64:65]
        mvy = m[:, 65:66]
        mvn = jnp.sqrt(mvx * mvx + mvy * mvy + 1e-12)
        hh = (jnp.dot(f, wx_ref[...], preferred_element_type=_f32)
              + jnp.dot(m, wm_ref[...], preferred_element_type=_f32)
              + mvn * wn_ref[...] + h1b_ref[...])
        hh = _silu(hh)
        up = jnp.dot(hh, h2w_ref[...], preferred_element_type=_f32) + h2b_ref[...]
        y = jnp.maximum(f + up, 0.0)
        mu = jnp.mean(y, axis=1, keepdims=True)
        yc = y - mu
        var = jnp.mean(yc * yc, axis=1, keepdims=True)
        h = yc * jax.lax.rsqrt(var + 1e-5) * g_ref[...] + b_ref[...]
        o_ref[...] = h
        p = jnp.dot(h, wc_ref[...], preferred_element_type=_f32) + bc_ref[...]
        pd_ref[...] = p[:, :128]
        ps_ref[...] = p[:, 128:]

    return pl.pallas_call(
        body,
        grid=(_N // nb,),
        in_specs=[
            pl.BlockSpec((nb, 128), lambda i: (i, 0)),
            pl.BlockSpec((2, nb, _MW), lambda i: (0, i, 0)),
            pl.BlockSpec((128, 64), lambda i: (0, 0)),
            pl.BlockSpec((_MW, 64), lambda i: (0, 0)),
            pl.BlockSpec((1, 64), lambda i: (0, 0)),
            pl.BlockSpec((1, 64), lambda i: (0, 0)),
            pl.BlockSpec((64, 128), lambda i: (0, 0)),
            pl.BlockSpec((1, 128), lambda i: (0, 0)),
            pl.BlockSpec((1, 128), lambda i: (0, 0)),
            pl.BlockSpec((1, 128), lambda i: (0, 0)),
            pl.BlockSpec((128, 256), lambda i: (0, 0)),
            pl.BlockSpec((1, 256), lambda i: (0, 0)),
        ],
        out_specs=[
            pl.BlockSpec((nb, 128), lambda i: (i, 0)),
            pl.BlockSpec((nb, 128), lambda i: (i, 0)),
            pl.BlockSpec((nb, 128), lambda i: (i, 0)),
        ],
        out_shape=[jax.ShapeDtypeStruct((_N, 128), _f32)] * 3,
    )(feat, msum, w['wx'], w['wm'], w['wn'], w['h1b'], w['h2w'], w['h2b'],
      w['g'], w['b'], wcat2, bcat2)


def _node_pool(feat, msum, w, bcol, pos, poolw, poolb, o1w, o1b, o2w, o2b,
               gain):
    nb = 2000
    nsteps = _N // nb

    def body(f_ref, ms_ref, wx_ref, wm_ref, wn_ref, h1b_ref, h2w_ref,
             h2b_ref, g_ref, b_ref, bcol_ref, p_ref, pw_ref, pb_ref,
             o1w_ref, o1b_ref, o2w_ref, o2b_ref, gn_ref,
             s_ref, lat_ref, mu_ref, loss_ref, acc_ref):
        f = f_ref[...]
        m = ms_ref[0] + ms_ref[1]
        mvx = m[:, 64:65]
        mvy = m[:, 65:66]
        mvn = jnp.sqrt(mvx * mvx + mvy * mvy + 1e-12)
        hh = (jnp.dot(f, wx_ref[...], preferred_element_type=_f32)
              + jnp.dot(m, wm_ref[...], preferred_element_type=_f32)
              + mvn * wn_ref[...] + h1b_ref[...])
        hh = _silu(hh)
        up = jnp.dot(hh, h2w_ref[...], preferred_element_type=_f32) + h2b_ref[...]
        y = jnp.maximum(f + up, 0.0)
        mu_ = jnp.mean(y, axis=1, keepdims=True)
        yc = y - mu_
        var = jnp.mean(yc * yc, axis=1, keepdims=True)
        hv = yc * jax.lax.rsqrt(var + 1e-5) * g_ref[...] + b_ref[...]

        logits = jnp.dot(hv, pw_ref[...], preferred_element_type=_f32) + pb_ref[...]
        mx = jnp.max(logits, axis=1, keepdims=True)
        ex = jnp.exp(logits - mx)
        s = ex / jnp.sum(ex, axis=1, keepdims=True)
        s_ref[...] = s
        bc = bcol_ref[...]
        lane = lax.broadcasted_iota(jnp.int32, (nb, 256), 1) // _K
        stile = jnp.concatenate([s] * _B, axis=1)
        wm_ = jnp.where(lane == bc, stile, 0.0)
        entcol = jnp.sum(s * jnp.log(s + 1e-8), axis=1, keepdims=True)
        ones = jnp.ones((nb, 1), _f32)
        w_ext = jnp.concatenate(
            [wm_, s, ones, jnp.zeros((nb, 7), _f32)], axis=1)
        r_ext = jnp.concatenate(
            [hv, p_ref[...], ones, entcol, jnp.zeros((nb, 4), _f32)],
            axis=1)
        acc = lax.dot_general(w_ext, r_ext, (((0,), (0,)), ((), ())),
                              preferred_element_type=_f32)

        @pl.when(pl.program_id(0) == 0)
        def _():
            acc_ref[...] = acc

        @pl.when(pl.program_id(0) != 0)
        def _():
            acc_ref[...] += acc

        @pl.when(pl.program_id(0) == nsteps - 1)
        def _():
            a = acc_ref[...]
            den = a[:256, 130:131] + 1e-8
            pooled = a[:256, :128] / den
            z = jnp.maximum(
                jnp.dot(pooled, o1w_ref[...], preferred_element_type=_f32)
                + o1b_ref[...], 0.0)
            lat_ref[...] = (jnp.dot(z, o2w_ref[...],
                                    preferred_element_type=_f32)
                            + o2b_ref[...]) * gn_ref[...]
            mu_ref[...] = a[:256, 128:130] / den
            usage = a[256:288, 130:131] * (1.0 / _N)
            lb = jnp.sum(usage * jnp.log(usage * _K + 1e-8), axis=0,
                         keepdims=True)
            ent = -a[288:289, 131:132] * (1.0 / _N)
            loss_ref[...] = ent + lb

    return pl.pallas_call(
        body,
        grid=(nsteps,),
        in_specs=[
            pl.BlockSpec((nb, 128), lambda i: (i, 0)),
            pl.BlockSpec((2, nb, _MW), lambda i: (0, i, 0)),
            pl.BlockSpec((128, 64), lambda i: (0, 0)),
            pl.BlockSpec((_MW, 64), lambda i: (0, 0)),
            pl.BlockSpec((1, 64), lambda i: (0, 0)),
            pl.BlockSpec((1, 64), lambda i: (0, 0)),
            pl.BlockSpec((64, 128), lambda i: (0, 0)),
            pl.BlockSpec((1, 128), lambda i: (0, 0)),
            pl.BlockSpec((1, 128), lambda i: (0, 0)),
            pl.BlockSpec((1, 128), lambda i: (0, 0)),
            pl.BlockSpec((nb, 1), lambda i: (i, 0)),
            pl.BlockSpec((nb, 2), lambda i: (i, 0)),
            pl.BlockSpec((128, _K), lambda i: (0, 0)),
            pl.BlockSpec((1, _K), lambda i: (0, 0)),
            pl.BlockSpec((128, 128), lambda i: (0, 0)),
            pl.BlockSpec((1, 128), lambda i: (0, 0)),
            pl.BlockSpec((128, _LAT), lambda i: (0, 0)),
            pl.BlockSpec((1, _LAT), lambda i: (0, 0)),
            pl.BlockSpec((1, _LAT), lambda i: (0, 0)),
        ],
        out_specs=[
            pl.BlockSpec((nb, _K), lambda i: (i, 0)),
            pl.BlockSpec((256, _LAT), lambda i: (0, 0)),
            pl.BlockSpec((256, 2), lambda i: (0, 0)),
            pl.BlockSpec((1, 1), lambda i: (0, 0)),
        ],
        out_shape=[
            jax.ShapeDtypeStruct((_N, _K), _f32),
            jax.ShapeDtypeStruct((256, _LAT), _f32),
            jax.ShapeDtypeStruct((256, 2), _f32),
            jax.ShapeDtypeStruct((1, 1), _f32),
        ],
        scratch_shapes=[pltpu.VMEM((296, 136), _f32)],
    )(feat, msum, w['wx'], w['wm'], w['wn'], w['h1b'], w['h2w'], w['h2b'],
      w['g'], w['b'], bcol, pos, poolw, poolb, o1w, o1b, o2w, o2b, gain)


def _layer_weights(p):
    e1w, e1b = p['e1']
    v1w, v1b = p['v1']
    wcat = jnp.concatenate(
        [e1w[:_F], v1w[:_F], e1w[_F:2 * _F], v1w[_F:2 * _F]], axis=1)
    bcat = jnp.concatenate(
        [e1b, v1b, jnp.zeros((2 * _HID,), _f32)]).reshape(1, 256)
    wg = jnp.concatenate([e1w[2 * _F:], v1w[2 * _F:]], axis=1)
    h1w, h1b = p['h1']
    wx = h1w[:_F]
    wm = jnp.concatenate([h1w[_F:_F + 64], jnp.zeros((_MW - 64, 64), _f32)],
                         axis=0)
    wn = h1w[_F + 64].reshape(1, 64)
    return dict(
        wcat=wcat, bcat=bcat, wg=wg,
        e2w=p['e2'][0], e2b=p['e2'][1].reshape(1, 64),
        e3w=p['e3'][0], e3b=p['e3'][1].reshape(1, 64),
        v2col=p['v2'][0], v2b=p['v2'][1].reshape(1, 1),
        wx=wx, wm=wm, wn=wn, h1b=h1b.reshape(1, 64),
        h2w=p['h2'][0], h2b=p['h2'][1].reshape(1, 128),
    )


def kernel(x, edge_index, batch, p1, p2, ln1, ln2, pool, out1, out2,
           latent_gain):
    src = edge_index[0]
    dst = edge_index[1]
    pos = x[:, :2]
    zeros_tab = jnp.zeros((_N, _MW), _f32)
    bcol = batch.reshape(_N, 1)

    w1 = _layer_weights(p1)
    w1['g'] = ln1[0].reshape(1, 128)
    w1['b'] = ln1[1].reshape(1, 128)
    w2 = _layer_weights(p2)
    w2['g'] = ln2[0].reshape(1, 128)
    w2['b'] = ln2[1].reshape(1, 128)

    rx, ry, dd, dt = _geo(x[:, 0], x[:, 1], x[:, 2], x[:, 3], src, dst)
    pd, ps = _proj(x, w1['wcat'], w1['bcat'])
    su1 = _gather(pd, ps, src, dst)
    geo4 = jnp.concatenate(
        [rx.reshape(_E, 1), ry.reshape(_E, 1), dd.reshape(_E, 1),
         dt.reshape(_E, 1)], axis=1)
    m1 = _edge_call(su1, geo4, w1['wg'], w1['e2w'], w1['e2b'],
                    w1['e3w'], w1['e3b'], w1['v2col'], w1['v2b'])
    msum1 = _scatter(m1, dst, zeros_tab)
    h1, pd2, ps2 = _node_proj(x, msum1, w1, w2['wcat'], w2['bcat'])

    su2 = _gather(pd2, ps2, src, dst)
    m2 = _edge_call(su2, geo4, w2['wg'], w2['e2w'], w2['e2b'],
                    w2['e3w'], w2['e3b'], w2['v2col'], w2['v2b'])
    msum2 = _scatter(m2, dst, zeros_tab)
    s, lat, mu, loss = _node_pool(
        h1, msum2, w2, bcol, pos, pool[0], pool[1].reshape(1, _K), out1[0],
        out1[1].reshape(1, 128), out2[0], out2[1].reshape(1, _LAT),
        latent_gain.reshape(1, _LAT))
    return (lat.reshape(_B, _K, _LAT), s, loss[0, 0],
            mu.reshape(_B, _K, 2))
